# Initial kernel scaffold; baseline (speedup 1.0000x reference)
import jax
import jax.numpy as jnp
from jax.experimental import pallas as pl


def kernel(node_features, edge_features, edge_index, node_tiers, w_node_score, b_node_score, w_edge_score, b_edge_score, wq, bq, wk, bk, wv, bv, wo, bo, w_c1, b_c1, w_c2, b_c2):
    raise NotImplementedError("write your pallas kernel here")


# speedup vs baseline: 23.5889x; 23.5889x over previous
import functools

import numpy as np
import jax
import jax.numpy as jnp
from jax import lax
from jax.experimental import pallas as pl
from jax.experimental.pallas import tpu as pltpu
from jax.experimental.pallas import tpu_sc as plsc

N = 10000
E = 320000
D_NODE = 128
D = 16
H = 4
DH = 4
NUM_CLASSES = 16
KN = int(0.4 * N)
KE = int(0.4 * E)

NC = 2
NS = 16
NW = NC * NS
BLK = 128
NBLK = E // BLK
BLK_PER_W = -(-NBLK // NW)
ROWS_PER_TILE = N // NS

NPAD = 10240

BE = 4000
GE = E // BE

_MSB = np.int32(-2147483648)
_LOW = np.int32(2147483647)


def _node_body(nf_ref, wns_ref, bns_ref, wk_ref, bk_ref, ns_ref, km_ref):
    nf = nf_ref[...]
    ns_ref[...] = nf @ wns_ref[...] + bns_ref[0:1, 0:1]
    km_ref[...] = nf @ wk_ref[...] + bk_ref[...]


def _edge_lin_body(ef_ref, wes_ref, bes_ref, out_ref):
    out_ref[...] = ef_ref[...] @ wes_ref[...] + bes_ref[0:1, 0:1]


def _f32_key(x):
    b = lax.bitcast_convert_type(x, jnp.int32)
    return jnp.where(b < 0, b ^ _LOW, b)


def _thresh_body(es_ref, ns_ref, out_ref):
    ekey = _f32_key(es_ref[...])
    nkey = _f32_key(ns_ref[...])

    def select(keys, kth):
        def body(i, prefix_bits):
            cand_bits = prefix_bits | lax.shift_left(np.int32(1), 31 - i)
            cand_s = cand_bits ^ _MSB
            cnt = jnp.sum((keys >= cand_s).astype(jnp.int32))
            return jnp.where(cnt >= kth, cand_bits, prefix_bits)
        bits = lax.fori_loop(0, 32, body, np.int32(0))
        return bits ^ _MSB

    eth_k = select(ekey, np.int32(KE))
    nth_k = select(nkey, np.int32(KN))
    row = jnp.concatenate(
        [jnp.full((1, 128), eth_k, jnp.int32),
         jnp.full((1, 128), nth_k, jnp.int32)], axis=0)
    inv = jnp.where(row < 0, row ^ _LOW, row)
    out_ref[...] = lax.bitcast_convert_type(inv, jnp.float32)


def _payload_body(ef_ref, kd_ref, es_ref, mn_ref, th_ref,
                  wq_ref, bq_ref, wv_ref, bv_ref,
                  wt_ref, pn_ref, pd_ref):
    eth = th_ref[0:1, 0:1]
    nth = th_ref[1:2, 0:1]
    m = ((es_ref[...] >= eth) & (mn_ref[...] >= nth)).astype(jnp.float32)
    w = ef_ref[...] * m
    q = w @ wq_ref[...] + bq_ref[...]
    v = w @ wv_ref[...] + bv_ref[...]
    p = q * kd_ref[...]
    s_i = lax.broadcasted_iota(jnp.int32, (D, H), 0) // DH
    s_h = lax.broadcasted_iota(jnp.int32, (D, H), 1)
    S = (s_i == s_h).astype(jnp.float32)
    logits = (p @ S) * (1.0 / (DH ** 0.5))
    ex = jnp.exp(logits)
    e_h = lax.broadcasted_iota(jnp.int32, (H, D), 0)
    e_j = lax.broadcasted_iota(jnp.int32, (H, D), 1) // DH
    S2 = (e_h == e_j).astype(jnp.float32)
    exb = ex @ S2
    wt_ref[...] = w
    pn_ref[...] = v * exb
    pd_ref[...] = jnp.concatenate(
        [ex, jnp.zeros((ex.shape[0], D - H), jnp.float32)], axis=1)


def _proj_body(an_ref, ad_ref, wo_ref, bo_ref, out_ref):
    numer = an_ref[0, :, :] + an_ref[1, :, :]
    den = ad_ref[0, :, :] + ad_ref[1, :, :]
    m_i = lax.broadcasted_iota(jnp.int32, (D, D), 0)
    m_j = lax.broadcasted_iota(jnp.int32, (D, D), 1) // DH
    M = (m_i == m_j).astype(jnp.float32)
    denb = den @ M
    pooled = numer / (denb + 1e-9)
    out_ref[...] = pooled @ wo_ref[...] + bo_ref[...]


def _erf(x):
    s = jnp.sign(x)
    a = jnp.abs(x)
    t = 1.0 / (1.0 + 0.3275911 * a)
    poly = ((((1.061405429 * t - 1.453152027) * t + 1.421413741) * t
             - 0.284496736) * t + 0.254829592) * t
    return s * (1.0 - poly * jnp.exp(-a * a))


def _head_body(wt_ref, pd_ref, w1_ref, b1_ref, w2_ref, b2_ref, out_ref):
    x = wt_ref[...] + pd_ref[...]
    h1 = x @ w1_ref[...] + b1_ref[...]
    h1 = 0.5 * h1 * (1.0 + _erf(h1 * 0.7071067811865476))
    out_ref[...] = h1 @ w2_ref[...] + b2_ref[...]


def _sc_score_gather_body(ns_hbm, src_hbm, dst_hbm, el_hbm, km_hbm,
                          es_hbm, mn_hbm, kd_hbm,
                          ns_v, src_v, dst_v, el_v, es_v, mn_v, kd_v, sem):
    wid = lax.axis_index("s") * NC + lax.axis_index("c")
    pltpu.sync_copy(ns_hbm, ns_v)

    def blk_body(i, _):
        blk = wid + i * NW

        @pl.when(blk < NBLK)
        def _():
            base = blk * BLK
            pltpu.sync_copy(src_hbm.at[pl.ds(base, BLK)], src_v)
            pltpu.sync_copy(dst_hbm.at[pl.ds(base, BLK)], dst_v)
            pltpu.sync_copy(el_hbm.at[pl.ds(base, BLK)], el_v)
            pltpu.async_copy(km_hbm.at[dst_v], kd_v, sem).wait()
            pltpu.sync_copy(kd_v, kd_hbm.at[pl.ds(base, BLK), :])

            def grp(g, _):
                sl = pl.ds(g * 16, 16)
                sidx = src_v[sl]
                didx = dst_v[sl]
                sv = plsc.load_gather(ns_v, [sidx])
                dv = plsc.load_gather(ns_v, [didx])
                es_v[sl] = el_v[sl] + 0.5 * (sv + dv)
                mn_v[sl] = jnp.minimum(sv, dv)
                return 0
            lax.fori_loop(0, BLK // 16, grp, 0, unroll=True)
            pltpu.sync_copy(es_v, es_hbm.at[pl.ds(base, BLK)])
            pltpu.sync_copy(mn_v, mn_hbm.at[pl.ds(base, BLK)])
        return 0
    lax.fori_loop(0, BLK_PER_W, blk_body, 0)


def _sc_scatter_body(pn_hbm, pd_hbm, dst_hbm, zz_hbm,
                     an_hbm, ad_hbm,
                     accn, accd, pn_v, pd_v, dst_v):
    cid = lax.axis_index("c")
    sid = lax.axis_index("s")
    wid = sid * NC + cid
    r0 = sid * ROWS_PER_TILE
    rsl = pl.ds(r0, ROWS_PER_TILE)
    pltpu.sync_copy(zz_hbm.at[rsl, :], accn.at[rsl, :])
    pltpu.sync_copy(zz_hbm.at[rsl, :], accd.at[rsl, :])
    plsc.subcore_barrier()

    def blk_body(i, _):
        blk = wid + i * NW

        @pl.when(blk < NBLK)
        def _():
            base = blk * BLK
            pltpu.sync_copy(dst_hbm.at[pl.ds(base, BLK)], dst_v)
            pltpu.sync_copy(pn_hbm.at[pl.ds(base, BLK), :], pn_v)
            pltpu.sync_copy(pd_hbm.at[pl.ds(base, BLK), :], pd_v)
            pltpu.sync_copy(pn_v, accn.at[dst_v], add=True)
            pltpu.sync_copy(pd_v, accd.at[dst_v], add=True)
        return 0
    lax.fori_loop(0, BLK_PER_W, blk_body, 0)
    plsc.subcore_barrier()
    pltpu.sync_copy(accn.at[rsl, :], an_hbm.at[cid, rsl, :])
    pltpu.sync_copy(accd.at[rsl, :], ad_hbm.at[cid, rsl, :])


def _sc_proj_gather_body(pr_hbm, dst_hbm, out_hbm, dst_v, row_v, sem):
    wid = lax.axis_index("s") * NC + lax.axis_index("c")

    def blk_body(i, _):
        blk = wid + i * NW

        @pl.when(blk < NBLK)
        def _():
            base = blk * BLK
            pltpu.sync_copy(dst_hbm.at[pl.ds(base, BLK)], dst_v)
            pltpu.async_copy(pr_hbm.at[dst_v], row_v, sem).wait()
            pltpu.sync_copy(row_v, out_hbm.at[pl.ds(base, BLK), :])
        return 0
    lax.fori_loop(0, BLK_PER_W, blk_body, 0)


_SC_MESH = plsc.VectorSubcoreMesh(core_axis_name="c", subcore_axis_name="s")
_SC_PARAMS = pltpu.CompilerParams(needs_layout_passes=False,
                                  use_tc_tiling_on_sc=False)
_f32 = jnp.float32


def kernel(node_features, edge_features, edge_index, node_tiers,
           w_node_score, b_node_score, w_edge_score, b_edge_score,
           wq, bq, wk, bk, wv, bv, wo, bo,
           w_c1, b_c1, w_c2, b_c2):
    del node_tiers
    src = edge_index[0].astype(jnp.int32)
    dst = edge_index[1].astype(jnp.int32)

    ns2, kmat = pl.pallas_call(
        _node_body,
        out_shape=[jax.ShapeDtypeStruct((N, 1), _f32),
                   jax.ShapeDtypeStruct((N, D), _f32)],
    )(node_features, w_node_score, b_node_score.reshape(1, 1),
      wk, bk.reshape(1, D))
    ns = ns2.reshape(N)

    el2 = pl.pallas_call(
        _edge_lin_body,
        grid=(GE,),
        in_specs=[pl.BlockSpec((BE, D), lambda i: (i, 0)),
                  pl.BlockSpec((D, 1), lambda i: (0, 0)),
                  pl.BlockSpec((1, 1), lambda i: (0, 0))],
        out_specs=pl.BlockSpec((BE, 1), lambda i: (i, 0)),
        out_shape=jax.ShapeDtypeStruct((E, 1), _f32),
    )(edge_features, w_edge_score, b_edge_score.reshape(1, 1))
    edge_lin = el2.reshape(E)

    sc_b = pl.kernel(
        _sc_score_gather_body,
        out_type=[jax.ShapeDtypeStruct((E,), _f32),
                  jax.ShapeDtypeStruct((E,), _f32),
                  jax.ShapeDtypeStruct((E, D), _f32)],
        mesh=_SC_MESH,
        scratch_types=[pltpu.VMEM((N,), _f32),
                       pltpu.VMEM((BLK,), jnp.int32),
                       pltpu.VMEM((BLK,), jnp.int32),
                       pltpu.VMEM((BLK,), _f32),
                       pltpu.VMEM((BLK,), _f32),
                       pltpu.VMEM((BLK,), _f32),
                       pltpu.VMEM((BLK, D), _f32),
                       pltpu.SemaphoreType.DMA],
        compiler_params=_SC_PARAMS,
    )
    edge_scores, min_ns, kdst = sc_b(ns, src, dst, edge_lin, kmat)

    ns_pad = jnp.pad(ns, (0, NPAD - N), constant_values=-jnp.inf)
    th = pl.pallas_call(
        _thresh_body,
        out_shape=jax.ShapeDtypeStruct((2, 128), _f32),
    )(edge_scores.reshape(NBLK, BLK), ns_pad.reshape(NPAD // 128, 128))

    weighted, pay_n, pay_d = pl.pallas_call(
        _payload_body,
        grid=(GE,),
        in_specs=[pl.BlockSpec((BE, D), lambda i: (i, 0)),
                  pl.BlockSpec((BE, D), lambda i: (i, 0)),
                  pl.BlockSpec((BE, 1), lambda i: (i, 0)),
                  pl.BlockSpec((BE, 1), lambda i: (i, 0)),
                  pl.BlockSpec((2, 128), lambda i: (0, 0)),
                  pl.BlockSpec((D, D), lambda i: (0, 0)),
                  pl.BlockSpec((1, D), lambda i: (0, 0)),
                  pl.BlockSpec((D, D), lambda i: (0, 0)),
                  pl.BlockSpec((1, D), lambda i: (0, 0))],
        out_specs=[pl.BlockSpec((BE, D), lambda i: (i, 0)),
                   pl.BlockSpec((BE, D), lambda i: (i, 0)),
                   pl.BlockSpec((BE, D), lambda i: (i, 0))],
        out_shape=[jax.ShapeDtypeStruct((E, D), _f32),
                   jax.ShapeDtypeStruct((E, D), _f32),
                   jax.ShapeDtypeStruct((E, D), _f32)],
    )(edge_features, kdst, edge_scores.reshape(E, 1), min_ns.reshape(E, 1),
      th, wq, bq.reshape(1, D), wv, bv.reshape(1, D))

    zeros_nd = jnp.zeros((N, D), _f32)
    sc_e = pl.kernel(
        _sc_scatter_body,
        out_type=[jax.ShapeDtypeStruct((NC, N, D), _f32),
                  jax.ShapeDtypeStruct((NC, N, D), _f32)],
        mesh=_SC_MESH,
        scratch_types=[pltpu.VMEM_SHARED((N, D), _f32),
                       pltpu.VMEM_SHARED((N, D), _f32),
                       pltpu.VMEM((BLK, D), _f32),
                       pltpu.VMEM((BLK, D), _f32),
                       pltpu.VMEM((BLK,), jnp.int32)],
        compiler_params=_SC_PARAMS,
    )
    acc_n, acc_d = sc_e(pay_n, pay_d, dst, zeros_nd)

    proj = pl.pallas_call(
        _proj_body,
        out_shape=jax.ShapeDtypeStruct((N, D), _f32),
    )(acc_n, acc_d, wo, bo.reshape(1, D))

    sc_g = pl.kernel(
        _sc_proj_gather_body,
        out_type=jax.ShapeDtypeStruct((E, D), _f32),
        mesh=_SC_MESH,
        scratch_types=[pltpu.VMEM((BLK,), jnp.int32),
                       pltpu.VMEM((BLK, D), _f32),
                       pltpu.SemaphoreType.DMA],
        compiler_params=_SC_PARAMS,
    )
    projd = sc_g(proj, dst)

    out = pl.pallas_call(
        _head_body,
        grid=(GE,),
        in_specs=[pl.BlockSpec((BE, D), lambda i: (i, 0)),
                  pl.BlockSpec((BE, D), lambda i: (i, 0)),
                  pl.BlockSpec((D, D), lambda i: (0, 0)),
                  pl.BlockSpec((1, D), lambda i: (0, 0)),
                  pl.BlockSpec((D, NUM_CLASSES), lambda i: (0, 0)),
                  pl.BlockSpec((1, NUM_CLASSES), lambda i: (0, 0))],
        out_specs=pl.BlockSpec((BE, NUM_CLASSES), lambda i: (i, 0)),
        out_shape=jax.ShapeDtypeStruct((E, NUM_CLASSES), _f32),
    )(weighted, projd, w_c1, b_c1.reshape(1, D), w_c2,
      b_c2.reshape(1, NUM_CLASSES))
    return out


# speedup vs baseline: 29.3811x; 1.2455x over previous
import functools

import numpy as np
import jax
import jax.numpy as jnp
from jax import lax
from jax.experimental import pallas as pl
from jax.experimental.pallas import tpu as pltpu
from jax.experimental.pallas import tpu_sc as plsc

N = 10000
E = 320000
D_NODE = 128
D = 16
H = 4
DH = 4
NUM_CLASSES = 16
KN = int(0.4 * N)
KE = int(0.4 * E)

NC = 2
NS = 16
NW = NC * NS
BLK = 128
NBLK = E // BLK
BLK_PER_W = -(-NBLK // NW)
ROWS_PER_TILE = N // NS

NPAD = 10240

BE = 6400
GE = E // BE

_MSB = np.int32(-2147483648)
_LOW = np.int32(2147483647)


def _node_body(nf_ref, wns_ref, bns_ref, wk_ref, bk_ref, ns_ref, km_ref):
    nf = nf_ref[...]
    ns_ref[...] = nf @ wns_ref[...] + bns_ref[0:1, 0:1]
    km_ref[...] = nf @ wk_ref[...] + bk_ref[...]


def _f32_key(x):
    b = lax.bitcast_convert_type(x, jnp.int32)
    return jnp.where(b < 0, b ^ _LOW, b)


def _thresh_body(ef_ref, sn_ref, mn_ref, ns_ref, wes_ref, bes_ref, mk_ref):
    k_i = lax.broadcasted_iota(jnp.int32, (BLK * D, D), 0)
    d_i = lax.broadcasted_iota(jnp.int32, (BLK * D, D), 1)
    M16T = ((k_i % D) == d_i).astype(jnp.float32)
    wtile = M16T @ wes_ref[...]
    b_k = lax.broadcasted_iota(jnp.int32, (BLK * D, BLK), 0) // D
    b_c = lax.broadcasted_iota(jnp.int32, (BLK * D, BLK), 1)
    W2 = (b_k == b_c).astype(jnp.float32) * wtile
    es2d = ef_ref[...] @ W2 + bes_ref[0:1, 0:1] + sn_ref[...]

    ekey = _f32_key(es2d)
    mkey = _f32_key(mn_ref[...])
    nkey = _f32_key(ns_ref[...])

    def select(keys, kth):
        def body(i, prefix_bits):
            cand_bits = prefix_bits | lax.shift_left(np.int32(1), 31 - i)
            cand_s = cand_bits ^ _MSB
            cnt = jnp.sum((keys >= cand_s).astype(jnp.int32))
            return jnp.where(cnt >= kth, cand_bits, prefix_bits)
        bits = lax.fori_loop(0, 32, body, np.int32(0))
        return bits ^ _MSB

    eth_k = select(ekey, np.int32(KE))
    nth_k = select(nkey, np.int32(KN))
    mk_ref[...] = ((ekey >= eth_k) & (mkey >= nth_k)).astype(jnp.float32)


def _payload_body(ef_ref, kd_ref, mr_ref,
                  wq_ref, bq_ref, wv_ref, bv_ref,
                  wt_ref, pn_ref, pd_ref):
    w = ef_ref[...] * mr_ref[...]
    q = w @ wq_ref[...] + bq_ref[...]
    v = w @ wv_ref[...] + bv_ref[...]
    p = q * kd_ref[...]
    s_i = lax.broadcasted_iota(jnp.int32, (D, H), 0) // DH
    s_h = lax.broadcasted_iota(jnp.int32, (D, H), 1)
    S = (s_i == s_h).astype(jnp.float32)
    logits = (p @ S) * (1.0 / (DH ** 0.5))
    ex = jnp.exp(logits)
    e_h = lax.broadcasted_iota(jnp.int32, (H, D), 0)
    e_j = lax.broadcasted_iota(jnp.int32, (H, D), 1) // DH
    S2 = (e_h == e_j).astype(jnp.float32)
    exb = ex @ S2
    wt_ref[...] = w
    pn_ref[...] = v * exb
    pd_ref[...] = jnp.concatenate(
        [ex, jnp.zeros((ex.shape[0], D - H), jnp.float32)], axis=1)


def _proj_body(an_ref, ad_ref, wo_ref, bo_ref, out_ref):
    numer = an_ref[0, :, :] + an_ref[1, :, :]
    den = ad_ref[0, :, :] + ad_ref[1, :, :]
    m_i = lax.broadcasted_iota(jnp.int32, (D, D), 0)
    m_j = lax.broadcasted_iota(jnp.int32, (D, D), 1) // DH
    M = (m_i == m_j).astype(jnp.float32)
    denb = den @ M
    pooled = numer / (denb + 1e-9)
    out_ref[...] = pooled @ wo_ref[...] + bo_ref[...]


def _erf(x):
    s = jnp.sign(x)
    a = jnp.abs(x)
    t = 1.0 / (1.0 + 0.3275911 * a)
    poly = ((((1.061405429 * t - 1.453152027) * t + 1.421413741) * t
             - 0.284496736) * t + 0.254829592) * t
    return s * (1.0 - poly * jnp.exp(-a * a))


def _head_body(wt_ref, pd_ref, w1_ref, b1_ref, w2_ref, b2_ref, out_ref):
    x = wt_ref[...] + pd_ref[...]
    h1 = x @ w1_ref[...] + b1_ref[...]
    h1 = 0.5 * h1 * (1.0 + _erf(h1 * 0.7071067811865476))
    out_ref[...] = h1 @ w2_ref[...] + b2_ref[...]


def _sc_score_gather_body(ns_hbm, src_hbm, dst_hbm, km_hbm,
                          sn_hbm, mn_hbm, kd_hbm,
                          ns_v, src_v, dst_v, sn_v, mn_v, kd_v, sem):
    wid = lax.axis_index("s") * NC + lax.axis_index("c")
    pltpu.sync_copy(ns_hbm, ns_v)

    def blk_body(i, _):
        blk = wid + i * NW

        @pl.when(blk < NBLK)
        def _():
            base = blk * BLK
            pltpu.sync_copy(src_hbm.at[pl.ds(base, BLK)], src_v)
            pltpu.sync_copy(dst_hbm.at[pl.ds(base, BLK)], dst_v)
            pltpu.async_copy(km_hbm.at[dst_v], kd_v, sem).wait()
            pltpu.sync_copy(kd_v, kd_hbm.at[pl.ds(base, BLK), :])

            def grp(g, _):
                sl = pl.ds(g * 16, 16)
                sidx = src_v[sl]
                didx = dst_v[sl]
                sv = plsc.load_gather(ns_v, [sidx])
                dv = plsc.load_gather(ns_v, [didx])
                sn_v[sl] = 0.5 * (sv + dv)
                mn_v[sl] = jnp.minimum(sv, dv)
                return 0
            lax.fori_loop(0, BLK // 16, grp, 0, unroll=True)
            pltpu.sync_copy(sn_v, sn_hbm.at[blk])
            pltpu.sync_copy(mn_v, mn_hbm.at[blk])
        return 0
    lax.fori_loop(0, BLK_PER_W, blk_body, 0)


def _sc_scatter_body(pn_hbm, pd_hbm, dst_hbm, zz_hbm,
                     an_hbm, ad_hbm,
                     accn, accd, pn_v, pd_v, dst_v):
    cid = lax.axis_index("c")
    sid = lax.axis_index("s")
    wid = sid * NC + cid
    r0 = sid * ROWS_PER_TILE
    rsl = pl.ds(r0, ROWS_PER_TILE)
    pltpu.sync_copy(zz_hbm.at[rsl, :], accn.at[rsl, :])
    pltpu.sync_copy(zz_hbm.at[rsl, :], accd.at[rsl, :])
    plsc.subcore_barrier()

    def blk_body(i, _):
        blk = wid + i * NW

        @pl.when(blk < NBLK)
        def _():
            base = blk * BLK
            pltpu.sync_copy(dst_hbm.at[pl.ds(base, BLK)], dst_v)
            pltpu.sync_copy(pn_hbm.at[pl.ds(base, BLK), :], pn_v)
            pltpu.sync_copy(pd_hbm.at[pl.ds(base, BLK), :], pd_v)
            pltpu.sync_copy(pn_v, accn.at[dst_v], add=True)
            pltpu.sync_copy(pd_v, accd.at[dst_v], add=True)
        return 0
    lax.fori_loop(0, BLK_PER_W, blk_body, 0)
    plsc.subcore_barrier()
    pltpu.sync_copy(accn.at[rsl, :], an_hbm.at[cid, rsl, :])
    pltpu.sync_copy(accd.at[rsl, :], ad_hbm.at[cid, rsl, :])


def _sc_proj_gather_body(pr_hbm, dst_hbm, out_hbm, dst_v, row_v, sem):
    wid = lax.axis_index("s") * NC + lax.axis_index("c")

    def blk_body(i, _):
        blk = wid + i * NW

        @pl.when(blk < NBLK)
        def _():
            base = blk * BLK
            pltpu.sync_copy(dst_hbm.at[pl.ds(base, BLK)], dst_v)
            pltpu.async_copy(pr_hbm.at[dst_v], row_v, sem).wait()
            pltpu.sync_copy(row_v, out_hbm.at[pl.ds(base, BLK), :])
        return 0
    lax.fori_loop(0, BLK_PER_W, blk_body, 0)


_SC_MESH = plsc.VectorSubcoreMesh(core_axis_name="c", subcore_axis_name="s")
_SC_PARAMS = pltpu.CompilerParams(needs_layout_passes=False,
                                  use_tc_tiling_on_sc=False)
_f32 = jnp.float32


def kernel(node_features, edge_features, edge_index, node_tiers,
           w_node_score, b_node_score, w_edge_score, b_edge_score,
           wq, bq, wk, bk, wv, bv, wo, bo,
           w_c1, b_c1, w_c2, b_c2):
    del node_tiers
    src = edge_index[0].astype(jnp.int32)
    dst = edge_index[1].astype(jnp.int32)

    ns2, kmat = pl.pallas_call(
        _node_body,
        out_shape=[jax.ShapeDtypeStruct((N, 1), _f32),
                   jax.ShapeDtypeStruct((N, D), _f32)],
    )(node_features, w_node_score, b_node_score.reshape(1, 1),
      wk, bk.reshape(1, D))
    ns = ns2.reshape(N)

    sc_b = pl.kernel(
        _sc_score_gather_body,
        out_type=[jax.ShapeDtypeStruct((NBLK, BLK), _f32),
                  jax.ShapeDtypeStruct((NBLK, BLK), _f32),
                  jax.ShapeDtypeStruct((E, D), _f32)],
        mesh=_SC_MESH,
        scratch_types=[pltpu.VMEM((N,), _f32),
                       pltpu.VMEM((BLK,), jnp.int32),
                       pltpu.VMEM((BLK,), jnp.int32),
                       pltpu.VMEM((BLK,), _f32),
                       pltpu.VMEM((BLK,), _f32),
                       pltpu.VMEM((BLK, D), _f32),
                       pltpu.SemaphoreType.DMA],
        compiler_params=_SC_PARAMS,
    )
    sumns, min_ns, kdst = sc_b(ns, src, dst, kmat)

    ns_pad = jnp.pad(ns, (0, NPAD - N), constant_values=-jnp.inf)
    mask2d = pl.pallas_call(
        _thresh_body,
        out_shape=jax.ShapeDtypeStruct((NBLK, BLK), _f32),
    )(edge_features.reshape(NBLK, BLK * D), sumns, min_ns,
      ns_pad.reshape(NPAD // 128, 128), w_edge_score,
      b_edge_score.reshape(1, 1))
    mask_rows = jnp.broadcast_to(mask2d.reshape(E, 1), (E, D))

    weighted, pay_n, pay_d = pl.pallas_call(
        _payload_body,
        grid=(GE,),
        in_specs=[pl.BlockSpec((BE, D), lambda i: (i, 0)),
                  pl.BlockSpec((BE, D), lambda i: (i, 0)),
                  pl.BlockSpec((BE, D), lambda i: (i, 0)),
                  pl.BlockSpec((D, D), lambda i: (0, 0)),
                  pl.BlockSpec((1, D), lambda i: (0, 0)),
                  pl.BlockSpec((D, D), lambda i: (0, 0)),
                  pl.BlockSpec((1, D), lambda i: (0, 0))],
        out_specs=[pl.BlockSpec((BE, D), lambda i: (i, 0)),
                   pl.BlockSpec((BE, D), lambda i: (i, 0)),
                   pl.BlockSpec((BE, D), lambda i: (i, 0))],
        out_shape=[jax.ShapeDtypeStruct((E, D), _f32),
                   jax.ShapeDtypeStruct((E, D), _f32),
                   jax.ShapeDtypeStruct((E, D), _f32)],
    )(edge_features, kdst, mask_rows,
      wq, bq.reshape(1, D), wv, bv.reshape(1, D))

    zeros_nd = jnp.zeros((N, D), _f32)
    sc_e = pl.kernel(
        _sc_scatter_body,
        out_type=[jax.ShapeDtypeStruct((NC, N, D), _f32),
                  jax.ShapeDtypeStruct((NC, N, D), _f32)],
        mesh=_SC_MESH,
        scratch_types=[pltpu.VMEM_SHARED((N, D), _f32),
                       pltpu.VMEM_SHARED((N, D), _f32),
                       pltpu.VMEM((BLK, D), _f32),
                       pltpu.VMEM((BLK, D), _f32),
                       pltpu.VMEM((BLK,), jnp.int32)],
        compiler_params=_SC_PARAMS,
    )
    acc_n, acc_d = sc_e(pay_n, pay_d, dst, zeros_nd)

    proj = pl.pallas_call(
        _proj_body,
        out_shape=jax.ShapeDtypeStruct((N, D), _f32),
    )(acc_n, acc_d, wo, bo.reshape(1, D))

    sc_g = pl.kernel(
        _sc_proj_gather_body,
        out_type=jax.ShapeDtypeStruct((E, D), _f32),
        mesh=_SC_MESH,
        scratch_types=[pltpu.VMEM((BLK,), jnp.int32),
                       pltpu.VMEM((BLK, D), _f32),
                       pltpu.SemaphoreType.DMA],
        compiler_params=_SC_PARAMS,
    )
    projd = sc_g(proj, dst)

    out = pl.pallas_call(
        _head_body,
        grid=(GE,),
        in_specs=[pl.BlockSpec((BE, D), lambda i: (i, 0)),
                  pl.BlockSpec((BE, D), lambda i: (i, 0)),
                  pl.BlockSpec((D, D), lambda i: (0, 0)),
                  pl.BlockSpec((1, D), lambda i: (0, 0)),
                  pl.BlockSpec((D, NUM_CLASSES), lambda i: (0, 0)),
                  pl.BlockSpec((1, NUM_CLASSES), lambda i: (0, 0))],
        out_specs=pl.BlockSpec((BE, NUM_CLASSES), lambda i: (i, 0)),
        out_shape=jax.ShapeDtypeStruct((E, NUM_CLASSES), _f32),
    )(weighted, projd, w_c1, b_c1.reshape(1, D), w_c2,
      b_c2.reshape(1, NUM_CLASSES))
    return out


# speedup vs baseline: 53.2525x; 1.8125x over previous
import functools

import numpy as np
import jax
import jax.numpy as jnp
from jax import lax
from jax.experimental import pallas as pl
from jax.experimental.pallas import tpu as pltpu
from jax.experimental.pallas import tpu_sc as plsc

N = 10000
E = 320000
D_NODE = 128
D = 16
H = 4
DH = 4
NUM_CLASSES = 16
KN = int(0.4 * N)
KE = int(0.4 * E)

NC = 2
NS = 16
NW = NC * NS
BLK = 128
NBLK = E // BLK
BLK_PER_W = -(-NBLK // NW)
ROWS_PER_TILE = N // NS

NPAD = 10240

BE = 6400
GE = E // BE
EP8 = E * D // 128
RB = BE * D // 128

_MSB = np.int32(-2147483648)
_LOW = np.int32(2147483647)


def _iota2(shape, dim):
    return lax.broadcasted_iota(jnp.int32, shape, dim)


def _blockdiag16(w16):
    p_r = _iota2((128, D), 0) % D
    p_c = _iota2((128, D), 1)
    P16 = (p_r == p_c).astype(jnp.float32)
    q_r = _iota2((D, 128), 0)
    q_c = _iota2((D, 128), 1) % D
    Q16 = (q_r == q_c).astype(jnp.float32)
    blk_ok = (_iota2((128, 128), 0) // D == _iota2((128, 128), 1) // D)
    return (P16 @ w16 @ Q16) * blk_ok.astype(jnp.float32), Q16


def _node_body(nf_ref, wns_ref, bns_ref, wk_ref, bk_ref, ns_ref, km_ref):
    nf = nf_ref[...]
    ns_ref[...] = nf @ wns_ref[...] + bns_ref[0:1, 0:1]
    km_ref[...] = nf @ wk_ref[...] + bk_ref[...]


def _f32_key(x):
    b = lax.bitcast_convert_type(x, jnp.int32)
    return jnp.where(b < 0, b ^ _LOW, b)


def _thresh_body(ef_ref, sn_ref, mn_ref, ns_ref, wes_ref, bes_ref, mk_ref):
    k_i = lax.broadcasted_iota(jnp.int32, (BLK * D, D), 0)
    d_i = lax.broadcasted_iota(jnp.int32, (BLK * D, D), 1)
    M16T = ((k_i % D) == d_i).astype(jnp.float32)
    wtile = M16T @ wes_ref[...]
    b_k = lax.broadcasted_iota(jnp.int32, (BLK * D, BLK), 0) // D
    b_c = lax.broadcasted_iota(jnp.int32, (BLK * D, BLK), 1)
    W2 = (b_k == b_c).astype(jnp.float32) * wtile
    es2d = ef_ref[...] @ W2 + bes_ref[0:1, 0:1] + sn_ref[...]

    ekey = _f32_key(es2d)
    mkey = _f32_key(mn_ref[...])
    nkey = _f32_key(ns_ref[...])

    def select(keys, kth):
        def body(i, prefix_bits):
            cand_bits = prefix_bits | lax.shift_left(np.int32(1), 31 - i)
            cand_s = cand_bits ^ _MSB
            cnt = jnp.sum((keys >= cand_s).astype(jnp.int32))
            return jnp.where(cnt >= kth, cand_bits, prefix_bits)
        bits = lax.fori_loop(0, 32, body, np.int32(0))
        return bits ^ _MSB

    eth_k = select(ekey, np.int32(KE))
    nth_k = select(nkey, np.int32(KN))
    mask2d = ((ekey >= eth_k) & (mkey >= nth_k)).astype(jnp.float32)
    e_c = _iota2((BLK, BLK * D), 0)
    e_j = _iota2((BLK, BLK * D), 1) // D
    EXPL = (e_c == e_j).astype(jnp.float32)
    mk_ref[...] = mask2d @ EXPL


def _payload_body(ef_ref, mk_ref, kd_ref, wq_ref, bq_ref, wv_ref, bv_ref,
                  wt_ref, pn_ref, pd_ref):
    w8 = ef_ref[...] * mk_ref[...]
    wt_ref[...] = w8
    W8q, Q16 = _blockdiag16(wq_ref[...])
    W8v, _ = _blockdiag16(wv_ref[...])
    q8 = w8 @ W8q + bq_ref[...] @ Q16
    v8 = w8 @ W8v + bv_ref[...] @ Q16
    p8 = q8 * kd_ref[...]
    r16 = _iota2((128, 128), 0) % D
    c16 = _iota2((128, 128), 1) % D
    blk_ok = (_iota2((128, 128), 0) // D == _iota2((128, 128), 1) // D)
    SB = (blk_ok & (r16 // DH == c16 // DH)).astype(jnp.float32)
    ex_big = jnp.exp((p8 @ SB) * (1.0 / (DH ** 0.5)))
    pn_ref[...] = v8 * ex_big
    SD = (blk_ok & (c16 < H) & (r16 // DH == c16)).astype(jnp.float32)
    lane4 = (_iota2((1, 128), 1) % D < H).astype(jnp.float32)
    pd_ref[...] = jnp.exp((p8 @ SD) * (1.0 / (DH ** 0.5))) * lane4


def _proj_body(an_ref, ad_ref, wo_ref, bo_ref, out_ref):
    numer = an_ref[0, :, :] + an_ref[1, :, :]
    den = ad_ref[0, :, :] + ad_ref[1, :, :]
    m_i = lax.broadcasted_iota(jnp.int32, (D, D), 0)
    m_j = lax.broadcasted_iota(jnp.int32, (D, D), 1) // DH
    M = (m_i == m_j).astype(jnp.float32)
    denb = den @ M
    pooled = numer / (denb + 1e-9)
    out_ref[...] = pooled @ wo_ref[...] + bo_ref[...]


def _erf(x):
    s = jnp.sign(x)
    a = jnp.abs(x)
    t = 1.0 / (1.0 + 0.3275911 * a)
    poly = ((((1.061405429 * t - 1.453152027) * t + 1.421413741) * t
             - 0.284496736) * t + 0.254829592) * t
    return s * (1.0 - poly * jnp.exp(-a * a))


def _head_body(wt_ref, pj_ref, w1_ref, b1_ref, w2_ref, b2_ref, out_ref):
    x8 = wt_ref[...] + pj_ref[...]
    W81, Q16 = _blockdiag16(w1_ref[...])
    W82, _ = _blockdiag16(w2_ref[...])
    h1 = x8 @ W81 + b1_ref[...] @ Q16
    h1 = 0.5 * h1 * (1.0 + _erf(h1 * 0.7071067811865476))
    out_ref[...] = h1 @ W82 + b2_ref[...] @ Q16


def _sc_score_gather_body(ns_hbm, src_hbm, dst_hbm, km_hbm,
                          sn_hbm, mn_hbm, kd_hbm,
                          ns_v, src_v, dst_v, sn_v, mn_v, kd_v, sem):
    wid = lax.axis_index("s") * NC + lax.axis_index("c")
    pltpu.sync_copy(ns_hbm, ns_v)

    def blk_body(i, _):
        blk = wid + i * NW

        @pl.when(blk < NBLK)
        def _():
            base = blk * BLK
            pltpu.sync_copy(src_hbm.at[pl.ds(base, BLK)], src_v)
            pltpu.sync_copy(dst_hbm.at[pl.ds(base, BLK)], dst_v)
            pltpu.async_copy(km_hbm.at[dst_v], kd_v, sem).wait()
            pltpu.sync_copy(kd_v, kd_hbm.at[pl.ds(base, BLK), :])

            def grp(g, _):
                sl = pl.ds(g * 16, 16)
                sidx = src_v[sl]
                didx = dst_v[sl]
                sv = plsc.load_gather(ns_v, [sidx])
                dv = plsc.load_gather(ns_v, [didx])
                sn_v[sl] = 0.5 * (sv + dv)
                mn_v[sl] = jnp.minimum(sv, dv)
                return 0
            lax.fori_loop(0, BLK // 16, grp, 0, unroll=True)
            pltpu.sync_copy(sn_v, sn_hbm.at[blk])
            pltpu.sync_copy(mn_v, mn_hbm.at[blk])
        return 0
    lax.fori_loop(0, BLK_PER_W, blk_body, 0)


def _sc_scatter_body(pn_hbm, pd_hbm, dst_hbm, zz_hbm,
                     an_hbm, ad_hbm,
                     accn, accd, pn_v, pd_v, dst_v):
    cid = lax.axis_index("c")
    sid = lax.axis_index("s")
    wid = sid * NC + cid
    r0 = sid * ROWS_PER_TILE
    rsl = pl.ds(r0, ROWS_PER_TILE)
    pltpu.sync_copy(zz_hbm.at[rsl, :], accn.at[rsl, :])
    pltpu.sync_copy(zz_hbm.at[rsl, :], accd.at[rsl, :])
    plsc.subcore_barrier()

    def blk_body(i, _):
        blk = wid + i * NW

        @pl.when(blk < NBLK)
        def _():
            base = blk * BLK
            pltpu.sync_copy(dst_hbm.at[pl.ds(base, BLK)], dst_v)
            pltpu.sync_copy(pn_hbm.at[pl.ds(base, BLK), :], pn_v)
            pltpu.sync_copy(pd_hbm.at[pl.ds(base, BLK), :], pd_v)
            pltpu.sync_copy(pn_v, accn.at[dst_v], add=True)
            pltpu.sync_copy(pd_v, accd.at[dst_v], add=True)
        return 0
    lax.fori_loop(0, BLK_PER_W, blk_body, 0)
    plsc.subcore_barrier()
    pltpu.sync_copy(accn.at[rsl, :], an_hbm.at[cid, rsl, :])
    pltpu.sync_copy(accd.at[rsl, :], ad_hbm.at[cid, rsl, :])


def _sc_proj_gather_body(pr_hbm, dst_hbm, out_hbm, dst_v, row_v, sem):
    wid = lax.axis_index("s") * NC + lax.axis_index("c")

    def blk_body(i, _):
        blk = wid + i * NW

        @pl.when(blk < NBLK)
        def _():
            base = blk * BLK
            pltpu.sync_copy(dst_hbm.at[pl.ds(base, BLK)], dst_v)
            pltpu.async_copy(pr_hbm.at[dst_v], row_v, sem).wait()
            pltpu.sync_copy(row_v, out_hbm.at[pl.ds(base, BLK), :])
        return 0
    lax.fori_loop(0, BLK_PER_W, blk_body, 0)


_SC_MESH = plsc.VectorSubcoreMesh(core_axis_name="c", subcore_axis_name="s")
_SC_PARAMS = pltpu.CompilerParams(needs_layout_passes=False,
                                  use_tc_tiling_on_sc=False)
_f32 = jnp.float32


def kernel(node_features, edge_features, edge_index, node_tiers,
           w_node_score, b_node_score, w_edge_score, b_edge_score,
           wq, bq, wk, bk, wv, bv, wo, bo,
           w_c1, b_c1, w_c2, b_c2):
    del node_tiers
    src = edge_index[0].astype(jnp.int32)
    dst = edge_index[1].astype(jnp.int32)

    ns2, kmat = pl.pallas_call(
        _node_body,
        out_shape=[jax.ShapeDtypeStruct((N, 1), _f32),
                   jax.ShapeDtypeStruct((N, D), _f32)],
    )(node_features, w_node_score, b_node_score.reshape(1, 1),
      wk, bk.reshape(1, D))
    ns = ns2.reshape(N)

    sc_b = pl.kernel(
        _sc_score_gather_body,
        out_type=[jax.ShapeDtypeStruct((NBLK, BLK), _f32),
                  jax.ShapeDtypeStruct((NBLK, BLK), _f32),
                  jax.ShapeDtypeStruct((E, D), _f32)],
        mesh=_SC_MESH,
        scratch_types=[pltpu.VMEM((N,), _f32),
                       pltpu.VMEM((BLK,), jnp.int32),
                       pltpu.VMEM((BLK,), jnp.int32),
                       pltpu.VMEM((BLK,), _f32),
                       pltpu.VMEM((BLK,), _f32),
                       pltpu.VMEM((BLK, D), _f32),
                       pltpu.SemaphoreType.DMA],
        compiler_params=_SC_PARAMS,
    )
    sumns, min_ns, kdst = sc_b(ns, src, dst, kmat)

    ns_pad = jnp.pad(ns, (0, NPAD - N), constant_values=-jnp.inf)
    mask2048 = pl.pallas_call(
        _thresh_body,
        out_shape=jax.ShapeDtypeStruct((NBLK, BLK * D), _f32),
    )(edge_features.reshape(NBLK, BLK * D), sumns, min_ns,
      ns_pad.reshape(NPAD // 128, 128), w_edge_score,
      b_edge_score.reshape(1, 1))
    ef8 = edge_features.reshape(EP8, 128)
    mk8 = mask2048.reshape(EP8, 128)
    kd8 = kdst.reshape(EP8, 128)

    w8, pay_n8, pay_d8 = pl.pallas_call(
        _payload_body,
        grid=(GE,),
        in_specs=[pl.BlockSpec((RB, 128), lambda i: (i, 0)),
                  pl.BlockSpec((RB, 128), lambda i: (i, 0)),
                  pl.BlockSpec((RB, 128), lambda i: (i, 0)),
                  pl.BlockSpec((D, D), lambda i: (0, 0)),
                  pl.BlockSpec((1, D), lambda i: (0, 0)),
                  pl.BlockSpec((D, D), lambda i: (0, 0)),
                  pl.BlockSpec((1, D), lambda i: (0, 0))],
        out_specs=[pl.BlockSpec((RB, 128), lambda i: (i, 0)),
                   pl.BlockSpec((RB, 128), lambda i: (i, 0)),
                   pl.BlockSpec((RB, 128), lambda i: (i, 0))],
        out_shape=[jax.ShapeDtypeStruct((EP8, 128), _f32),
                   jax.ShapeDtypeStruct((EP8, 128), _f32),
                   jax.ShapeDtypeStruct((EP8, 128), _f32)],
    )(ef8, mk8, kd8, wq, bq.reshape(1, D), wv, bv.reshape(1, D))
    pay_n = pay_n8.reshape(E, D)
    pay_d = pay_d8.reshape(E, D)

    zeros_nd = jnp.zeros((N, D), _f32)
    sc_e = pl.kernel(
        _sc_scatter_body,
        out_type=[jax.ShapeDtypeStruct((NC, N, D), _f32),
                  jax.ShapeDtypeStruct((NC, N, D), _f32)],
        mesh=_SC_MESH,
        scratch_types=[pltpu.VMEM_SHARED((N, D), _f32),
                       pltpu.VMEM_SHARED((N, D), _f32),
                       pltpu.VMEM((BLK, D), _f32),
                       pltpu.VMEM((BLK, D), _f32),
                       pltpu.VMEM((BLK,), jnp.int32)],
        compiler_params=_SC_PARAMS,
    )
    acc_n, acc_d = sc_e(pay_n, pay_d, dst, zeros_nd)

    proj = pl.pallas_call(
        _proj_body,
        out_shape=jax.ShapeDtypeStruct((N, D), _f32),
    )(acc_n, acc_d, wo, bo.reshape(1, D))

    sc_g = pl.kernel(
        _sc_proj_gather_body,
        out_type=jax.ShapeDtypeStruct((E, D), _f32),
        mesh=_SC_MESH,
        scratch_types=[pltpu.VMEM((BLK,), jnp.int32),
                       pltpu.VMEM((BLK, D), _f32),
                       pltpu.SemaphoreType.DMA],
        compiler_params=_SC_PARAMS,
    )
    projd = sc_g(proj, dst)

    pj8 = projd.reshape(EP8, 128)
    out8 = pl.pallas_call(
        _head_body,
        grid=(GE,),
        in_specs=[pl.BlockSpec((RB, 128), lambda i: (i, 0)),
                  pl.BlockSpec((RB, 128), lambda i: (i, 0)),
                  pl.BlockSpec((D, D), lambda i: (0, 0)),
                  pl.BlockSpec((1, D), lambda i: (0, 0)),
                  pl.BlockSpec((D, NUM_CLASSES), lambda i: (0, 0)),
                  pl.BlockSpec((1, NUM_CLASSES), lambda i: (0, 0))],
        out_specs=pl.BlockSpec((RB, 128), lambda i: (i, 0)),
        out_shape=jax.ShapeDtypeStruct((EP8, 128), _f32),
    )(w8, pj8, w_c1, b_c1.reshape(1, D), w_c2,
      b_c2.reshape(1, NUM_CLASSES))
    return out8.reshape(E, NUM_CLASSES)


# speedup vs baseline: 67.6433x; 1.2702x over previous
import functools

import numpy as np
import jax
import jax.numpy as jnp
from jax import lax
from jax.experimental import pallas as pl
from jax.experimental.pallas import tpu as pltpu
from jax.experimental.pallas import tpu_sc as plsc

N = 10000
E = 320000
D_NODE = 128
D = 16
H = 4
DH = 4
NUM_CLASSES = 16
KN = int(0.4 * N)
KE = int(0.4 * E)

NC = 2
NS = 16
NW = NC * NS
BLK = 128
NBLK = E // BLK
WB = NBLK // NW
TAIL0 = NW * WB
NTAIL = NBLK - TAIL0
GRP = 6
NGRP = WB // GRP
ROWS_PER_TILE = N // NS

NPAD = 10240

BE = 6400
GE = E // BE
EP8 = E * D // 128
RB = BE * D // 128

_MSB = np.int32(-2147483648)
_LOW = np.int32(2147483647)


def _iota2(shape, dim):
    return lax.broadcasted_iota(jnp.int32, shape, dim)


def _blockdiag16(w16):
    p_r = _iota2((128, D), 0) % D
    p_c = _iota2((128, D), 1)
    P16 = (p_r == p_c).astype(jnp.float32)
    q_r = _iota2((D, 128), 0)
    q_c = _iota2((D, 128), 1) % D
    Q16 = (q_r == q_c).astype(jnp.float32)
    blk_ok = (_iota2((128, 128), 0) // D == _iota2((128, 128), 1) // D)
    return (P16 @ w16 @ Q16) * blk_ok.astype(jnp.float32), Q16


def _node_body(nf_ref, wns_ref, bns_ref, wk_ref, bk_ref, ns_ref, km_ref):
    nf = nf_ref[...]
    ns_ref[...] = nf @ wns_ref[...] + bns_ref[0:1, 0:1]
    km_ref[...] = nf @ wk_ref[...] + bk_ref[...]


def _f32_key(x):
    b = lax.bitcast_convert_type(x, jnp.int32)
    return jnp.where(b < 0, b ^ _LOW, b)


def _thresh_body(ef_ref, sn_ref, mn_ref, ns_ref, wes_ref, bes_ref, mk_ref):
    k_i = lax.broadcasted_iota(jnp.int32, (BLK * D, D), 0)
    d_i = lax.broadcasted_iota(jnp.int32, (BLK * D, D), 1)
    M16T = ((k_i % D) == d_i).astype(jnp.float32)
    wtile = M16T @ wes_ref[...]
    b_k = lax.broadcasted_iota(jnp.int32, (BLK * D, BLK), 0) // D
    b_c = lax.broadcasted_iota(jnp.int32, (BLK * D, BLK), 1)
    W2 = (b_k == b_c).astype(jnp.float32) * wtile
    es2d = ef_ref[...] @ W2 + bes_ref[0:1, 0:1] + sn_ref[...]

    ekey = _f32_key(es2d)
    mkey = _f32_key(mn_ref[...])
    nkey = _f32_key(ns_ref[...])

    def select(keys, kth):
        def body(i, prefix_bits):
            cand_bits = prefix_bits | lax.shift_left(np.int32(1), 31 - i)
            cand_s = cand_bits ^ _MSB
            cnt = jnp.sum((keys >= cand_s).astype(jnp.int32))
            return jnp.where(cnt >= kth, cand_bits, prefix_bits)
        bits = lax.fori_loop(0, 32, body, np.int32(0))
        return bits ^ _MSB

    eth_k = select(ekey, np.int32(KE))
    nth_k = select(nkey, np.int32(KN))
    mask2d = ((ekey >= eth_k) & (mkey >= nth_k)).astype(jnp.float32)
    e_c = _iota2((BLK, BLK * D), 0)
    e_j = _iota2((BLK, BLK * D), 1) // D
    EXPL = (e_c == e_j).astype(jnp.float32)
    mk_ref[...] = mask2d @ EXPL


def _payload_body(ef_ref, mk_ref, kd_ref, wq_ref, bq_ref, wv_ref, bv_ref,
                  wt_ref, pn_ref, pd_ref):
    w8 = ef_ref[...] * mk_ref[...]
    wt_ref[...] = w8
    W8q, Q16 = _blockdiag16(wq_ref[...])
    W8v, _ = _blockdiag16(wv_ref[...])
    q8 = w8 @ W8q + bq_ref[...] @ Q16
    v8 = w8 @ W8v + bv_ref[...] @ Q16
    p8 = q8 * kd_ref[...]
    r16 = _iota2((128, 128), 0) % D
    c16 = _iota2((128, 128), 1) % D
    blk_ok = (_iota2((128, 128), 0) // D == _iota2((128, 128), 1) // D)
    SB = (blk_ok & (r16 // DH == c16 // DH)).astype(jnp.float32)
    ex_big = jnp.exp((p8 @ SB) * (1.0 / (DH ** 0.5)))
    pn_ref[...] = v8 * ex_big
    SD = (blk_ok & (c16 < H) & (r16 // DH == c16)).astype(jnp.float32)
    lane4 = (_iota2((1, 128), 1) % D < H).astype(jnp.float32)
    pd_ref[...] = jnp.exp((p8 @ SD) * (1.0 / (DH ** 0.5))) * lane4


def _proj_body(an_ref, ad_ref, wo_ref, bo_ref, out_ref):
    numer = an_ref[0, :, :] + an_ref[1, :, :]
    den = ad_ref[0, :, :] + ad_ref[1, :, :]
    m_i = lax.broadcasted_iota(jnp.int32, (D, D), 0)
    m_j = lax.broadcasted_iota(jnp.int32, (D, D), 1) // DH
    M = (m_i == m_j).astype(jnp.float32)
    denb = den @ M
    pooled = numer / (denb + 1e-9)
    out_ref[...] = pooled @ wo_ref[...] + bo_ref[...]


def _erf(x):
    s = jnp.sign(x)
    a = jnp.abs(x)
    t = 1.0 / (1.0 + 0.3275911 * a)
    poly = ((((1.061405429 * t - 1.453152027) * t + 1.421413741) * t
             - 0.284496736) * t + 0.254829592) * t
    return s * (1.0 - poly * jnp.exp(-a * a))


def _head_body(wt_ref, pj_ref, w1_ref, b1_ref, w2_ref, b2_ref, out_ref):
    x8 = wt_ref[...] + pj_ref[...]
    W81, Q16 = _blockdiag16(w1_ref[...])
    W82, _ = _blockdiag16(w2_ref[...])
    h1 = x8 @ W81 + b1_ref[...] @ Q16
    h1 = 0.5 * h1 * (1.0 + _erf(h1 * 0.7071067811865476))
    out_ref[...] = h1 @ W82 + b2_ref[...] @ Q16


def _sc_score_gather_body(ns_hbm, src_hbm, dst_hbm, km_hbm,
                          sn_hbm, mn_hbm, kd_hbm,
                          ns_v, src_v, dst_v, sn_v, mn_v,
                          kd0, kd1, sem_g, sw0, sw1):
    wid = lax.axis_index("s") * NC + lax.axis_index("c")
    b0 = wid * WB
    pltpu.sync_copy(ns_hbm, ns_v)
    pltpu.sync_copy(src_hbm.at[pl.ds(b0, WB), :], src_v)
    pltpu.sync_copy(dst_hbm.at[pl.ds(b0, WB), :], dst_v)

    def blk_compute(j, _):
        def grp(g, _):
            sl = pl.ds(g * 16, 16)
            sv = plsc.load_gather(ns_v, [src_v[j, sl]])
            dv = plsc.load_gather(ns_v, [dst_v[j, sl]])
            sn_v[j, sl] = 0.5 * (sv + dv)
            mn_v[j, sl] = jnp.minimum(sv, dv)
            return 0
        lax.fori_loop(0, BLK // 16, grp, 0, unroll=True)
        return 0
    lax.fori_loop(0, WB, blk_compute, 0)
    pltpu.sync_copy(sn_v, sn_hbm.at[pl.ds(b0, WB), :])
    pltpu.sync_copy(mn_v, mn_hbm.at[pl.ds(b0, WB), :])

    def run_group(g, kd_v, sem_w, first):
        if not first:
            pltpu.make_async_copy(
                kd_v, kd_hbm.at[pl.ds(0, GRP * BLK), :], sem_w).wait()
        for b in range(GRP):
            pltpu.async_copy(km_hbm.at[dst_v.at[g * GRP + b]],
                             kd_v.at[pl.ds(b * BLK, BLK), :], sem_g)
        pltpu.make_async_copy(
            km_hbm.at[pl.ds(0, GRP * BLK), :], kd_v, sem_g).wait()
        pltpu.async_copy(kd_v, kd_hbm.at[pl.ds((b0 + g * GRP) * BLK,
                                               GRP * BLK), :], sem_w)

    def pair_body(i, _):
        @pl.when(i == 0)
        def _():
            run_group(2 * i, kd0, sw0, True)
            run_group(2 * i + 1, kd1, sw1, True)

        @pl.when(i > 0)
        def _():
            run_group(2 * i, kd0, sw0, False)
            run_group(2 * i + 1, kd1, sw1, False)
        return 0
    lax.fori_loop(0, (NGRP - 1) // 2, pair_body, 0)
    run_group(NGRP - 1, kd0, sw0, False)
    pltpu.make_async_copy(kd0, kd_hbm.at[pl.ds(0, GRP * BLK), :], sw0).wait()
    pltpu.make_async_copy(kd1, kd_hbm.at[pl.ds(0, GRP * BLK), :], sw1).wait()

    @pl.when(wid < NTAIL)
    def _():
        tb = TAIL0 + wid
        pltpu.sync_copy(src_hbm.at[tb], src_v.at[0])
        pltpu.sync_copy(dst_hbm.at[tb], dst_v.at[0])
        pltpu.async_copy(km_hbm.at[dst_v.at[0]],
                         kd0.at[pl.ds(0, BLK), :], sem_g).wait()
        pltpu.sync_copy(kd0.at[pl.ds(0, BLK), :],
                        kd_hbm.at[pl.ds(tb * BLK, BLK), :])

        def grp(g, _):
            sl = pl.ds(g * 16, 16)
            sv = plsc.load_gather(ns_v, [src_v[0, sl]])
            dv = plsc.load_gather(ns_v, [dst_v[0, sl]])
            sn_v[0, sl] = 0.5 * (sv + dv)
            mn_v[0, sl] = jnp.minimum(sv, dv)
            return 0
        lax.fori_loop(0, BLK // 16, grp, 0, unroll=True)
        pltpu.sync_copy(sn_v.at[0], sn_hbm.at[tb])
        pltpu.sync_copy(mn_v.at[0], mn_hbm.at[tb])


def _sc_scatter_body(pn_hbm, pd_hbm, dst_hbm, zz_hbm,
                     an_hbm, ad_hbm,
                     accn, accd, pn0, pn1, pd0, pd1, dst_v, semA, semB):
    cid = lax.axis_index("c")
    sid = lax.axis_index("s")
    wid = sid * NC + cid
    b0 = wid * WB
    r0 = sid * ROWS_PER_TILE
    rsl = pl.ds(r0, ROWS_PER_TILE)
    pltpu.sync_copy(zz_hbm.at[rsl, :], accn.at[rsl, :])
    pltpu.sync_copy(zz_hbm.at[rsl, :], accd.at[rsl, :])
    pltpu.sync_copy(dst_hbm.at[pl.ds(b0, WB), :], dst_v)
    plsc.subcore_barrier()

    def drain(pn_v, pd_v, sem):
        for b in range(GRP):
            pltpu.make_async_copy(pn_v.at[pl.ds(b * BLK, BLK), :],
                                  accn.at[dst_v.at[0]], sem).wait()
            pltpu.make_async_copy(pd_v.at[pl.ds(b * BLK, BLK), :],
                                  accd.at[dst_v.at[0]], sem).wait()

    def run_group(g, pn_v, pd_v, sem, first):
        if not first:
            drain(pn_v, pd_v, sem)
        base = (b0 + g * GRP) * BLK
        pltpu.sync_copy(pn_hbm.at[pl.ds(base, GRP * BLK), :], pn_v)
        pltpu.sync_copy(pd_hbm.at[pl.ds(base, GRP * BLK), :], pd_v)
        for b in range(GRP):
            idx = dst_v.at[g * GRP + b]
            pltpu.async_copy(pn_v.at[pl.ds(b * BLK, BLK), :],
                             accn.at[idx], sem, add=True)
            pltpu.async_copy(pd_v.at[pl.ds(b * BLK, BLK), :],
                             accd.at[idx], sem, add=True)

    def pair_body(i, _):
        @pl.when(i == 0)
        def _():
            run_group(2 * i, pn0, pd0, semA, True)
            run_group(2 * i + 1, pn1, pd1, semB, True)

        @pl.when(i > 0)
        def _():
            run_group(2 * i, pn0, pd0, semA, False)
            run_group(2 * i + 1, pn1, pd1, semB, False)
        return 0
    lax.fori_loop(0, (NGRP - 1) // 2, pair_body, 0)
    run_group(NGRP - 1, pn0, pd0, semA, False)
    drain(pn0, pd0, semA)
    drain(pn1, pd1, semB)

    @pl.when(wid < NTAIL)
    def _():
        tb = TAIL0 + wid
        pltpu.sync_copy(dst_hbm.at[tb], dst_v.at[0])
        pltpu.sync_copy(pn_hbm.at[pl.ds(tb * BLK, BLK), :],
                        pn0.at[pl.ds(0, BLK), :])
        pltpu.sync_copy(pd_hbm.at[pl.ds(tb * BLK, BLK), :],
                        pd0.at[pl.ds(0, BLK), :])
        pltpu.sync_copy(pn0.at[pl.ds(0, BLK), :],
                        accn.at[dst_v.at[0]], add=True)
        pltpu.sync_copy(pd0.at[pl.ds(0, BLK), :],
                        accd.at[dst_v.at[0]], add=True)

    plsc.subcore_barrier()
    pltpu.sync_copy(accn.at[rsl, :], an_hbm.at[cid, rsl, :])
    pltpu.sync_copy(accd.at[rsl, :], ad_hbm.at[cid, rsl, :])


def _sc_proj_gather_body(pr_hbm, dst_hbm, out_hbm,
                         dst_v, pr0, pr1, sem_g, sw0, sw1):
    wid = lax.axis_index("s") * NC + lax.axis_index("c")
    b0 = wid * WB
    pltpu.sync_copy(dst_hbm.at[pl.ds(b0, WB), :], dst_v)

    def run_group(g, pr_v, sem_w, first):
        if not first:
            pltpu.make_async_copy(
                pr_v, out_hbm.at[pl.ds(0, GRP * BLK), :], sem_w).wait()
        for b in range(GRP):
            pltpu.async_copy(pr_hbm.at[dst_v.at[g * GRP + b]],
                             pr_v.at[pl.ds(b * BLK, BLK), :], sem_g)
        pltpu.make_async_copy(
            pr_hbm.at[pl.ds(0, GRP * BLK), :], pr_v, sem_g).wait()
        pltpu.async_copy(pr_v, out_hbm.at[pl.ds((b0 + g * GRP) * BLK,
                                                GRP * BLK), :], sem_w)

    def pair_body(i, _):
        @pl.when(i == 0)
        def _():
            run_group(2 * i, pr0, sw0, True)
            run_group(2 * i + 1, pr1, sw1, True)

        @pl.when(i > 0)
        def _():
            run_group(2 * i, pr0, sw0, False)
            run_group(2 * i + 1, pr1, sw1, False)
        return 0
    lax.fori_loop(0, (NGRP - 1) // 2, pair_body, 0)
    run_group(NGRP - 1, pr0, sw0, False)
    pltpu.make_async_copy(pr0, out_hbm.at[pl.ds(0, GRP * BLK), :], sw0).wait()
    pltpu.make_async_copy(pr1, out_hbm.at[pl.ds(0, GRP * BLK), :], sw1).wait()

    @pl.when(wid < NTAIL)
    def _():
        tb = TAIL0 + wid
        pltpu.sync_copy(dst_hbm.at[tb], dst_v.at[0])
        pltpu.async_copy(pr_hbm.at[dst_v.at[0]],
                         pr0.at[pl.ds(0, BLK), :], sem_g).wait()
        pltpu.sync_copy(pr0.at[pl.ds(0, BLK), :],
                        out_hbm.at[pl.ds(tb * BLK, BLK), :])


_SC_MESH = plsc.VectorSubcoreMesh(core_axis_name="c", subcore_axis_name="s")
_SC_PARAMS = pltpu.CompilerParams(needs_layout_passes=False,
                                  use_tc_tiling_on_sc=False)
_f32 = jnp.float32


def kernel(node_features, edge_features, edge_index, node_tiers,
           w_node_score, b_node_score, w_edge_score, b_edge_score,
           wq, bq, wk, bk, wv, bv, wo, bo,
           w_c1, b_c1, w_c2, b_c2):
    del node_tiers
    src2d = edge_index[0].astype(jnp.int32).reshape(NBLK, BLK)
    dst2d = edge_index[1].astype(jnp.int32).reshape(NBLK, BLK)

    ns2, kmat = pl.pallas_call(
        _node_body,
        out_shape=[jax.ShapeDtypeStruct((N, 1), _f32),
                   jax.ShapeDtypeStruct((N, D), _f32)],
    )(node_features, w_node_score, b_node_score.reshape(1, 1),
      wk, bk.reshape(1, D))
    ns = ns2.reshape(N)

    sc_b = pl.kernel(
        _sc_score_gather_body,
        out_type=[jax.ShapeDtypeStruct((NBLK, BLK), _f32),
                  jax.ShapeDtypeStruct((NBLK, BLK), _f32),
                  jax.ShapeDtypeStruct((E, D), _f32)],
        mesh=_SC_MESH,
        scratch_types=[pltpu.VMEM((N,), _f32),
                       pltpu.VMEM((WB, BLK), jnp.int32),
                       pltpu.VMEM((WB, BLK), jnp.int32),
                       pltpu.VMEM((WB, BLK), _f32),
                       pltpu.VMEM((WB, BLK), _f32),
                       pltpu.VMEM((GRP * BLK, D), _f32),
                       pltpu.VMEM((GRP * BLK, D), _f32),
                       pltpu.SemaphoreType.DMA,
                       pltpu.SemaphoreType.DMA,
                       pltpu.SemaphoreType.DMA],
        compiler_params=_SC_PARAMS,
    )
    sumns, min_ns, kdst = sc_b(ns, src2d, dst2d, kmat)

    ns_pad = jnp.pad(ns, (0, NPAD - N), constant_values=-jnp.inf)
    ef2048 = edge_features.reshape(NBLK, BLK * D)
    mask2048 = pl.pallas_call(
        _thresh_body,
        out_shape=jax.ShapeDtypeStruct((NBLK, BLK * D), _f32),
    )(ef2048, sumns, min_ns,
      ns_pad.reshape(NPAD // 128, 128), w_edge_score,
      b_edge_score.reshape(1, 1))
    ef8 = ef2048.reshape(EP8, 128)
    mk8 = mask2048.reshape(EP8, 128)
    kd8 = kdst.reshape(EP8, 128)

    w8, pay_n8, pay_d8 = pl.pallas_call(
        _payload_body,
        grid=(GE,),
        in_specs=[pl.BlockSpec((RB, 128), lambda i: (i, 0)),
                  pl.BlockSpec((RB, 128), lambda i: (i, 0)),
                  pl.BlockSpec((RB, 128), lambda i: (i, 0)),
                  pl.BlockSpec((D, D), lambda i: (0, 0)),
                  pl.BlockSpec((1, D), lambda i: (0, 0)),
                  pl.BlockSpec((D, D), lambda i: (0, 0)),
                  pl.BlockSpec((1, D), lambda i: (0, 0))],
        out_specs=[pl.BlockSpec((RB, 128), lambda i: (i, 0)),
                   pl.BlockSpec((RB, 128), lambda i: (i, 0)),
                   pl.BlockSpec((RB, 128), lambda i: (i, 0))],
        out_shape=[jax.ShapeDtypeStruct((EP8, 128), _f32),
                   jax.ShapeDtypeStruct((EP8, 128), _f32),
                   jax.ShapeDtypeStruct((EP8, 128), _f32)],
    )(ef8, mk8, kd8, wq, bq.reshape(1, D), wv, bv.reshape(1, D))
    pay_n = pay_n8.reshape(E, D)
    pay_d = pay_d8.reshape(E, D)

    zeros_nd = jnp.zeros((N, D), _f32)
    sc_e = pl.kernel(
        _sc_scatter_body,
        out_type=[jax.ShapeDtypeStruct((NC, N, D), _f32),
                  jax.ShapeDtypeStruct((NC, N, D), _f32)],
        mesh=_SC_MESH,
        scratch_types=[pltpu.VMEM_SHARED((N, D), _f32),
                       pltpu.VMEM_SHARED((N, D), _f32),
                       pltpu.VMEM((GRP * BLK, D), _f32),
                       pltpu.VMEM((GRP * BLK, D), _f32),
                       pltpu.VMEM((GRP * BLK, D), _f32),
                       pltpu.VMEM((GRP * BLK, D), _f32),
                       pltpu.VMEM((WB, BLK), jnp.int32),
                       pltpu.SemaphoreType.DMA,
                       pltpu.SemaphoreType.DMA],
        compiler_params=_SC_PARAMS,
    )
    acc_n, acc_d = sc_e(pay_n, pay_d, dst2d, zeros_nd)

    proj = pl.pallas_call(
        _proj_body,
        out_shape=jax.ShapeDtypeStruct((N, D), _f32),
    )(acc_n, acc_d, wo, bo.reshape(1, D))

    sc_g = pl.kernel(
        _sc_proj_gather_body,
        out_type=jax.ShapeDtypeStruct((E, D), _f32),
        mesh=_SC_MESH,
        scratch_types=[pltpu.VMEM((WB, BLK), jnp.int32),
                       pltpu.VMEM((GRP * BLK, D), _f32),
                       pltpu.VMEM((GRP * BLK, D), _f32),
                       pltpu.SemaphoreType.DMA,
                       pltpu.SemaphoreType.DMA,
                       pltpu.SemaphoreType.DMA],
        compiler_params=_SC_PARAMS,
    )
    projd = sc_g(proj, dst2d)

    pj8 = projd.reshape(EP8, 128)
    out8 = pl.pallas_call(
        _head_body,
        grid=(GE,),
        in_specs=[pl.BlockSpec((RB, 128), lambda i: (i, 0)),
                  pl.BlockSpec((RB, 128), lambda i: (i, 0)),
                  pl.BlockSpec((D, D), lambda i: (0, 0)),
                  pl.BlockSpec((1, D), lambda i: (0, 0)),
                  pl.BlockSpec((D, NUM_CLASSES), lambda i: (0, 0)),
                  pl.BlockSpec((1, NUM_CLASSES), lambda i: (0, 0))],
        out_specs=pl.BlockSpec((RB, 128), lambda i: (i, 0)),
        out_shape=jax.ShapeDtypeStruct((EP8, 128), _f32),
    )(w8, pj8, w_c1, b_c1.reshape(1, D), w_c2,
      b_c2.reshape(1, NUM_CLASSES))
    return out8.reshape(E, NUM_CLASSES)


# speedup vs baseline: 79.2927x; 1.1722x over previous
import functools

import numpy as np
import jax
import jax.numpy as jnp
from jax import lax
from jax.experimental import pallas as pl
from jax.experimental.pallas import tpu as pltpu
from jax.experimental.pallas import tpu_sc as plsc

N = 10000
E = 320000
D_NODE = 128
D = 16
H = 4
DH = 4
NUM_CLASSES = 16
KN = int(0.4 * N)
KE = int(0.4 * E)

NC = 2
NS = 16
NW = NC * NS
BLK = 128
NBLK = E // BLK
WB = NBLK // NW
TAIL0 = NW * WB
NTAIL = NBLK - TAIL0
GRP = 6
NGRP = WB // GRP
ROWS_PER_TILE = N // NS

NPAD = 10240

BE = 6400
GE = E // BE
EP8 = E * D // 128
RB = BE * D // 128

_MSB = np.int32(-2147483648)
_LOW = np.int32(2147483647)


def _iota2(shape, dim):
    return lax.broadcasted_iota(jnp.int32, shape, dim)


def _blockdiag16(w16):
    p_r = _iota2((128, D), 0) % D
    p_c = _iota2((128, D), 1)
    P16 = (p_r == p_c).astype(jnp.float32)
    q_r = _iota2((D, 128), 0)
    q_c = _iota2((D, 128), 1) % D
    Q16 = (q_r == q_c).astype(jnp.float32)
    blk_ok = (_iota2((128, 128), 0) // D == _iota2((128, 128), 1) // D)
    return (P16 @ w16 @ Q16) * blk_ok.astype(jnp.float32), Q16


def _node_body(nf_ref, wns_ref, bns_ref, wk_ref, bk_ref, ns_ref, km_ref):
    nf = nf_ref[...]
    ns_ref[...] = nf @ wns_ref[...] + bns_ref[0:1, 0:1]
    km_ref[...] = nf @ wk_ref[...] + bk_ref[...]


def _f32_key(x):
    b = lax.bitcast_convert_type(x, jnp.int32)
    return jnp.where(b < 0, b ^ _LOW, b)


def _thresh_body(ef_ref, sn_ref, mn_ref, ns_ref, wes_ref, bes_ref, mk_ref):
    k_i = lax.broadcasted_iota(jnp.int32, (BLK * D, D), 0)
    d_i = lax.broadcasted_iota(jnp.int32, (BLK * D, D), 1)
    M16T = ((k_i % D) == d_i).astype(jnp.float32)
    wtile = M16T @ wes_ref[...]
    b_k = lax.broadcasted_iota(jnp.int32, (BLK * D, BLK), 0) // D
    b_c = lax.broadcasted_iota(jnp.int32, (BLK * D, BLK), 1)
    W2 = (b_k == b_c).astype(jnp.float32) * wtile
    es2d = ef_ref[...] @ W2 + bes_ref[0:1, 0:1] + sn_ref[...]

    ekey = _f32_key(es2d)
    mkey = _f32_key(mn_ref[...])
    nkey = _f32_key(ns_ref[...])

    def select(keys, kth):
        def body(i, prefix_bits):
            cand_bits = prefix_bits | lax.shift_left(np.int32(1), 31 - i)
            cand_s = cand_bits ^ _MSB
            cnt = jnp.sum((keys >= cand_s).astype(jnp.int32))
            return jnp.where(cnt >= kth, cand_bits, prefix_bits)
        bits = lax.fori_loop(0, 32, body, np.int32(0))
        return bits ^ _MSB

    eth_k = select(ekey, np.int32(KE))
    nth_k = select(nkey, np.int32(KN))
    mask2d = ((ekey >= eth_k) & (mkey >= nth_k)).astype(jnp.float32)
    e_c = _iota2((BLK, BLK * D), 0)
    e_j = _iota2((BLK, BLK * D), 1) // D
    EXPL = (e_c == e_j).astype(jnp.float32)
    for st, sz in ((0, 624), (624, 624), (1248, 624), (1872, 628)):
        sl = pl.ds(st, sz)
        mk_ref[sl, :] = ef_ref[sl, :] * (mask2d[st:st + sz, :] @ EXPL)


def _payload_body(w_ref, kd_ref, wq_ref, bq_ref, wv_ref, bv_ref,
                  pn_ref, pd_ref):
    w8 = w_ref[...]
    W8q, Q16 = _blockdiag16(wq_ref[...])
    W8v, _ = _blockdiag16(wv_ref[...])
    q8 = w8 @ W8q + bq_ref[...] @ Q16
    v8 = w8 @ W8v + bv_ref[...] @ Q16
    p8 = q8 * kd_ref[...]
    r16 = _iota2((128, 128), 0) % D
    c16 = _iota2((128, 128), 1) % D
    blk_ok = (_iota2((128, 128), 0) // D == _iota2((128, 128), 1) // D)
    SB = (blk_ok & (r16 // DH == c16 // DH)).astype(jnp.float32)
    ex_big = jnp.exp((p8 @ SB) * (1.0 / (DH ** 0.5)))
    pn_ref[...] = v8 * ex_big
    SD = (blk_ok & (c16 < H) & (r16 // DH == c16)).astype(jnp.float32)
    lane4 = (_iota2((1, 128), 1) % D < H).astype(jnp.float32)
    pd_ref[...] = jnp.exp((p8 @ SD) * (1.0 / (DH ** 0.5))) * lane4


def _proj_body(an_ref, ad_ref, wo_ref, bo_ref, out_ref):
    numer = an_ref[0, :, :] + an_ref[1, :, :]
    den = ad_ref[0, :, :] + ad_ref[1, :, :]
    m_i = lax.broadcasted_iota(jnp.int32, (D, D), 0)
    m_j = lax.broadcasted_iota(jnp.int32, (D, D), 1) // DH
    M = (m_i == m_j).astype(jnp.float32)
    denb = den @ M
    pooled = numer / (denb + 1e-9)
    out_ref[...] = pooled @ wo_ref[...] + bo_ref[...]


def _erf(x):
    s = jnp.sign(x)
    a = jnp.abs(x)
    t = 1.0 / (1.0 + 0.3275911 * a)
    poly = ((((1.061405429 * t - 1.453152027) * t + 1.421413741) * t
             - 0.284496736) * t + 0.254829592) * t
    return s * (1.0 - poly * jnp.exp(-a * a))


def _head_body(wt_ref, pj_ref, w1_ref, b1_ref, w2_ref, b2_ref, out_ref):
    x8 = wt_ref[...] + pj_ref[...]
    W81, Q16 = _blockdiag16(w1_ref[...])
    W82, _ = _blockdiag16(w2_ref[...])
    h1 = x8 @ W81 + b1_ref[...] @ Q16
    h1 = 0.5 * h1 * (1.0 + _erf(h1 * 0.7071067811865476))
    out_ref[...] = h1 @ W82 + b2_ref[...] @ Q16


def _sc_score_gather_body(ns_hbm, src_hbm, dst_hbm, km_hbm,
                          sn_hbm, mn_hbm, kd_hbm,
                          ns_v, src_v, dst_v, sn_v, mn_v,
                          kd0, kd1, sem_g, sw0, sw1):
    wid = lax.axis_index("s") * NC + lax.axis_index("c")
    b0 = wid * WB
    pltpu.sync_copy(ns_hbm, ns_v)
    pltpu.sync_copy(src_hbm.at[pl.ds(b0, WB), :], src_v)
    pltpu.sync_copy(dst_hbm.at[pl.ds(b0, WB), :], dst_v)

    def blk_compute(j, _):
        def grp(g, _):
            sl = pl.ds(g * 16, 16)
            sv = plsc.load_gather(ns_v, [src_v[j, sl]])
            dv = plsc.load_gather(ns_v, [dst_v[j, sl]])
            sn_v[j, sl] = 0.5 * (sv + dv)
            mn_v[j, sl] = jnp.minimum(sv, dv)
            return 0
        lax.fori_loop(0, BLK // 16, grp, 0, unroll=True)
        return 0
    lax.fori_loop(0, WB, blk_compute, 0)
    pltpu.sync_copy(sn_v, sn_hbm.at[pl.ds(b0, WB), :])
    pltpu.sync_copy(mn_v, mn_hbm.at[pl.ds(b0, WB), :])

    def run_group(g, kd_v, sem_w, first):
        if not first:
            pltpu.make_async_copy(
                kd_v, kd_hbm.at[pl.ds(0, GRP * BLK), :], sem_w).wait()
        for b in range(GRP):
            pltpu.async_copy(km_hbm.at[dst_v.at[g * GRP + b]],
                             kd_v.at[pl.ds(b * BLK, BLK), :], sem_g)
        pltpu.make_async_copy(
            km_hbm.at[pl.ds(0, GRP * BLK), :], kd_v, sem_g).wait()
        pltpu.async_copy(kd_v, kd_hbm.at[pl.ds((b0 + g * GRP) * BLK,
                                               GRP * BLK), :], sem_w)

    def pair_body(i, _):
        @pl.when(i == 0)
        def _():
            run_group(2 * i, kd0, sw0, True)
            run_group(2 * i + 1, kd1, sw1, True)

        @pl.when(i > 0)
        def _():
            run_group(2 * i, kd0, sw0, False)
            run_group(2 * i + 1, kd1, sw1, False)
        return 0
    lax.fori_loop(0, (NGRP - 1) // 2, pair_body, 0)
    run_group(NGRP - 1, kd0, sw0, False)
    pltpu.make_async_copy(kd0, kd_hbm.at[pl.ds(0, GRP * BLK), :], sw0).wait()
    pltpu.make_async_copy(kd1, kd_hbm.at[pl.ds(0, GRP * BLK), :], sw1).wait()

    @pl.when(wid < NTAIL)
    def _():
        tb = TAIL0 + wid
        pltpu.sync_copy(src_hbm.at[tb], src_v.at[0])
        pltpu.sync_copy(dst_hbm.at[tb], dst_v.at[0])
        pltpu.async_copy(km_hbm.at[dst_v.at[0]],
                         kd0.at[pl.ds(0, BLK), :], sem_g).wait()
        pltpu.sync_copy(kd0.at[pl.ds(0, BLK), :],
                        kd_hbm.at[pl.ds(tb * BLK, BLK), :])

        def grp(g, _):
            sl = pl.ds(g * 16, 16)
            sv = plsc.load_gather(ns_v, [src_v[0, sl]])
            dv = plsc.load_gather(ns_v, [dst_v[0, sl]])
            sn_v[0, sl] = 0.5 * (sv + dv)
            mn_v[0, sl] = jnp.minimum(sv, dv)
            return 0
        lax.fori_loop(0, BLK // 16, grp, 0, unroll=True)
        pltpu.sync_copy(sn_v.at[0], sn_hbm.at[tb])
        pltpu.sync_copy(mn_v.at[0], mn_hbm.at[tb])


def _sc_scatter_body(pn_hbm, pd_hbm, dst_hbm, zz_hbm,
                     an_hbm, ad_hbm,
                     accn, accd, pn0, pn1, pd0, pd1, dst_v, semA, semB):
    cid = lax.axis_index("c")
    sid = lax.axis_index("s")
    wid = sid * NC + cid
    b0 = wid * WB
    r0 = sid * ROWS_PER_TILE
    rsl = pl.ds(r0, ROWS_PER_TILE)
    pltpu.sync_copy(zz_hbm.at[rsl, :], accn.at[rsl, :])
    pltpu.sync_copy(zz_hbm.at[rsl, :], accd.at[rsl, :])
    pltpu.sync_copy(dst_hbm.at[pl.ds(b0, WB), :], dst_v)
    plsc.subcore_barrier()

    def drain(pn_v, pd_v, sem):
        for b in range(GRP):
            pltpu.make_async_copy(pn_v.at[pl.ds(b * BLK, BLK), :],
                                  accn.at[dst_v.at[0]], sem).wait()
            pltpu.make_async_copy(pd_v.at[pl.ds(b * BLK, BLK), :],
                                  accd.at[dst_v.at[0]], sem).wait()

    def run_group(g, pn_v, pd_v, sem, first):
        if not first:
            drain(pn_v, pd_v, sem)
        base = (b0 + g * GRP) * BLK
        pltpu.sync_copy(pn_hbm.at[pl.ds(base, GRP * BLK), :], pn_v)
        pltpu.sync_copy(pd_hbm.at[pl.ds(base, GRP * BLK), :], pd_v)
        for b in range(GRP):
            idx = dst_v.at[g * GRP + b]
            pltpu.async_copy(pn_v.at[pl.ds(b * BLK, BLK), :],
                             accn.at[idx], sem, add=True)
            pltpu.async_copy(pd_v.at[pl.ds(b * BLK, BLK), :],
                             accd.at[idx], sem, add=True)

    def pair_body(i, _):
        @pl.when(i == 0)
        def _():
            run_group(2 * i, pn0, pd0, semA, True)
            run_group(2 * i + 1, pn1, pd1, semB, True)

        @pl.when(i > 0)
        def _():
            run_group(2 * i, pn0, pd0, semA, False)
            run_group(2 * i + 1, pn1, pd1, semB, False)
        return 0
    lax.fori_loop(0, (NGRP - 1) // 2, pair_body, 0)
    run_group(NGRP - 1, pn0, pd0, semA, False)
    drain(pn0, pd0, semA)
    drain(pn1, pd1, semB)

    @pl.when(wid < NTAIL)
    def _():
        tb = TAIL0 + wid
        pltpu.sync_copy(dst_hbm.at[tb], dst_v.at[0])
        pltpu.sync_copy(pn_hbm.at[pl.ds(tb * BLK, BLK), :],
                        pn0.at[pl.ds(0, BLK), :])
        pltpu.sync_copy(pd_hbm.at[pl.ds(tb * BLK, BLK), :],
                        pd0.at[pl.ds(0, BLK), :])
        pltpu.sync_copy(pn0.at[pl.ds(0, BLK), :],
                        accn.at[dst_v.at[0]], add=True)
        pltpu.sync_copy(pd0.at[pl.ds(0, BLK), :],
                        accd.at[dst_v.at[0]], add=True)

    plsc.subcore_barrier()
    pltpu.sync_copy(accn.at[rsl, :], an_hbm.at[cid, rsl, :])
    pltpu.sync_copy(accd.at[rsl, :], ad_hbm.at[cid, rsl, :])


def _sc_proj_gather_body(pr_hbm, dst_hbm, out_hbm,
                         dst_v, pr0, pr1, sem_g, sw0, sw1):
    wid = lax.axis_index("s") * NC + lax.axis_index("c")
    b0 = wid * WB
    pltpu.sync_copy(dst_hbm.at[pl.ds(b0, WB), :], dst_v)

    def run_group(g, pr_v, sem_w, first):
        if not first:
            pltpu.make_async_copy(
                pr_v, out_hbm.at[pl.ds(0, GRP * BLK), :], sem_w).wait()
        for b in range(GRP):
            pltpu.async_copy(pr_hbm.at[dst_v.at[g * GRP + b]],
                             pr_v.at[pl.ds(b * BLK, BLK), :], sem_g)
        pltpu.make_async_copy(
            pr_hbm.at[pl.ds(0, GRP * BLK), :], pr_v, sem_g).wait()
        pltpu.async_copy(pr_v, out_hbm.at[pl.ds((b0 + g * GRP) * BLK,
                                                GRP * BLK), :], sem_w)

    def pair_body(i, _):
        @pl.when(i == 0)
        def _():
            run_group(2 * i, pr0, sw0, True)
            run_group(2 * i + 1, pr1, sw1, True)

        @pl.when(i > 0)
        def _():
            run_group(2 * i, pr0, sw0, False)
            run_group(2 * i + 1, pr1, sw1, False)
        return 0
    lax.fori_loop(0, (NGRP - 1) // 2, pair_body, 0)
    run_group(NGRP - 1, pr0, sw0, False)
    pltpu.make_async_copy(pr0, out_hbm.at[pl.ds(0, GRP * BLK), :], sw0).wait()
    pltpu.make_async_copy(pr1, out_hbm.at[pl.ds(0, GRP * BLK), :], sw1).wait()

    @pl.when(wid < NTAIL)
    def _():
        tb = TAIL0 + wid
        pltpu.sync_copy(dst_hbm.at[tb], dst_v.at[0])
        pltpu.async_copy(pr_hbm.at[dst_v.at[0]],
                         pr0.at[pl.ds(0, BLK), :], sem_g).wait()
        pltpu.sync_copy(pr0.at[pl.ds(0, BLK), :],
                        out_hbm.at[pl.ds(tb * BLK, BLK), :])


_SC_MESH = plsc.VectorSubcoreMesh(core_axis_name="c", subcore_axis_name="s")
_SC_PARAMS = pltpu.CompilerParams(needs_layout_passes=False,
                                  use_tc_tiling_on_sc=False)
_f32 = jnp.float32


def kernel(node_features, edge_features, edge_index, node_tiers,
           w_node_score, b_node_score, w_edge_score, b_edge_score,
           wq, bq, wk, bk, wv, bv, wo, bo,
           w_c1, b_c1, w_c2, b_c2):
    del node_tiers
    src2d = edge_index[0].astype(jnp.int32).reshape(NBLK, BLK)
    dst2d = edge_index[1].astype(jnp.int32).reshape(NBLK, BLK)

    ns2, kmat = pl.pallas_call(
        _node_body,
        out_shape=[jax.ShapeDtypeStruct((N, 1), _f32),
                   jax.ShapeDtypeStruct((N, D), _f32)],
    )(node_features, w_node_score, b_node_score.reshape(1, 1),
      wk, bk.reshape(1, D))
    ns = ns2.reshape(N)

    sc_b = pl.kernel(
        _sc_score_gather_body,
        out_type=[jax.ShapeDtypeStruct((NBLK, BLK), _f32),
                  jax.ShapeDtypeStruct((NBLK, BLK), _f32),
                  jax.ShapeDtypeStruct((E, D), _f32)],
        mesh=_SC_MESH,
        scratch_types=[pltpu.VMEM((N,), _f32),
                       pltpu.VMEM((WB, BLK), jnp.int32),
                       pltpu.VMEM((WB, BLK), jnp.int32),
                       pltpu.VMEM((WB, BLK), _f32),
                       pltpu.VMEM((WB, BLK), _f32),
                       pltpu.VMEM((GRP * BLK, D), _f32),
                       pltpu.VMEM((GRP * BLK, D), _f32),
                       pltpu.SemaphoreType.DMA,
                       pltpu.SemaphoreType.DMA,
                       pltpu.SemaphoreType.DMA],
        compiler_params=_SC_PARAMS,
    )
    sumns, min_ns, kdst = sc_b(ns, src2d, dst2d, kmat)

    ns_pad = jnp.pad(ns, (0, NPAD - N), constant_values=-jnp.inf)
    ef2048 = edge_features.reshape(NBLK, BLK * D)
    w2048 = pl.pallas_call(
        _thresh_body,
        out_shape=jax.ShapeDtypeStruct((NBLK, BLK * D), _f32),
    )(ef2048, sumns, min_ns,
      ns_pad.reshape(NPAD // 128, 128), w_edge_score,
      b_edge_score.reshape(1, 1))
    w8 = w2048.reshape(EP8, 128)
    kd8 = kdst.reshape(EP8, 128)

    pay_n8, pay_d8 = pl.pallas_call(
        _payload_body,
        grid=(GE,),
        in_specs=[pl.BlockSpec((RB, 128), lambda i: (i, 0)),
                  pl.BlockSpec((RB, 128), lambda i: (i, 0)),
                  pl.BlockSpec((D, D), lambda i: (0, 0)),
                  pl.BlockSpec((1, D), lambda i: (0, 0)),
                  pl.BlockSpec((D, D), lambda i: (0, 0)),
                  pl.BlockSpec((1, D), lambda i: (0, 0))],
        out_specs=[pl.BlockSpec((RB, 128), lambda i: (i, 0)),
                   pl.BlockSpec((RB, 128), lambda i: (i, 0))],
        out_shape=[jax.ShapeDtypeStruct((EP8, 128), _f32),
                   jax.ShapeDtypeStruct((EP8, 128), _f32)],
    )(w8, kd8, wq, bq.reshape(1, D), wv, bv.reshape(1, D))
    pay_n = pay_n8.reshape(E, D)
    pay_d = pay_d8.reshape(E, D)

    zeros_nd = jnp.zeros((N, D), _f32)
    sc_e = pl.kernel(
        _sc_scatter_body,
        out_type=[jax.ShapeDtypeStruct((NC, N, D), _f32),
                  jax.ShapeDtypeStruct((NC, N, D), _f32)],
        mesh=_SC_MESH,
        scratch_types=[pltpu.VMEM_SHARED((N, D), _f32),
                       pltpu.VMEM_SHARED((N, D), _f32),
                       pltpu.VMEM((GRP * BLK, D), _f32),
                       pltpu.VMEM((GRP * BLK, D), _f32),
                       pltpu.VMEM((GRP * BLK, D), _f32),
                       pltpu.VMEM((GRP * BLK, D), _f32),
                       pltpu.VMEM((WB, BLK), jnp.int32),
                       pltpu.SemaphoreType.DMA,
                       pltpu.SemaphoreType.DMA],
        compiler_params=_SC_PARAMS,
    )
    acc_n, acc_d = sc_e(pay_n, pay_d, dst2d, zeros_nd)

    proj = pl.pallas_call(
        _proj_body,
        out_shape=jax.ShapeDtypeStruct((N, D), _f32),
    )(acc_n, acc_d, wo, bo.reshape(1, D))

    sc_g = pl.kernel(
        _sc_proj_gather_body,
        out_type=jax.ShapeDtypeStruct((E, D), _f32),
        mesh=_SC_MESH,
        scratch_types=[pltpu.VMEM((WB, BLK), jnp.int32),
                       pltpu.VMEM((GRP * BLK, D), _f32),
                       pltpu.VMEM((GRP * BLK, D), _f32),
                       pltpu.SemaphoreType.DMA,
                       pltpu.SemaphoreType.DMA,
                       pltpu.SemaphoreType.DMA],
        compiler_params=_SC_PARAMS,
    )
    projd = sc_g(proj, dst2d)

    pj8 = projd.reshape(EP8, 128)
    out8 = pl.pallas_call(
        _head_body,
        grid=(GE,),
        in_specs=[pl.BlockSpec((RB, 128), lambda i: (i, 0)),
                  pl.BlockSpec((RB, 128), lambda i: (i, 0)),
                  pl.BlockSpec((D, D), lambda i: (0, 0)),
                  pl.BlockSpec((1, D), lambda i: (0, 0)),
                  pl.BlockSpec((D, NUM_CLASSES), lambda i: (0, 0)),
                  pl.BlockSpec((1, NUM_CLASSES), lambda i: (0, 0))],
        out_specs=pl.BlockSpec((RB, 128), lambda i: (i, 0)),
        out_shape=jax.ShapeDtypeStruct((EP8, 128), _f32),
    )(w8, pj8, w_c1, b_c1.reshape(1, D), w_c2,
      b_c2.reshape(1, NUM_CLASSES))
    return out8.reshape(E, NUM_CLASSES)


# speedup vs baseline: 79.5255x; 1.0029x over previous
import functools

import numpy as np
import jax
import jax.numpy as jnp
from jax import lax
from jax.experimental import pallas as pl
from jax.experimental.pallas import tpu as pltpu
from jax.experimental.pallas import tpu_sc as plsc

N = 10000
E = 320000
D_NODE = 128
D = 16
H = 4
DH = 4
NUM_CLASSES = 16
KN = int(0.4 * N)
KE = int(0.4 * E)

NC = 2
NS = 16
NW = NC * NS
BLK = 128
NBLK = E // BLK
WB = NBLK // NW
TAIL0 = NW * WB
NTAIL = NBLK - TAIL0
GRP = 13
NGRP = WB // GRP
GRPE = 6
NGRPE = WB // GRPE
ROWS_PER_TILE = N // NS

NPAD = 10240

BE = 6400
GE = E // BE
EP8 = E * D // 128
RB = BE * D // 128

_MSB = np.int32(-2147483648)
_LOW = np.int32(2147483647)


def _iota2(shape, dim):
    return lax.broadcasted_iota(jnp.int32, shape, dim)


def _blockdiag16(w16):
    p_r = _iota2((128, D), 0) % D
    p_c = _iota2((128, D), 1)
    P16 = (p_r == p_c).astype(jnp.float32)
    q_r = _iota2((D, 128), 0)
    q_c = _iota2((D, 128), 1) % D
    Q16 = (q_r == q_c).astype(jnp.float32)
    blk_ok = (_iota2((128, 128), 0) // D == _iota2((128, 128), 1) // D)
    return (P16 @ w16 @ Q16) * blk_ok.astype(jnp.float32), Q16


def _node_body(nf_ref, wns_ref, bns_ref, wk_ref, bk_ref, ns_ref, km_ref):
    nf = nf_ref[...]
    ns_ref[...] = nf @ wns_ref[...] + bns_ref[0:1, 0:1]
    km_ref[...] = nf @ wk_ref[...] + bk_ref[...]


def _f32_key(x):
    b = lax.bitcast_convert_type(x, jnp.int32)
    return jnp.where(b < 0, b ^ _LOW, b)


def _thresh_body(ef_ref, sn_ref, mn_ref, ns_ref, wes_ref, bes_ref, mk_ref):
    k_i = lax.broadcasted_iota(jnp.int32, (BLK * D, D), 0)
    d_i = lax.broadcasted_iota(jnp.int32, (BLK * D, D), 1)
    M16T = ((k_i % D) == d_i).astype(jnp.float32)
    wtile = M16T @ wes_ref[...]
    b_k = lax.broadcasted_iota(jnp.int32, (BLK * D, BLK), 0) // D
    b_c = lax.broadcasted_iota(jnp.int32, (BLK * D, BLK), 1)
    W2 = (b_k == b_c).astype(jnp.float32) * wtile
    es2d = ef_ref[...] @ W2 + bes_ref[0:1, 0:1] + sn_ref[...]

    ekey = _f32_key(es2d)
    mkey = _f32_key(mn_ref[...])
    nkey = _f32_key(ns_ref[...])

    def select(keys, kth):
        def body(i, prefix_bits):
            cand_bits = prefix_bits | lax.shift_left(np.int32(1), 31 - i)
            cand_s = cand_bits ^ _MSB
            cnt = jnp.sum((keys >= cand_s).astype(jnp.int32))
            return jnp.where(cnt >= kth, cand_bits, prefix_bits)
        bits = lax.fori_loop(0, 32, body, np.int32(0))
        return bits ^ _MSB

    eth_k = select(ekey, np.int32(KE))
    nth_k = select(nkey, np.int32(KN))
    mask2d = ((ekey >= eth_k) & (mkey >= nth_k)).astype(jnp.float32)
    e_c = _iota2((BLK, BLK * D), 0)
    e_j = _iota2((BLK, BLK * D), 1) // D
    EXPL = (e_c == e_j).astype(jnp.float32)
    for st, sz in ((0, 624), (624, 624), (1248, 624), (1872, 628)):
        sl = pl.ds(st, sz)
        mk_ref[sl, :] = ef_ref[sl, :] * (mask2d[st:st + sz, :] @ EXPL)


def _payload_body(w_ref, kd_ref, wq_ref, bq_ref, wv_ref, bv_ref,
                  pn_ref, pd_ref):
    w8 = w_ref[...]
    W8q, Q16 = _blockdiag16(wq_ref[...])
    W8v, _ = _blockdiag16(wv_ref[...])
    q8 = w8 @ W8q + bq_ref[...] @ Q16
    v8 = w8 @ W8v + bv_ref[...] @ Q16
    p8 = q8 * kd_ref[...]
    r16 = _iota2((128, 128), 0) % D
    c16 = _iota2((128, 128), 1) % D
    blk_ok = (_iota2((128, 128), 0) // D == _iota2((128, 128), 1) // D)
    SB = (blk_ok & (r16 // DH == c16 // DH)).astype(jnp.float32)
    ex_big = jnp.exp((p8 @ SB) * (1.0 / (DH ** 0.5)))
    pn_ref[...] = v8 * ex_big
    SD = (blk_ok & (c16 < H) & (r16 // DH == c16)).astype(jnp.float32)
    lane4 = (_iota2((1, 128), 1) % D < H).astype(jnp.float32)
    pd_ref[...] = jnp.exp((p8 @ SD) * (1.0 / (DH ** 0.5))) * lane4


def _proj_body(an_ref, ad_ref, wo_ref, bo_ref, out_ref):
    numer = an_ref[0, :, :] + an_ref[1, :, :]
    den = ad_ref[0, :, :] + ad_ref[1, :, :]
    m_i = lax.broadcasted_iota(jnp.int32, (D, D), 0)
    m_j = lax.broadcasted_iota(jnp.int32, (D, D), 1) // DH
    M = (m_i == m_j).astype(jnp.float32)
    denb = den @ M
    pooled = numer / (denb + 1e-9)
    out_ref[...] = pooled @ wo_ref[...] + bo_ref[...]


def _erf(x):
    s = jnp.sign(x)
    a = jnp.abs(x)
    t = 1.0 / (1.0 + 0.3275911 * a)
    poly = ((((1.061405429 * t - 1.453152027) * t + 1.421413741) * t
             - 0.284496736) * t + 0.254829592) * t
    return s * (1.0 - poly * jnp.exp(-a * a))


def _head_body(wt_ref, pj_ref, w1_ref, b1_ref, w2_ref, b2_ref, out_ref):
    x8 = wt_ref[...] + pj_ref[...]
    W81, Q16 = _blockdiag16(w1_ref[...])
    W82, _ = _blockdiag16(w2_ref[...])
    h1 = x8 @ W81 + b1_ref[...] @ Q16
    h1 = 0.5 * h1 * (1.0 + _erf(h1 * 0.7071067811865476))
    out_ref[...] = h1 @ W82 + b2_ref[...] @ Q16


def _sc_score_gather_body(ns_hbm, src_hbm, dst_hbm, km_hbm,
                          sn_hbm, mn_hbm, kd_hbm,
                          ns_v, src_v, dst_v, sn_v, mn_v,
                          kd0, kd1, sem_g, sw0, sw1):
    wid = lax.axis_index("s") * NC + lax.axis_index("c")
    b0 = wid * WB
    pltpu.sync_copy(ns_hbm, ns_v)
    pltpu.sync_copy(src_hbm.at[pl.ds(b0, WB), :], src_v)
    pltpu.sync_copy(dst_hbm.at[pl.ds(b0, WB), :], dst_v)

    def blk_compute(j, _):
        def grp(g, _):
            sl = pl.ds(g * 16, 16)
            sv = plsc.load_gather(ns_v, [src_v[j, sl]])
            dv = plsc.load_gather(ns_v, [dst_v[j, sl]])
            sn_v[j, sl] = 0.5 * (sv + dv)
            mn_v[j, sl] = jnp.minimum(sv, dv)
            return 0
        lax.fori_loop(0, BLK // 16, grp, 0, unroll=True)
        return 0
    lax.fori_loop(0, WB, blk_compute, 0)
    pltpu.sync_copy(sn_v, sn_hbm.at[pl.ds(b0, WB), :])
    pltpu.sync_copy(mn_v, mn_hbm.at[pl.ds(b0, WB), :])

    def run_group(g, kd_v, sem_w, first):
        if not first:
            pltpu.make_async_copy(
                kd_v, kd_hbm.at[pl.ds(0, GRP * BLK), :], sem_w).wait()
        for b in range(GRP):
            pltpu.async_copy(km_hbm.at[dst_v.at[g * GRP + b]],
                             kd_v.at[pl.ds(b * BLK, BLK), :], sem_g)
        pltpu.make_async_copy(
            km_hbm.at[pl.ds(0, GRP * BLK), :], kd_v, sem_g).wait()
        pltpu.async_copy(kd_v, kd_hbm.at[pl.ds((b0 + g * GRP) * BLK,
                                               GRP * BLK), :], sem_w)

    def pair_body(i, _):
        @pl.when(i == 0)
        def _():
            run_group(2 * i, kd0, sw0, True)
            run_group(2 * i + 1, kd1, sw1, True)

        @pl.when(i > 0)
        def _():
            run_group(2 * i, kd0, sw0, False)
            run_group(2 * i + 1, kd1, sw1, False)
        return 0
    lax.fori_loop(0, NGRP // 2, pair_body, 0)
    pltpu.make_async_copy(kd0, kd_hbm.at[pl.ds(0, GRP * BLK), :], sw0).wait()
    pltpu.make_async_copy(kd1, kd_hbm.at[pl.ds(0, GRP * BLK), :], sw1).wait()

    @pl.when(wid < NTAIL)
    def _():
        tb = TAIL0 + wid
        pltpu.sync_copy(src_hbm.at[tb], src_v.at[0])
        pltpu.sync_copy(dst_hbm.at[tb], dst_v.at[0])
        pltpu.async_copy(km_hbm.at[dst_v.at[0]],
                         kd0.at[pl.ds(0, BLK), :], sem_g).wait()
        pltpu.sync_copy(kd0.at[pl.ds(0, BLK), :],
                        kd_hbm.at[pl.ds(tb * BLK, BLK), :])

        def grp(g, _):
            sl = pl.ds(g * 16, 16)
            sv = plsc.load_gather(ns_v, [src_v[0, sl]])
            dv = plsc.load_gather(ns_v, [dst_v[0, sl]])
            sn_v[0, sl] = 0.5 * (sv + dv)
            mn_v[0, sl] = jnp.minimum(sv, dv)
            return 0
        lax.fori_loop(0, BLK // 16, grp, 0, unroll=True)
        pltpu.sync_copy(sn_v.at[0], sn_hbm.at[tb])
        pltpu.sync_copy(mn_v.at[0], mn_hbm.at[tb])


def _sc_scatter_body(pn_hbm, pd_hbm, dst_hbm, zz_hbm,
                     an_hbm, ad_hbm,
                     accn, accd, pn0, pn1, pd0, pd1, dst_v, semA, semB):
    cid = lax.axis_index("c")
    sid = lax.axis_index("s")
    wid = sid * NC + cid
    b0 = wid * WB
    r0 = sid * ROWS_PER_TILE
    rsl = pl.ds(r0, ROWS_PER_TILE)
    pltpu.sync_copy(zz_hbm.at[rsl, :], accn.at[rsl, :])
    pltpu.sync_copy(zz_hbm.at[rsl, :], accd.at[rsl, :])
    pltpu.sync_copy(dst_hbm.at[pl.ds(b0, WB), :], dst_v)
    plsc.subcore_barrier()

    def drain(pn_v, pd_v, sem):
        for b in range(GRPE):
            pltpu.make_async_copy(pn_v.at[pl.ds(b * BLK, BLK), :],
                                  accn.at[dst_v.at[0]], sem).wait()
            pltpu.make_async_copy(pd_v.at[pl.ds(b * BLK, BLK), :],
                                  accd.at[dst_v.at[0]], sem).wait()

    def run_group(g, pn_v, pd_v, sem, first):
        if not first:
            drain(pn_v, pd_v, sem)
        base = (b0 + g * GRPE) * BLK
        pltpu.sync_copy(pn_hbm.at[pl.ds(base, GRPE * BLK), :], pn_v)
        pltpu.sync_copy(pd_hbm.at[pl.ds(base, GRPE * BLK), :], pd_v)
        for b in range(GRPE):
            idx = dst_v.at[g * GRPE + b]
            pltpu.async_copy(pn_v.at[pl.ds(b * BLK, BLK), :],
                             accn.at[idx], sem, add=True)
            pltpu.async_copy(pd_v.at[pl.ds(b * BLK, BLK), :],
                             accd.at[idx], sem, add=True)

    def pair_body(i, _):
        @pl.when(i == 0)
        def _():
            run_group(2 * i, pn0, pd0, semA, True)
            run_group(2 * i + 1, pn1, pd1, semB, True)

        @pl.when(i > 0)
        def _():
            run_group(2 * i, pn0, pd0, semA, False)
            run_group(2 * i + 1, pn1, pd1, semB, False)
        return 0
    lax.fori_loop(0, (NGRPE - 1) // 2, pair_body, 0)
    run_group(NGRPE - 1, pn0, pd0, semA, False)
    drain(pn0, pd0, semA)
    drain(pn1, pd1, semB)

    @pl.when(wid < NTAIL)
    def _():
        tb = TAIL0 + wid
        pltpu.sync_copy(dst_hbm.at[tb], dst_v.at[0])
        pltpu.sync_copy(pn_hbm.at[pl.ds(tb * BLK, BLK), :],
                        pn0.at[pl.ds(0, BLK), :])
        pltpu.sync_copy(pd_hbm.at[pl.ds(tb * BLK, BLK), :],
                        pd0.at[pl.ds(0, BLK), :])
        pltpu.sync_copy(pn0.at[pl.ds(0, BLK), :],
                        accn.at[dst_v.at[0]], add=True)
        pltpu.sync_copy(pd0.at[pl.ds(0, BLK), :],
                        accd.at[dst_v.at[0]], add=True)

    plsc.subcore_barrier()
    pltpu.sync_copy(accn.at[rsl, :], an_hbm.at[cid, rsl, :])
    pltpu.sync_copy(accd.at[rsl, :], ad_hbm.at[cid, rsl, :])


def _sc_proj_gather_body(pr_hbm, dst_hbm, out_hbm,
                         dst_v, pr0, pr1, sem_g, sw0, sw1):
    wid = lax.axis_index("s") * NC + lax.axis_index("c")
    b0 = wid * WB
    pltpu.sync_copy(dst_hbm.at[pl.ds(b0, WB), :], dst_v)

    def run_group(g, pr_v, sem_w, first):
        if not first:
            pltpu.make_async_copy(
                pr_v, out_hbm.at[pl.ds(0, GRP * BLK), :], sem_w).wait()
        for b in range(GRP):
            pltpu.async_copy(pr_hbm.at[dst_v.at[g * GRP + b]],
                             pr_v.at[pl.ds(b * BLK, BLK), :], sem_g)
        pltpu.make_async_copy(
            pr_hbm.at[pl.ds(0, GRP * BLK), :], pr_v, sem_g).wait()
        pltpu.async_copy(pr_v, out_hbm.at[pl.ds((b0 + g * GRP) * BLK,
                                                GRP * BLK), :], sem_w)

    def pair_body(i, _):
        @pl.when(i == 0)
        def _():
            run_group(2 * i, pr0, sw0, True)
            run_group(2 * i + 1, pr1, sw1, True)

        @pl.when(i > 0)
        def _():
            run_group(2 * i, pr0, sw0, False)
            run_group(2 * i + 1, pr1, sw1, False)
        return 0
    lax.fori_loop(0, NGRP // 2, pair_body, 0)
    pltpu.make_async_copy(pr0, out_hbm.at[pl.ds(0, GRP * BLK), :], sw0).wait()
    pltpu.make_async_copy(pr1, out_hbm.at[pl.ds(0, GRP * BLK), :], sw1).wait()

    @pl.when(wid < NTAIL)
    def _():
        tb = TAIL0 + wid
        pltpu.sync_copy(dst_hbm.at[tb], dst_v.at[0])
        pltpu.async_copy(pr_hbm.at[dst_v.at[0]],
                         pr0.at[pl.ds(0, BLK), :], sem_g).wait()
        pltpu.sync_copy(pr0.at[pl.ds(0, BLK), :],
                        out_hbm.at[pl.ds(tb * BLK, BLK), :])


_SC_MESH = plsc.VectorSubcoreMesh(core_axis_name="c", subcore_axis_name="s")
_SC_PARAMS = pltpu.CompilerParams(needs_layout_passes=False,
                                  use_tc_tiling_on_sc=False)
_f32 = jnp.float32


def kernel(node_features, edge_features, edge_index, node_tiers,
           w_node_score, b_node_score, w_edge_score, b_edge_score,
           wq, bq, wk, bk, wv, bv, wo, bo,
           w_c1, b_c1, w_c2, b_c2):
    del node_tiers
    src2d = edge_index[0].astype(jnp.int32).reshape(NBLK, BLK)
    dst2d = edge_index[1].astype(jnp.int32).reshape(NBLK, BLK)

    ns2, kmat = pl.pallas_call(
        _node_body,
        out_shape=[jax.ShapeDtypeStruct((N, 1), _f32),
                   jax.ShapeDtypeStruct((N, D), _f32)],
    )(node_features, w_node_score, b_node_score.reshape(1, 1),
      wk, bk.reshape(1, D))
    ns = ns2.reshape(N)

    sc_b = pl.kernel(
        _sc_score_gather_body,
        out_type=[jax.ShapeDtypeStruct((NBLK, BLK), _f32),
                  jax.ShapeDtypeStruct((NBLK, BLK), _f32),
                  jax.ShapeDtypeStruct((E, D), _f32)],
        mesh=_SC_MESH,
        scratch_types=[pltpu.VMEM((N,), _f32),
                       pltpu.VMEM((WB, BLK), jnp.int32),
                       pltpu.VMEM((WB, BLK), jnp.int32),
                       pltpu.VMEM((WB, BLK), _f32),
                       pltpu.VMEM((WB, BLK), _f32),
                       pltpu.VMEM((GRP * BLK, D), _f32),
                       pltpu.VMEM((GRP * BLK, D), _f32),
                       pltpu.SemaphoreType.DMA,
                       pltpu.SemaphoreType.DMA,
                       pltpu.SemaphoreType.DMA],
        compiler_params=_SC_PARAMS,
    )
    sumns, min_ns, kdst = sc_b(ns, src2d, dst2d, kmat)

    ns_pad = jnp.pad(ns, (0, NPAD - N), constant_values=-jnp.inf)
    ef2048 = edge_features.reshape(NBLK, BLK * D)
    w2048 = pl.pallas_call(
        _thresh_body,
        out_shape=jax.ShapeDtypeStruct((NBLK, BLK * D), _f32),
    )(ef2048, sumns, min_ns,
      ns_pad.reshape(NPAD // 128, 128), w_edge_score,
      b_edge_score.reshape(1, 1))
    w8 = w2048.reshape(EP8, 128)
    kd8 = kdst.reshape(EP8, 128)

    pay_n8, pay_d8 = pl.pallas_call(
        _payload_body,
        grid=(GE,),
        in_specs=[pl.BlockSpec((RB, 128), lambda i: (i, 0)),
                  pl.BlockSpec((RB, 128), lambda i: (i, 0)),
                  pl.BlockSpec((D, D), lambda i: (0, 0)),
                  pl.BlockSpec((1, D), lambda i: (0, 0)),
                  pl.BlockSpec((D, D), lambda i: (0, 0)),
                  pl.BlockSpec((1, D), lambda i: (0, 0))],
        out_specs=[pl.BlockSpec((RB, 128), lambda i: (i, 0)),
                   pl.BlockSpec((RB, 128), lambda i: (i, 0))],
        out_shape=[jax.ShapeDtypeStruct((EP8, 128), _f32),
                   jax.ShapeDtypeStruct((EP8, 128), _f32)],
    )(w8, kd8, wq, bq.reshape(1, D), wv, bv.reshape(1, D))
    pay_n = pay_n8.reshape(E, D)
    pay_d = pay_d8.reshape(E, D)

    zeros_nd = jnp.zeros((N, D), _f32)
    sc_e = pl.kernel(
        _sc_scatter_body,
        out_type=[jax.ShapeDtypeStruct((NC, N, D), _f32),
                  jax.ShapeDtypeStruct((NC, N, D), _f32)],
        mesh=_SC_MESH,
        scratch_types=[pltpu.VMEM_SHARED((N, D), _f32),
                       pltpu.VMEM_SHARED((N, D), _f32),
                       pltpu.VMEM((GRPE * BLK, D), _f32),
                       pltpu.VMEM((GRPE * BLK, D), _f32),
                       pltpu.VMEM((GRPE * BLK, D), _f32),
                       pltpu.VMEM((GRPE * BLK, D), _f32),
                       pltpu.VMEM((WB, BLK), jnp.int32),
                       pltpu.SemaphoreType.DMA,
                       pltpu.SemaphoreType.DMA],
        compiler_params=_SC_PARAMS,
    )
    acc_n, acc_d = sc_e(pay_n, pay_d, dst2d, zeros_nd)

    proj = pl.pallas_call(
        _proj_body,
        out_shape=jax.ShapeDtypeStruct((N, D), _f32),
    )(acc_n, acc_d, wo, bo.reshape(1, D))

    sc_g = pl.kernel(
        _sc_proj_gather_body,
        out_type=jax.ShapeDtypeStruct((E, D), _f32),
        mesh=_SC_MESH,
        scratch_types=[pltpu.VMEM((WB, BLK), jnp.int32),
                       pltpu.VMEM((GRP * BLK, D), _f32),
                       pltpu.VMEM((GRP * BLK, D), _f32),
                       pltpu.SemaphoreType.DMA,
                       pltpu.SemaphoreType.DMA,
                       pltpu.SemaphoreType.DMA],
        compiler_params=_SC_PARAMS,
    )
    projd = sc_g(proj, dst2d)

    pj8 = projd.reshape(EP8, 128)
    out8 = pl.pallas_call(
        _head_body,
        grid=(GE,),
        in_specs=[pl.BlockSpec((RB, 128), lambda i: (i, 0)),
                  pl.BlockSpec((RB, 128), lambda i: (i, 0)),
                  pl.BlockSpec((D, D), lambda i: (0, 0)),
                  pl.BlockSpec((1, D), lambda i: (0, 0)),
                  pl.BlockSpec((D, NUM_CLASSES), lambda i: (0, 0)),
                  pl.BlockSpec((1, NUM_CLASSES), lambda i: (0, 0))],
        out_specs=pl.BlockSpec((RB, 128), lambda i: (i, 0)),
        out_shape=jax.ShapeDtypeStruct((EP8, 128), _f32),
    )(w8, pj8, w_c1, b_c1.reshape(1, D), w_c2,
      b_c2.reshape(1, NUM_CLASSES))
    return out8.reshape(E, NUM_CLASSES)


# speedup vs baseline: 84.9929x; 1.0688x over previous
import functools

import numpy as np
import jax
import jax.numpy as jnp
from jax import lax
from jax.experimental import pallas as pl
from jax.experimental.pallas import tpu as pltpu
from jax.experimental.pallas import tpu_sc as plsc

N = 10000
E = 320000
D_NODE = 128
D = 16
H = 4
DH = 4
NUM_CLASSES = 16
KN = int(0.4 * N)
KE = int(0.4 * E)

NC = 2
NS = 16
NW = NC * NS
BLK = 128
NBLK = E // BLK
WB = NBLK // NW
TAIL0 = NW * WB
NTAIL = NBLK - TAIL0
GRP = 13
NGRP = WB // GRP
GRPE = 6
NGRPE = WB // GRPE
ROWS_PER_TILE = N // NS

NPAD = 10240

BE = 16000
GE = E // BE
EP8 = E * D // 128
RB = BE * D // 128

_MSB = np.int32(-2147483648)
_LOW = np.int32(2147483647)


def _iota2(shape, dim):
    return lax.broadcasted_iota(jnp.int32, shape, dim)


def _blockdiag16(w16):
    p_r = _iota2((128, D), 0) % D
    p_c = _iota2((128, D), 1)
    P16 = (p_r == p_c).astype(jnp.float32)
    q_r = _iota2((D, 128), 0)
    q_c = _iota2((D, 128), 1) % D
    Q16 = (q_r == q_c).astype(jnp.float32)
    blk_ok = (_iota2((128, 128), 0) // D == _iota2((128, 128), 1) // D)
    return (P16 @ w16 @ Q16) * blk_ok.astype(jnp.float32), Q16


def _node_body(nf_ref, wns_ref, bns_ref, wk_ref, bk_ref, ns_ref, km_ref):
    nf = nf_ref[...]
    ns_ref[...] = nf @ wns_ref[...] + bns_ref[0:1, 0:1]
    km_ref[...] = nf @ wk_ref[...] + bk_ref[...]


def _f32_key(x):
    b = lax.bitcast_convert_type(x, jnp.int32)
    return jnp.where(b < 0, b ^ _LOW, b)


def _thresh_body(ef_ref, sn_ref, mn_ref, ns_ref, wes_ref, bes_ref, mk_ref):
    k_i = lax.broadcasted_iota(jnp.int32, (BLK * D, D), 0)
    d_i = lax.broadcasted_iota(jnp.int32, (BLK * D, D), 1)
    M16T = ((k_i % D) == d_i).astype(jnp.float32)
    wtile = M16T @ wes_ref[...]
    b_k = lax.broadcasted_iota(jnp.int32, (BLK * D, BLK), 0) // D
    b_c = lax.broadcasted_iota(jnp.int32, (BLK * D, BLK), 1)
    W2 = (b_k == b_c).astype(jnp.float32) * wtile
    es2d = ef_ref[...] @ W2 + bes_ref[0:1, 0:1] + sn_ref[...]

    ekey = _f32_key(es2d)
    mkey = _f32_key(mn_ref[...])
    nkey = _f32_key(ns_ref[...])

    def select(keys, kth):
        def body(i, prefix_bits):
            cand_bits = prefix_bits | lax.shift_left(np.int32(1), 31 - i)
            cand_s = cand_bits ^ _MSB
            cnt = jnp.sum((keys >= cand_s).astype(jnp.int32))
            return jnp.where(cnt >= kth, cand_bits, prefix_bits)
        bits = lax.fori_loop(0, 32, body, np.int32(0))
        return bits ^ _MSB

    eth_k = select(ekey, np.int32(KE))
    nth_k = select(nkey, np.int32(KN))
    mask2d = ((ekey >= eth_k) & (mkey >= nth_k)).astype(jnp.float32)
    e_c = _iota2((BLK, BLK * D), 0)
    e_j = _iota2((BLK, BLK * D), 1) // D
    EXPL = (e_c == e_j).astype(jnp.float32)
    for st, sz in ((0, 624), (624, 624), (1248, 624), (1872, 628)):
        sl = pl.ds(st, sz)
        mk_ref[sl, :] = ef_ref[sl, :] * (mask2d[st:st + sz, :] @ EXPL)


def _payload_body(w_ref, kd_ref, wq_ref, bq_ref, wv_ref, bv_ref,
                  pn_ref, pd_ref):
    w8 = w_ref[...]
    W8q, Q16 = _blockdiag16(wq_ref[...])
    W8v, _ = _blockdiag16(wv_ref[...])
    q8 = w8 @ W8q + bq_ref[...] @ Q16
    v8 = w8 @ W8v + bv_ref[...] @ Q16
    p8 = q8 * kd_ref[...]
    r16 = _iota2((128, 128), 0) % D
    c16 = _iota2((128, 128), 1) % D
    blk_ok = (_iota2((128, 128), 0) // D == _iota2((128, 128), 1) // D)
    SB = (blk_ok & (r16 // DH == c16 // DH)).astype(jnp.float32)
    ex_big = jnp.exp((p8 @ SB) * (1.0 / (DH ** 0.5)))
    pn_ref[...] = v8 * ex_big
    SD = (blk_ok & (c16 < H) & (r16 // DH == c16)).astype(jnp.float32)
    lane4 = (_iota2((1, 128), 1) % D < H).astype(jnp.float32)
    pd_ref[...] = jnp.exp((p8 @ SD) * (1.0 / (DH ** 0.5))) * lane4


def _proj_body(an_ref, ad_ref, wo_ref, bo_ref, out_ref):
    numer = an_ref[0, :, :] + an_ref[1, :, :]
    den = ad_ref[0, :, :] + ad_ref[1, :, :]
    m_i = lax.broadcasted_iota(jnp.int32, (D, D), 0)
    m_j = lax.broadcasted_iota(jnp.int32, (D, D), 1) // DH
    M = (m_i == m_j).astype(jnp.float32)
    denb = den @ M
    pooled = numer / (denb + 1e-9)
    out_ref[...] = pooled @ wo_ref[...] + bo_ref[...]


def _erf(x):
    s = jnp.sign(x)
    a = jnp.abs(x)
    t = 1.0 / (1.0 + 0.3275911 * a)
    poly = ((((1.061405429 * t - 1.453152027) * t + 1.421413741) * t
             - 0.284496736) * t + 0.254829592) * t
    return s * (1.0 - poly * jnp.exp(-a * a))


def _head_body(wt_ref, pj_ref, w1_ref, b1_ref, w2_ref, b2_ref, out_ref):
    x8 = wt_ref[...] + pj_ref[...]
    W81, Q16 = _blockdiag16(w1_ref[...])
    W82, _ = _blockdiag16(w2_ref[...])
    h1 = x8 @ W81 + b1_ref[...] @ Q16
    h1 = 0.5 * h1 * (1.0 + _erf(h1 * 0.7071067811865476))
    out_ref[...] = h1 @ W82 + b2_ref[...] @ Q16


def _sc_score_gather_body(ns_hbm, src_hbm, dst_hbm, km_hbm,
                          sn_hbm, mn_hbm, kd_hbm,
                          ns_v, src_v, dst_v, sn_v, mn_v,
                          kd0, kd1, sem_g, sw0, sw1):
    wid = lax.axis_index("s") * NC + lax.axis_index("c")
    b0 = wid * WB
    pltpu.sync_copy(ns_hbm, ns_v)
    pltpu.sync_copy(src_hbm.at[pl.ds(b0, WB), :], src_v)
    pltpu.sync_copy(dst_hbm.at[pl.ds(b0, WB), :], dst_v)

    def blk_compute(j, _):
        def grp(g, _):
            sl = pl.ds(g * 16, 16)
            sv = plsc.load_gather(ns_v, [src_v[j, sl]])
            dv = plsc.load_gather(ns_v, [dst_v[j, sl]])
            sn_v[j, sl] = 0.5 * (sv + dv)
            mn_v[j, sl] = jnp.minimum(sv, dv)
            return 0
        lax.fori_loop(0, BLK // 16, grp, 0, unroll=True)
        return 0
    lax.fori_loop(0, WB, blk_compute, 0)
    pltpu.sync_copy(sn_v, sn_hbm.at[pl.ds(b0, WB), :])
    pltpu.sync_copy(mn_v, mn_hbm.at[pl.ds(b0, WB), :])

    def run_group(g, kd_v, sem_w, first):
        if not first:
            pltpu.make_async_copy(
                kd_v, kd_hbm.at[pl.ds(0, GRP * BLK), :], sem_w).wait()
        for b in range(GRP):
            pltpu.async_copy(km_hbm.at[dst_v.at[g * GRP + b]],
                             kd_v.at[pl.ds(b * BLK, BLK), :], sem_g)
        pltpu.make_async_copy(
            km_hbm.at[pl.ds(0, GRP * BLK), :], kd_v, sem_g).wait()
        pltpu.async_copy(kd_v, kd_hbm.at[pl.ds((b0 + g * GRP) * BLK,
                                               GRP * BLK), :], sem_w)

    def pair_body(i, _):
        @pl.when(i == 0)
        def _():
            run_group(2 * i, kd0, sw0, True)
            run_group(2 * i + 1, kd1, sw1, True)

        @pl.when(i > 0)
        def _():
            run_group(2 * i, kd0, sw0, False)
            run_group(2 * i + 1, kd1, sw1, False)
        return 0
    lax.fori_loop(0, NGRP // 2, pair_body, 0)
    pltpu.make_async_copy(kd0, kd_hbm.at[pl.ds(0, GRP * BLK), :], sw0).wait()
    pltpu.make_async_copy(kd1, kd_hbm.at[pl.ds(0, GRP * BLK), :], sw1).wait()

    @pl.when(wid < NTAIL)
    def _():
        tb = TAIL0 + wid
        pltpu.sync_copy(src_hbm.at[tb], src_v.at[0])
        pltpu.sync_copy(dst_hbm.at[tb], dst_v.at[0])
        pltpu.async_copy(km_hbm.at[dst_v.at[0]],
                         kd0.at[pl.ds(0, BLK), :], sem_g).wait()
        pltpu.sync_copy(kd0.at[pl.ds(0, BLK), :],
                        kd_hbm.at[pl.ds(tb * BLK, BLK), :])

        def grp(g, _):
            sl = pl.ds(g * 16, 16)
            sv = plsc.load_gather(ns_v, [src_v[0, sl]])
            dv = plsc.load_gather(ns_v, [dst_v[0, sl]])
            sn_v[0, sl] = 0.5 * (sv + dv)
            mn_v[0, sl] = jnp.minimum(sv, dv)
            return 0
        lax.fori_loop(0, BLK // 16, grp, 0, unroll=True)
        pltpu.sync_copy(sn_v.at[0], sn_hbm.at[tb])
        pltpu.sync_copy(mn_v.at[0], mn_hbm.at[tb])


def _sc_scatter_body(pn_hbm, pd_hbm, dst_hbm, zz_hbm,
                     an_hbm, ad_hbm,
                     accn, accd, pn0, pn1, pd0, pd1, dst_v, semA, semB):
    cid = lax.axis_index("c")
    sid = lax.axis_index("s")
    wid = sid * NC + cid
    b0 = wid * WB
    r0 = sid * ROWS_PER_TILE
    rsl = pl.ds(r0, ROWS_PER_TILE)
    pltpu.sync_copy(zz_hbm.at[rsl, :], accn.at[rsl, :])
    pltpu.sync_copy(zz_hbm.at[rsl, :], accd.at[rsl, :])
    pltpu.sync_copy(dst_hbm.at[pl.ds(b0, WB), :], dst_v)
    plsc.subcore_barrier()

    def drain(pn_v, pd_v, sem):
        for b in range(GRPE):
            pltpu.make_async_copy(pn_v.at[pl.ds(b * BLK, BLK), :],
                                  accn.at[dst_v.at[0]], sem).wait()
            pltpu.make_async_copy(pd_v.at[pl.ds(b * BLK, BLK), :],
                                  accd.at[dst_v.at[0]], sem).wait()

    def run_group(g, pn_v, pd_v, sem, first):
        if not first:
            drain(pn_v, pd_v, sem)
        base = (b0 + g * GRPE) * BLK
        pltpu.sync_copy(pn_hbm.at[pl.ds(base, GRPE * BLK), :], pn_v)
        pltpu.sync_copy(pd_hbm.at[pl.ds(base, GRPE * BLK), :], pd_v)
        for b in range(GRPE):
            idx = dst_v.at[g * GRPE + b]
            pltpu.async_copy(pn_v.at[pl.ds(b * BLK, BLK), :],
                             accn.at[idx], sem, add=True)
            pltpu.async_copy(pd_v.at[pl.ds(b * BLK, BLK), :],
                             accd.at[idx], sem, add=True)

    def pair_body(i, _):
        @pl.when(i == 0)
        def _():
            run_group(2 * i, pn0, pd0, semA, True)
            run_group(2 * i + 1, pn1, pd1, semB, True)

        @pl.when(i > 0)
        def _():
            run_group(2 * i, pn0, pd0, semA, False)
            run_group(2 * i + 1, pn1, pd1, semB, False)
        return 0
    lax.fori_loop(0, (NGRPE - 1) // 2, pair_body, 0)
    run_group(NGRPE - 1, pn0, pd0, semA, False)
    drain(pn0, pd0, semA)
    drain(pn1, pd1, semB)

    @pl.when(wid < NTAIL)
    def _():
        tb = TAIL0 + wid
        pltpu.sync_copy(dst_hbm.at[tb], dst_v.at[0])
        pltpu.sync_copy(pn_hbm.at[pl.ds(tb * BLK, BLK), :],
                        pn0.at[pl.ds(0, BLK), :])
        pltpu.sync_copy(pd_hbm.at[pl.ds(tb * BLK, BLK), :],
                        pd0.at[pl.ds(0, BLK), :])
        pltpu.sync_copy(pn0.at[pl.ds(0, BLK), :],
                        accn.at[dst_v.at[0]], add=True)
        pltpu.sync_copy(pd0.at[pl.ds(0, BLK), :],
                        accd.at[dst_v.at[0]], add=True)

    plsc.subcore_barrier()
    pltpu.sync_copy(accn.at[rsl, :], an_hbm.at[cid, rsl, :])
    pltpu.sync_copy(accd.at[rsl, :], ad_hbm.at[cid, rsl, :])


def _sc_proj_gather_body(pr_hbm, dst_hbm, out_hbm,
                         dst_v, pr0, pr1, sem_g, sw0, sw1):
    wid = lax.axis_index("s") * NC + lax.axis_index("c")
    b0 = wid * WB
    pltpu.sync_copy(dst_hbm.at[pl.ds(b0, WB), :], dst_v)

    def run_group(g, pr_v, sem_w, first):
        if not first:
            pltpu.make_async_copy(
                pr_v, out_hbm.at[pl.ds(0, GRP * BLK), :], sem_w).wait()
        for b in range(GRP):
            pltpu.async_copy(pr_hbm.at[dst_v.at[g * GRP + b]],
                             pr_v.at[pl.ds(b * BLK, BLK), :], sem_g)
        pltpu.make_async_copy(
            pr_hbm.at[pl.ds(0, GRP * BLK), :], pr_v, sem_g).wait()
        pltpu.async_copy(pr_v, out_hbm.at[pl.ds((b0 + g * GRP) * BLK,
                                                GRP * BLK), :], sem_w)

    def pair_body(i, _):
        @pl.when(i == 0)
        def _():
            run_group(2 * i, pr0, sw0, True)
            run_group(2 * i + 1, pr1, sw1, True)

        @pl.when(i > 0)
        def _():
            run_group(2 * i, pr0, sw0, False)
            run_group(2 * i + 1, pr1, sw1, False)
        return 0
    lax.fori_loop(0, NGRP // 2, pair_body, 0)
    pltpu.make_async_copy(pr0, out_hbm.at[pl.ds(0, GRP * BLK), :], sw0).wait()
    pltpu.make_async_copy(pr1, out_hbm.at[pl.ds(0, GRP * BLK), :], sw1).wait()

    @pl.when(wid < NTAIL)
    def _():
        tb = TAIL0 + wid
        pltpu.sync_copy(dst_hbm.at[tb], dst_v.at[0])
        pltpu.async_copy(pr_hbm.at[dst_v.at[0]],
                         pr0.at[pl.ds(0, BLK), :], sem_g).wait()
        pltpu.sync_copy(pr0.at[pl.ds(0, BLK), :],
                        out_hbm.at[pl.ds(tb * BLK, BLK), :])


_SC_MESH = plsc.VectorSubcoreMesh(core_axis_name="c", subcore_axis_name="s")
_SC_PARAMS = pltpu.CompilerParams(needs_layout_passes=False,
                                  use_tc_tiling_on_sc=False)
_f32 = jnp.float32


def kernel(node_features, edge_features, edge_index, node_tiers,
           w_node_score, b_node_score, w_edge_score, b_edge_score,
           wq, bq, wk, bk, wv, bv, wo, bo,
           w_c1, b_c1, w_c2, b_c2):
    del node_tiers
    src2d = edge_index[0].astype(jnp.int32).reshape(NBLK, BLK)
    dst2d = edge_index[1].astype(jnp.int32).reshape(NBLK, BLK)

    ns2, kmat = pl.pallas_call(
        _node_body,
        out_shape=[jax.ShapeDtypeStruct((N, 1), _f32),
                   jax.ShapeDtypeStruct((N, D), _f32)],
    )(node_features, w_node_score, b_node_score.reshape(1, 1),
      wk, bk.reshape(1, D))
    ns = ns2.reshape(N)

    sc_b = pl.kernel(
        _sc_score_gather_body,
        out_type=[jax.ShapeDtypeStruct((NBLK, BLK), _f32),
                  jax.ShapeDtypeStruct((NBLK, BLK), _f32),
                  jax.ShapeDtypeStruct((E, D), _f32)],
        mesh=_SC_MESH,
        scratch_types=[pltpu.VMEM((N,), _f32),
                       pltpu.VMEM((WB, BLK), jnp.int32),
                       pltpu.VMEM((WB, BLK), jnp.int32),
                       pltpu.VMEM((WB, BLK), _f32),
                       pltpu.VMEM((WB, BLK), _f32),
                       pltpu.VMEM((GRP * BLK, D), _f32),
                       pltpu.VMEM((GRP * BLK, D), _f32),
                       pltpu.SemaphoreType.DMA,
                       pltpu.SemaphoreType.DMA,
                       pltpu.SemaphoreType.DMA],
        compiler_params=_SC_PARAMS,
    )
    sumns, min_ns, kdst = sc_b(ns, src2d, dst2d, kmat)

    ns_pad = jnp.pad(ns, (0, NPAD - N), constant_values=-jnp.inf)
    ef2048 = edge_features.reshape(NBLK, BLK * D)
    w2048 = pl.pallas_call(
        _thresh_body,
        out_shape=jax.ShapeDtypeStruct((NBLK, BLK * D), _f32),
    )(ef2048, sumns, min_ns,
      ns_pad.reshape(NPAD // 128, 128), w_edge_score,
      b_edge_score.reshape(1, 1))
    w8 = w2048.reshape(EP8, 128)
    kd8 = kdst.reshape(EP8, 128)

    pay_n8, pay_d8 = pl.pallas_call(
        _payload_body,
        grid=(GE,),
        in_specs=[pl.BlockSpec((RB, 128), lambda i: (i, 0)),
                  pl.BlockSpec((RB, 128), lambda i: (i, 0)),
                  pl.BlockSpec((D, D), lambda i: (0, 0)),
                  pl.BlockSpec((1, D), lambda i: (0, 0)),
                  pl.BlockSpec((D, D), lambda i: (0, 0)),
                  pl.BlockSpec((1, D), lambda i: (0, 0))],
        out_specs=[pl.BlockSpec((RB, 128), lambda i: (i, 0)),
                   pl.BlockSpec((RB, 128), lambda i: (i, 0))],
        out_shape=[jax.ShapeDtypeStruct((EP8, 128), _f32),
                   jax.ShapeDtypeStruct((EP8, 128), _f32)],
    )(w8, kd8, wq, bq.reshape(1, D), wv, bv.reshape(1, D))
    pay_n = pay_n8.reshape(E, D)
    pay_d = pay_d8.reshape(E, D)

    zeros_nd = jnp.zeros((N, D), _f32)
    sc_e = pl.kernel(
        _sc_scatter_body,
        out_type=[jax.ShapeDtypeStruct((NC, N, D), _f32),
                  jax.ShapeDtypeStruct((NC, N, D), _f32)],
        mesh=_SC_MESH,
        scratch_types=[pltpu.VMEM_SHARED((N, D), _f32),
                       pltpu.VMEM_SHARED((N, D), _f32),
                       pltpu.VMEM((GRPE * BLK, D), _f32),
                       pltpu.VMEM((GRPE * BLK, D), _f32),
                       pltpu.VMEM((GRPE * BLK, D), _f32),
                       pltpu.VMEM((GRPE * BLK, D), _f32),
                       pltpu.VMEM((WB, BLK), jnp.int32),
                       pltpu.SemaphoreType.DMA,
                       pltpu.SemaphoreType.DMA],
        compiler_params=_SC_PARAMS,
    )
    acc_n, acc_d = sc_e(pay_n, pay_d, dst2d, zeros_nd)

    proj = pl.pallas_call(
        _proj_body,
        out_shape=jax.ShapeDtypeStruct((N, D), _f32),
    )(acc_n, acc_d, wo, bo.reshape(1, D))

    sc_g = pl.kernel(
        _sc_proj_gather_body,
        out_type=jax.ShapeDtypeStruct((E, D), _f32),
        mesh=_SC_MESH,
        scratch_types=[pltpu.VMEM((WB, BLK), jnp.int32),
                       pltpu.VMEM((GRP * BLK, D), _f32),
                       pltpu.VMEM((GRP * BLK, D), _f32),
                       pltpu.SemaphoreType.DMA,
                       pltpu.SemaphoreType.DMA,
                       pltpu.SemaphoreType.DMA],
        compiler_params=_SC_PARAMS,
    )
    projd = sc_g(proj, dst2d)

    pj8 = projd.reshape(EP8, 128)
    out8 = pl.pallas_call(
        _head_body,
        grid=(GE,),
        in_specs=[pl.BlockSpec((RB, 128), lambda i: (i, 0)),
                  pl.BlockSpec((RB, 128), lambda i: (i, 0)),
                  pl.BlockSpec((D, D), lambda i: (0, 0)),
                  pl.BlockSpec((1, D), lambda i: (0, 0)),
                  pl.BlockSpec((D, NUM_CLASSES), lambda i: (0, 0)),
                  pl.BlockSpec((1, NUM_CLASSES), lambda i: (0, 0))],
        out_specs=pl.BlockSpec((RB, 128), lambda i: (i, 0)),
        out_shape=jax.ShapeDtypeStruct((EP8, 128), _f32),
    )(w8, pj8, w_c1, b_c1.reshape(1, D), w_c2,
      b_c2.reshape(1, NUM_CLASSES))
    return out8.reshape(E, NUM_CLASSES)


# speedup vs baseline: 86.8431x; 1.0218x over previous
import functools

import numpy as np
import jax
import jax.numpy as jnp
from jax import lax
from jax.experimental import pallas as pl
from jax.experimental.pallas import tpu as pltpu
from jax.experimental.pallas import tpu_sc as plsc

N = 10000
E = 320000
D_NODE = 128
D = 16
H = 4
DH = 4
NUM_CLASSES = 16
KN = int(0.4 * N)
KE = int(0.4 * E)

NC = 2
NS = 16
NW = NC * NS
BLK = 128
NBLK = E // BLK
WB = NBLK // NW
TAIL0 = NW * WB
NTAIL = NBLK - TAIL0
GRP = 13
NGRP = WB // GRP
GRPE = 6
NGRPE = WB // GRPE
ROWS_PER_TILE = N // NS

NPAD = 10240

BE = 32000
GE = E // BE
EP8 = E * D // 128
RB = BE * D // 128

_MSB = np.int32(-2147483648)
_LOW = np.int32(2147483647)


def _iota2(shape, dim):
    return lax.broadcasted_iota(jnp.int32, shape, dim)


def _blockdiag16(w16):
    p_r = _iota2((128, D), 0) % D
    p_c = _iota2((128, D), 1)
    P16 = (p_r == p_c).astype(jnp.float32)
    q_r = _iota2((D, 128), 0)
    q_c = _iota2((D, 128), 1) % D
    Q16 = (q_r == q_c).astype(jnp.float32)
    blk_ok = (_iota2((128, 128), 0) // D == _iota2((128, 128), 1) // D)
    return (P16 @ w16 @ Q16) * blk_ok.astype(jnp.float32), Q16


def _node_body(nf_ref, wns_ref, bns_ref, wk_ref, bk_ref, ns_ref, km_ref):
    nf = nf_ref[...]
    ns_ref[...] = nf @ wns_ref[...] + bns_ref[0:1, 0:1]
    km_ref[...] = nf @ wk_ref[...] + bk_ref[...]


def _f32_key(x):
    b = lax.bitcast_convert_type(x, jnp.int32)
    return jnp.where(b < 0, b ^ _LOW, b)


def _thresh_body(ef_ref, sn_ref, mn_ref, ns_ref, wes_ref, bes_ref, mk_ref):
    k_i = lax.broadcasted_iota(jnp.int32, (BLK * D, D), 0)
    d_i = lax.broadcasted_iota(jnp.int32, (BLK * D, D), 1)
    M16T = ((k_i % D) == d_i).astype(jnp.float32)
    wtile = M16T @ wes_ref[...]
    b_k = lax.broadcasted_iota(jnp.int32, (BLK * D, BLK), 0) // D
    b_c = lax.broadcasted_iota(jnp.int32, (BLK * D, BLK), 1)
    W2 = (b_k == b_c).astype(jnp.float32) * wtile
    es2d = ef_ref[...] @ W2 + bes_ref[0:1, 0:1] + sn_ref[...]

    ekey = _f32_key(es2d)
    mkey = _f32_key(mn_ref[...])
    nkey = _f32_key(ns_ref[...])

    def select(keys, kth):
        def body(i, prefix_bits):
            cand_bits = prefix_bits | lax.shift_left(np.int32(1), 31 - i)
            cand_s = cand_bits ^ _MSB
            cnt = jnp.sum((keys >= cand_s).astype(jnp.int32))
            return jnp.where(cnt >= kth, cand_bits, prefix_bits)
        bits = lax.fori_loop(0, 32, body, np.int32(0))
        return bits ^ _MSB

    eth_k = select(ekey, np.int32(KE))
    nth_k = select(nkey, np.int32(KN))
    mask2d = ((ekey >= eth_k) & (mkey >= nth_k)).astype(jnp.float32)
    e_c = _iota2((BLK, BLK * D), 0)
    e_j = _iota2((BLK, BLK * D), 1) // D
    EXPL = (e_c == e_j).astype(jnp.float32)
    for st, sz in ((0, 624), (624, 624), (1248, 624), (1872, 628)):
        sl = pl.ds(st, sz)
        mk_ref[sl, :] = ef_ref[sl, :] * (mask2d[st:st + sz, :] @ EXPL)


def _payload_body(w_ref, kd_ref, wq_ref, bq_ref, wv_ref, bv_ref,
                  pn_ref, pd_ref):
    w8 = w_ref[...]
    W8q, Q16 = _blockdiag16(wq_ref[...])
    W8v, _ = _blockdiag16(wv_ref[...])
    q8 = w8 @ W8q + bq_ref[...] @ Q16
    v8 = w8 @ W8v + bv_ref[...] @ Q16
    p8 = q8 * kd_ref[...]
    r16 = _iota2((128, 128), 0) % D
    c16 = _iota2((128, 128), 1) % D
    blk_ok = (_iota2((128, 128), 0) // D == _iota2((128, 128), 1) // D)
    SB = (blk_ok & (r16 // DH == c16 // DH)).astype(jnp.float32)
    ex_big = jnp.exp((p8 @ SB) * (1.0 / (DH ** 0.5)))
    pn_ref[...] = v8 * ex_big
    SD = (blk_ok & (c16 < H) & (r16 // DH == c16)).astype(jnp.float32)
    lane4 = (_iota2((1, 128), 1) % D < H).astype(jnp.float32)
    pd_ref[...] = jnp.exp((p8 @ SD) * (1.0 / (DH ** 0.5))) * lane4


def _proj_body(an_ref, ad_ref, wo_ref, bo_ref, out_ref):
    numer = an_ref[0, :, :] + an_ref[1, :, :]
    den = ad_ref[0, :, :] + ad_ref[1, :, :]
    m_i = lax.broadcasted_iota(jnp.int32, (D, D), 0)
    m_j = lax.broadcasted_iota(jnp.int32, (D, D), 1) // DH
    M = (m_i == m_j).astype(jnp.float32)
    denb = den @ M
    pooled = numer / (denb + 1e-9)
    out_ref[...] = pooled @ wo_ref[...] + bo_ref[...]


def _erf(x):
    s = jnp.sign(x)
    a = jnp.abs(x)
    t = 1.0 / (1.0 + 0.3275911 * a)
    poly = ((((1.061405429 * t - 1.453152027) * t + 1.421413741) * t
             - 0.284496736) * t + 0.254829592) * t
    return s * (1.0 - poly * jnp.exp(-a * a))


def _head_body(wt_ref, pj_ref, w1_ref, b1_ref, w2_ref, b2_ref, out_ref):
    x8 = wt_ref[...] + pj_ref[...]
    W81, Q16 = _blockdiag16(w1_ref[...])
    W82, _ = _blockdiag16(w2_ref[...])
    h1 = x8 @ W81 + b1_ref[...] @ Q16
    h1 = 0.5 * h1 * (1.0 + _erf(h1 * 0.7071067811865476))
    out_ref[...] = h1 @ W82 + b2_ref[...] @ Q16


def _sc_score_gather_body(ns_hbm, src_hbm, dst_hbm, km_hbm,
                          sn_hbm, mn_hbm, kd_hbm,
                          ns_v, src_v, dst_v, sn_v, mn_v,
                          kd0, kd1, sem_g, sw0, sw1):
    wid = lax.axis_index("s") * NC + lax.axis_index("c")
    b0 = wid * WB
    pltpu.sync_copy(ns_hbm, ns_v)
    pltpu.sync_copy(src_hbm.at[pl.ds(b0, WB), :], src_v)
    pltpu.sync_copy(dst_hbm.at[pl.ds(b0, WB), :], dst_v)

    def blk_compute(j, _):
        def grp(g, _):
            sl = pl.ds(g * 16, 16)
            sv = plsc.load_gather(ns_v, [src_v[j, sl]])
            dv = plsc.load_gather(ns_v, [dst_v[j, sl]])
            sn_v[j, sl] = 0.5 * (sv + dv)
            mn_v[j, sl] = jnp.minimum(sv, dv)
            return 0
        lax.fori_loop(0, BLK // 16, grp, 0, unroll=True)
        return 0
    lax.fori_loop(0, WB, blk_compute, 0)
    pltpu.sync_copy(sn_v, sn_hbm.at[pl.ds(b0, WB), :])
    pltpu.sync_copy(mn_v, mn_hbm.at[pl.ds(b0, WB), :])

    def run_group(g, kd_v, sem_w, first):
        if not first:
            pltpu.make_async_copy(
                kd_v, kd_hbm.at[pl.ds(0, GRP * BLK), :], sem_w).wait()
        for b in range(GRP):
            pltpu.async_copy(km_hbm.at[dst_v.at[g * GRP + b]],
                             kd_v.at[pl.ds(b * BLK, BLK), :], sem_g)
        pltpu.make_async_copy(
            km_hbm.at[pl.ds(0, GRP * BLK), :], kd_v, sem_g).wait()
        pltpu.async_copy(kd_v, kd_hbm.at[pl.ds((b0 + g * GRP) * BLK,
                                               GRP * BLK), :], sem_w)

    def pair_body(i, _):
        @pl.when(i == 0)
        def _():
            run_group(2 * i, kd0, sw0, True)
            run_group(2 * i + 1, kd1, sw1, True)

        @pl.when(i > 0)
        def _():
            run_group(2 * i, kd0, sw0, False)
            run_group(2 * i + 1, kd1, sw1, False)
        return 0
    lax.fori_loop(0, NGRP // 2, pair_body, 0)
    pltpu.make_async_copy(kd0, kd_hbm.at[pl.ds(0, GRP * BLK), :], sw0).wait()
    pltpu.make_async_copy(kd1, kd_hbm.at[pl.ds(0, GRP * BLK), :], sw1).wait()

    @pl.when(wid < NTAIL)
    def _():
        tb = TAIL0 + wid
        pltpu.sync_copy(src_hbm.at[tb], src_v.at[0])
        pltpu.sync_copy(dst_hbm.at[tb], dst_v.at[0])
        pltpu.async_copy(km_hbm.at[dst_v.at[0]],
                         kd0.at[pl.ds(0, BLK), :], sem_g).wait()
        pltpu.sync_copy(kd0.at[pl.ds(0, BLK), :],
                        kd_hbm.at[pl.ds(tb * BLK, BLK), :])

        def grp(g, _):
            sl = pl.ds(g * 16, 16)
            sv = plsc.load_gather(ns_v, [src_v[0, sl]])
            dv = plsc.load_gather(ns_v, [dst_v[0, sl]])
            sn_v[0, sl] = 0.5 * (sv + dv)
            mn_v[0, sl] = jnp.minimum(sv, dv)
            return 0
        lax.fori_loop(0, BLK // 16, grp, 0, unroll=True)
        pltpu.sync_copy(sn_v.at[0], sn_hbm.at[tb])
        pltpu.sync_copy(mn_v.at[0], mn_hbm.at[tb])


def _sc_scatter_body(pn_hbm, pd_hbm, dst_hbm, zz_hbm,
                     an_hbm, ad_hbm,
                     accn, accd, pn0, pn1, pd0, pd1, dst_v, semA, semB):
    cid = lax.axis_index("c")
    sid = lax.axis_index("s")
    wid = sid * NC + cid
    b0 = wid * WB
    r0 = sid * ROWS_PER_TILE
    rsl = pl.ds(r0, ROWS_PER_TILE)
    pltpu.sync_copy(zz_hbm.at[rsl, :], accn.at[rsl, :])
    pltpu.sync_copy(zz_hbm.at[rsl, :], accd.at[rsl, :])
    pltpu.sync_copy(dst_hbm.at[pl.ds(b0, WB), :], dst_v)
    plsc.subcore_barrier()

    def drain(pn_v, pd_v, sem):
        for b in range(GRPE):
            pltpu.make_async_copy(pn_v.at[pl.ds(b * BLK, BLK), :],
                                  accn.at[dst_v.at[0]], sem).wait()
            pltpu.make_async_copy(pd_v.at[pl.ds(b * BLK, BLK), :],
                                  accd.at[dst_v.at[0]], sem).wait()

    def run_group(g, pn_v, pd_v, sem, first):
        if not first:
            drain(pn_v, pd_v, sem)
        base = (b0 + g * GRPE) * BLK
        pltpu.sync_copy(pn_hbm.at[pl.ds(base, GRPE * BLK), :], pn_v)
        pltpu.sync_copy(pd_hbm.at[pl.ds(base, GRPE * BLK), :], pd_v)
        for b in range(GRPE):
            idx = dst_v.at[g * GRPE + b]
            pltpu.async_copy(pn_v.at[pl.ds(b * BLK, BLK), :],
                             accn.at[idx], sem, add=True)
            pltpu.async_copy(pd_v.at[pl.ds(b * BLK, BLK), :],
                             accd.at[idx], sem, add=True)

    def pair_body(i, _):
        @pl.when(i == 0)
        def _():
            run_group(2 * i, pn0, pd0, semA, True)
            run_group(2 * i + 1, pn1, pd1, semB, True)

        @pl.when(i > 0)
        def _():
            run_group(2 * i, pn0, pd0, semA, False)
            run_group(2 * i + 1, pn1, pd1, semB, False)
        return 0
    lax.fori_loop(0, (NGRPE - 1) // 2, pair_body, 0)
    run_group(NGRPE - 1, pn0, pd0, semA, False)
    drain(pn0, pd0, semA)
    drain(pn1, pd1, semB)

    @pl.when(wid < NTAIL)
    def _():
        tb = TAIL0 + wid
        pltpu.sync_copy(dst_hbm.at[tb], dst_v.at[0])
        pltpu.sync_copy(pn_hbm.at[pl.ds(tb * BLK, BLK), :],
                        pn0.at[pl.ds(0, BLK), :])
        pltpu.sync_copy(pd_hbm.at[pl.ds(tb * BLK, BLK), :],
                        pd0.at[pl.ds(0, BLK), :])
        pltpu.sync_copy(pn0.at[pl.ds(0, BLK), :],
                        accn.at[dst_v.at[0]], add=True)
        pltpu.sync_copy(pd0.at[pl.ds(0, BLK), :],
                        accd.at[dst_v.at[0]], add=True)

    plsc.subcore_barrier()
    pltpu.sync_copy(accn.at[rsl, :], an_hbm.at[cid, rsl, :])
    pltpu.sync_copy(accd.at[rsl, :], ad_hbm.at[cid, rsl, :])


def _sc_proj_gather_body(pr_hbm, dst_hbm, out_hbm,
                         dst_v, pr0, pr1, sem_g, sw0, sw1):
    wid = lax.axis_index("s") * NC + lax.axis_index("c")
    b0 = wid * WB
    pltpu.sync_copy(dst_hbm.at[pl.ds(b0, WB), :], dst_v)

    def run_group(g, pr_v, sem_w, first):
        if not first:
            pltpu.make_async_copy(
                pr_v, out_hbm.at[pl.ds(0, GRP * BLK), :], sem_w).wait()
        for b in range(GRP):
            pltpu.async_copy(pr_hbm.at[dst_v.at[g * GRP + b]],
                             pr_v.at[pl.ds(b * BLK, BLK), :], sem_g)
        pltpu.make_async_copy(
            pr_hbm.at[pl.ds(0, GRP * BLK), :], pr_v, sem_g).wait()
        pltpu.async_copy(pr_v, out_hbm.at[pl.ds((b0 + g * GRP) * BLK,
                                                GRP * BLK), :], sem_w)

    def pair_body(i, _):
        @pl.when(i == 0)
        def _():
            run_group(2 * i, pr0, sw0, True)
            run_group(2 * i + 1, pr1, sw1, True)

        @pl.when(i > 0)
        def _():
            run_group(2 * i, pr0, sw0, False)
            run_group(2 * i + 1, pr1, sw1, False)
        return 0
    lax.fori_loop(0, NGRP // 2, pair_body, 0)
    pltpu.make_async_copy(pr0, out_hbm.at[pl.ds(0, GRP * BLK), :], sw0).wait()
    pltpu.make_async_copy(pr1, out_hbm.at[pl.ds(0, GRP * BLK), :], sw1).wait()

    @pl.when(wid < NTAIL)
    def _():
        tb = TAIL0 + wid
        pltpu.sync_copy(dst_hbm.at[tb], dst_v.at[0])
        pltpu.async_copy(pr_hbm.at[dst_v.at[0]],
                         pr0.at[pl.ds(0, BLK), :], sem_g).wait()
        pltpu.sync_copy(pr0.at[pl.ds(0, BLK), :],
                        out_hbm.at[pl.ds(tb * BLK, BLK), :])


_SC_MESH = plsc.VectorSubcoreMesh(core_axis_name="c", subcore_axis_name="s")
_SC_PARAMS = pltpu.CompilerParams(needs_layout_passes=False,
                                  use_tc_tiling_on_sc=False)
_f32 = jnp.float32


def kernel(node_features, edge_features, edge_index, node_tiers,
           w_node_score, b_node_score, w_edge_score, b_edge_score,
           wq, bq, wk, bk, wv, bv, wo, bo,
           w_c1, b_c1, w_c2, b_c2):
    del node_tiers
    src2d = edge_index[0].astype(jnp.int32).reshape(NBLK, BLK)
    dst2d = edge_index[1].astype(jnp.int32).reshape(NBLK, BLK)

    ns2, kmat = pl.pallas_call(
        _node_body,
        out_shape=[jax.ShapeDtypeStruct((N, 1), _f32),
                   jax.ShapeDtypeStruct((N, D), _f32)],
    )(node_features, w_node_score, b_node_score.reshape(1, 1),
      wk, bk.reshape(1, D))
    ns = ns2.reshape(N)

    sc_b = pl.kernel(
        _sc_score_gather_body,
        out_type=[jax.ShapeDtypeStruct((NBLK, BLK), _f32),
                  jax.ShapeDtypeStruct((NBLK, BLK), _f32),
                  jax.ShapeDtypeStruct((E, D), _f32)],
        mesh=_SC_MESH,
        scratch_types=[pltpu.VMEM((N,), _f32),
                       pltpu.VMEM((WB, BLK), jnp.int32),
                       pltpu.VMEM((WB, BLK), jnp.int32),
                       pltpu.VMEM((WB, BLK), _f32),
                       pltpu.VMEM((WB, BLK), _f32),
                       pltpu.VMEM((GRP * BLK, D), _f32),
                       pltpu.VMEM((GRP * BLK, D), _f32),
                       pltpu.SemaphoreType.DMA,
                       pltpu.SemaphoreType.DMA,
                       pltpu.SemaphoreType.DMA],
        compiler_params=_SC_PARAMS,
    )
    sumns, min_ns, kdst = sc_b(ns, src2d, dst2d, kmat)

    ns_pad = jnp.pad(ns, (0, NPAD - N), constant_values=-jnp.inf)
    ef2048 = edge_features.reshape(NBLK, BLK * D)
    w2048 = pl.pallas_call(
        _thresh_body,
        out_shape=jax.ShapeDtypeStruct((NBLK, BLK * D), _f32),
    )(ef2048, sumns, min_ns,
      ns_pad.reshape(NPAD // 128, 128), w_edge_score,
      b_edge_score.reshape(1, 1))
    w8 = w2048.reshape(EP8, 128)
    kd8 = kdst.reshape(EP8, 128)

    pay_n8, pay_d8 = pl.pallas_call(
        _payload_body,
        grid=(GE,),
        in_specs=[pl.BlockSpec((RB, 128), lambda i: (i, 0)),
                  pl.BlockSpec((RB, 128), lambda i: (i, 0)),
                  pl.BlockSpec((D, D), lambda i: (0, 0)),
                  pl.BlockSpec((1, D), lambda i: (0, 0)),
                  pl.BlockSpec((D, D), lambda i: (0, 0)),
                  pl.BlockSpec((1, D), lambda i: (0, 0))],
        out_specs=[pl.BlockSpec((RB, 128), lambda i: (i, 0)),
                   pl.BlockSpec((RB, 128), lambda i: (i, 0))],
        out_shape=[jax.ShapeDtypeStruct((EP8, 128), _f32),
                   jax.ShapeDtypeStruct((EP8, 128), _f32)],
    )(w8, kd8, wq, bq.reshape(1, D), wv, bv.reshape(1, D))
    pay_n = pay_n8.reshape(E, D)
    pay_d = pay_d8.reshape(E, D)

    zeros_nd = jnp.zeros((N, D), _f32)
    sc_e = pl.kernel(
        _sc_scatter_body,
        out_type=[jax.ShapeDtypeStruct((NC, N, D), _f32),
                  jax.ShapeDtypeStruct((NC, N, D), _f32)],
        mesh=_SC_MESH,
        scratch_types=[pltpu.VMEM_SHARED((N, D), _f32),
                       pltpu.VMEM_SHARED((N, D), _f32),
                       pltpu.VMEM((GRPE * BLK, D), _f32),
                       pltpu.VMEM((GRPE * BLK, D), _f32),
                       pltpu.VMEM((GRPE * BLK, D), _f32),
                       pltpu.VMEM((GRPE * BLK, D), _f32),
                       pltpu.VMEM((WB, BLK), jnp.int32),
                       pltpu.SemaphoreType.DMA,
                       pltpu.SemaphoreType.DMA],
        compiler_params=_SC_PARAMS,
    )
    acc_n, acc_d = sc_e(pay_n, pay_d, dst2d, zeros_nd)

    proj = pl.pallas_call(
        _proj_body,
        out_shape=jax.ShapeDtypeStruct((N, D), _f32),
    )(acc_n, acc_d, wo, bo.reshape(1, D))

    sc_g = pl.kernel(
        _sc_proj_gather_body,
        out_type=jax.ShapeDtypeStruct((E, D), _f32),
        mesh=_SC_MESH,
        scratch_types=[pltpu.VMEM((WB, BLK), jnp.int32),
                       pltpu.VMEM((GRP * BLK, D), _f32),
                       pltpu.VMEM((GRP * BLK, D), _f32),
                       pltpu.SemaphoreType.DMA,
                       pltpu.SemaphoreType.DMA,
                       pltpu.SemaphoreType.DMA],
        compiler_params=_SC_PARAMS,
    )
    projd = sc_g(proj, dst2d)

    pj8 = projd.reshape(EP8, 128)
    out8 = pl.pallas_call(
        _head_body,
        grid=(GE,),
        in_specs=[pl.BlockSpec((RB, 128), lambda i: (i, 0)),
                  pl.BlockSpec((RB, 128), lambda i: (i, 0)),
                  pl.BlockSpec((D, D), lambda i: (0, 0)),
                  pl.BlockSpec((1, D), lambda i: (0, 0)),
                  pl.BlockSpec((D, NUM_CLASSES), lambda i: (0, 0)),
                  pl.BlockSpec((1, NUM_CLASSES), lambda i: (0, 0))],
        out_specs=pl.BlockSpec((RB, 128), lambda i: (i, 0)),
        out_shape=jax.ShapeDtypeStruct((EP8, 128), _f32),
    )(w8, pj8, w_c1, b_c1.reshape(1, D), w_c2,
      b_c2.reshape(1, NUM_CLASSES))
    return out8.reshape(E, NUM_CLASSES)


# speedup vs baseline: 87.1763x; 1.0038x over previous
import functools

import numpy as np
import jax
import jax.numpy as jnp
from jax import lax
from jax.experimental import pallas as pl
from jax.experimental.pallas import tpu as pltpu
from jax.experimental.pallas import tpu_sc as plsc

N = 10000
E = 320000
D_NODE = 128
D = 16
H = 4
DH = 4
NUM_CLASSES = 16
KN = int(0.4 * N)
KE = int(0.4 * E)

NC = 2
NS = 16
NW = NC * NS
BLK = 128
NBLK = E // BLK
WB = NBLK // NW
TAIL0 = NW * WB
NTAIL = NBLK - TAIL0
GRP = 13
NGRP = WB // GRP
GRPE = 6
NGRPE = WB // GRPE
ROWS_PER_TILE = N // NS

NPAD = 10240

BE = 64000
GE = E // BE
EP8 = E * D // 128
RB = BE * D // 128

_MSB = np.int32(-2147483648)
_LOW = np.int32(2147483647)


def _iota2(shape, dim):
    return lax.broadcasted_iota(jnp.int32, shape, dim)


def _blockdiag16(w16):
    p_r = _iota2((128, D), 0) % D
    p_c = _iota2((128, D), 1)
    P16 = (p_r == p_c).astype(jnp.float32)
    q_r = _iota2((D, 128), 0)
    q_c = _iota2((D, 128), 1) % D
    Q16 = (q_r == q_c).astype(jnp.float32)
    blk_ok = (_iota2((128, 128), 0) // D == _iota2((128, 128), 1) // D)
    return (P16 @ w16 @ Q16) * blk_ok.astype(jnp.float32), Q16


def _node_body(nf_ref, wns_ref, bns_ref, wk_ref, bk_ref, ns_ref, km_ref):
    nf = nf_ref[...]
    ns_ref[...] = nf @ wns_ref[...] + bns_ref[0:1, 0:1]
    km_ref[...] = nf @ wk_ref[...] + bk_ref[...]


def _f32_key(x):
    b = lax.bitcast_convert_type(x, jnp.int32)
    return jnp.where(b < 0, b ^ _LOW, b)


def _thresh_body(ef_ref, sn_ref, mn_ref, ns_ref, wes_ref, bes_ref, mk_ref):
    k_i = lax.broadcasted_iota(jnp.int32, (BLK * D, D), 0)
    d_i = lax.broadcasted_iota(jnp.int32, (BLK * D, D), 1)
    M16T = ((k_i % D) == d_i).astype(jnp.float32)
    wtile = M16T @ wes_ref[...]
    b_k = lax.broadcasted_iota(jnp.int32, (BLK * D, BLK), 0) // D
    b_c = lax.broadcasted_iota(jnp.int32, (BLK * D, BLK), 1)
    W2 = (b_k == b_c).astype(jnp.float32) * wtile
    es2d = ef_ref[...] @ W2 + bes_ref[0:1, 0:1] + sn_ref[...]

    ekey = _f32_key(es2d)
    mkey = _f32_key(mn_ref[...])
    nkey = _f32_key(ns_ref[...])

    def select(keys, kth):
        def body(i, prefix_bits):
            cand_bits = prefix_bits | lax.shift_left(np.int32(1), 31 - i)
            cand_s = cand_bits ^ _MSB
            cnt = jnp.sum((keys >= cand_s).astype(jnp.int32))
            return jnp.where(cnt >= kth, cand_bits, prefix_bits)
        bits = lax.fori_loop(0, 32, body, np.int32(0))
        return bits ^ _MSB

    eth_k = select(ekey, np.int32(KE))
    nth_k = select(nkey, np.int32(KN))
    mask2d = ((ekey >= eth_k) & (mkey >= nth_k)).astype(jnp.float32)
    e_c = _iota2((BLK, BLK * D), 0)
    e_j = _iota2((BLK, BLK * D), 1) // D
    EXPL = (e_c == e_j).astype(jnp.float32)
    for st, sz in ((0, 624), (624, 624), (1248, 624), (1872, 628)):
        sl = pl.ds(st, sz)
        mk_ref[sl, :] = ef_ref[sl, :] * (mask2d[st:st + sz, :] @ EXPL)


def _payload_body(w_ref, kd_ref, wq_ref, bq_ref, wv_ref, bv_ref,
                  pn_ref, pd_ref):
    w8 = w_ref[...]
    W8q, Q16 = _blockdiag16(wq_ref[...])
    W8v, _ = _blockdiag16(wv_ref[...])
    q8 = w8 @ W8q + bq_ref[...] @ Q16
    v8 = w8 @ W8v + bv_ref[...] @ Q16
    p8 = q8 * kd_ref[...]
    r16 = _iota2((128, 128), 0) % D
    c16 = _iota2((128, 128), 1) % D
    blk_ok = (_iota2((128, 128), 0) // D == _iota2((128, 128), 1) // D)
    SB = (blk_ok & (r16 // DH == c16 // DH)).astype(jnp.float32)
    ex_big = jnp.exp((p8 @ SB) * (1.0 / (DH ** 0.5)))
    pn_ref[...] = v8 * ex_big
    SD = (blk_ok & (c16 < H) & (r16 // DH == c16)).astype(jnp.float32)
    lane4 = (_iota2((1, 128), 1) % D < H).astype(jnp.float32)
    pd_ref[...] = jnp.exp((p8 @ SD) * (1.0 / (DH ** 0.5))) * lane4


def _proj_body(an_ref, ad_ref, wo_ref, bo_ref, out_ref):
    numer = an_ref[0, :, :] + an_ref[1, :, :]
    den = ad_ref[0, :, :] + ad_ref[1, :, :]
    m_i = lax.broadcasted_iota(jnp.int32, (D, D), 0)
    m_j = lax.broadcasted_iota(jnp.int32, (D, D), 1) // DH
    M = (m_i == m_j).astype(jnp.float32)
    denb = den @ M
    pooled = numer / (denb + 1e-9)
    out_ref[...] = pooled @ wo_ref[...] + bo_ref[...]


def _erf(x):
    s = jnp.sign(x)
    a = jnp.abs(x)
    t = 1.0 / (1.0 + 0.3275911 * a)
    poly = ((((1.061405429 * t - 1.453152027) * t + 1.421413741) * t
             - 0.284496736) * t + 0.254829592) * t
    return s * (1.0 - poly * jnp.exp(-a * a))


def _head_body(wt_ref, pj_ref, w1_ref, b1_ref, w2_ref, b2_ref, out_ref):
    x8 = wt_ref[...] + pj_ref[...]
    W81, Q16 = _blockdiag16(w1_ref[...])
    W82, _ = _blockdiag16(w2_ref[...])
    h1 = x8 @ W81 + b1_ref[...] @ Q16
    h1 = 0.5 * h1 * (1.0 + _erf(h1 * 0.7071067811865476))
    out_ref[...] = h1 @ W82 + b2_ref[...] @ Q16


def _sc_score_gather_body(ns_hbm, src_hbm, dst_hbm, km_hbm,
                          sn_hbm, mn_hbm, kd_hbm,
                          ns_v, src_v, dst_v, sn_v, mn_v,
                          kd0, kd1, sem_g, sw0, sw1):
    wid = lax.axis_index("s") * NC + lax.axis_index("c")
    b0 = wid * WB
    pltpu.sync_copy(ns_hbm, ns_v)
    pltpu.sync_copy(src_hbm.at[pl.ds(b0, WB), :], src_v)
    pltpu.sync_copy(dst_hbm.at[pl.ds(b0, WB), :], dst_v)

    def blk_compute(j, _):
        def grp(g, _):
            sl = pl.ds(g * 16, 16)
            sv = plsc.load_gather(ns_v, [src_v[j, sl]])
            dv = plsc.load_gather(ns_v, [dst_v[j, sl]])
            sn_v[j, sl] = 0.5 * (sv + dv)
            mn_v[j, sl] = jnp.minimum(sv, dv)
            return 0
        lax.fori_loop(0, BLK // 16, grp, 0, unroll=True)
        return 0
    lax.fori_loop(0, WB, blk_compute, 0)
    pltpu.sync_copy(sn_v, sn_hbm.at[pl.ds(b0, WB), :])
    pltpu.sync_copy(mn_v, mn_hbm.at[pl.ds(b0, WB), :])

    def run_group(g, kd_v, sem_w, first):
        if not first:
            pltpu.make_async_copy(
                kd_v, kd_hbm.at[pl.ds(0, GRP * BLK), :], sem_w).wait()
        for b in range(GRP):
            pltpu.async_copy(km_hbm.at[dst_v.at[g * GRP + b]],
                             kd_v.at[pl.ds(b * BLK, BLK), :], sem_g)
        pltpu.make_async_copy(
            km_hbm.at[pl.ds(0, GRP * BLK), :], kd_v, sem_g).wait()
        pltpu.async_copy(kd_v, kd_hbm.at[pl.ds((b0 + g * GRP) * BLK,
                                               GRP * BLK), :], sem_w)

    def pair_body(i, _):
        @pl.when(i == 0)
        def _():
            run_group(2 * i, kd0, sw0, True)
            run_group(2 * i + 1, kd1, sw1, True)

        @pl.when(i > 0)
        def _():
            run_group(2 * i, kd0, sw0, False)
            run_group(2 * i + 1, kd1, sw1, False)
        return 0
    lax.fori_loop(0, NGRP // 2, pair_body, 0)
    pltpu.make_async_copy(kd0, kd_hbm.at[pl.ds(0, GRP * BLK), :], sw0).wait()
    pltpu.make_async_copy(kd1, kd_hbm.at[pl.ds(0, GRP * BLK), :], sw1).wait()

    @pl.when(wid < NTAIL)
    def _():
        tb = TAIL0 + wid
        pltpu.sync_copy(src_hbm.at[tb], src_v.at[0])
        pltpu.sync_copy(dst_hbm.at[tb], dst_v.at[0])
        pltpu.async_copy(km_hbm.at[dst_v.at[0]],
                         kd0.at[pl.ds(0, BLK), :], sem_g).wait()
        pltpu.sync_copy(kd0.at[pl.ds(0, BLK), :],
                        kd_hbm.at[pl.ds(tb * BLK, BLK), :])

        def grp(g, _):
            sl = pl.ds(g * 16, 16)
            sv = plsc.load_gather(ns_v, [src_v[0, sl]])
            dv = plsc.load_gather(ns_v, [dst_v[0, sl]])
            sn_v[0, sl] = 0.5 * (sv + dv)
            mn_v[0, sl] = jnp.minimum(sv, dv)
            return 0
        lax.fori_loop(0, BLK // 16, grp, 0, unroll=True)
        pltpu.sync_copy(sn_v.at[0], sn_hbm.at[tb])
        pltpu.sync_copy(mn_v.at[0], mn_hbm.at[tb])


def _sc_scatter_body(pn_hbm, pd_hbm, dst_hbm, zz_hbm,
                     an_hbm, ad_hbm,
                     accn, accd, pn0, pn1, pd0, pd1, dst_v, semA, semB):
    cid = lax.axis_index("c")
    sid = lax.axis_index("s")
    wid = sid * NC + cid
    b0 = wid * WB
    r0 = sid * ROWS_PER_TILE
    rsl = pl.ds(r0, ROWS_PER_TILE)
    pltpu.sync_copy(zz_hbm.at[rsl, :], accn.at[rsl, :])
    pltpu.sync_copy(zz_hbm.at[rsl, :], accd.at[rsl, :])
    pltpu.sync_copy(dst_hbm.at[pl.ds(b0, WB), :], dst_v)
    plsc.subcore_barrier()

    def drain(pn_v, pd_v, sem):
        for b in range(GRPE):
            pltpu.make_async_copy(pn_v.at[pl.ds(b * BLK, BLK), :],
                                  accn.at[dst_v.at[0]], sem).wait()
            pltpu.make_async_copy(pd_v.at[pl.ds(b * BLK, BLK), :],
                                  accd.at[dst_v.at[0]], sem).wait()

    def run_group(g, pn_v, pd_v, sem, first):
        if not first:
            drain(pn_v, pd_v, sem)
        base = (b0 + g * GRPE) * BLK
        pltpu.sync_copy(pn_hbm.at[pl.ds(base, GRPE * BLK), :], pn_v)
        pltpu.sync_copy(pd_hbm.at[pl.ds(base, GRPE * BLK), :], pd_v)
        for b in range(GRPE):
            idx = dst_v.at[g * GRPE + b]
            pltpu.async_copy(pn_v.at[pl.ds(b * BLK, BLK), :],
                             accn.at[idx], sem, add=True)
            pltpu.async_copy(pd_v.at[pl.ds(b * BLK, BLK), :],
                             accd.at[idx], sem, add=True)

    def pair_body(i, _):
        @pl.when(i == 0)
        def _():
            run_group(2 * i, pn0, pd0, semA, True)
            run_group(2 * i + 1, pn1, pd1, semB, True)

        @pl.when(i > 0)
        def _():
            run_group(2 * i, pn0, pd0, semA, False)
            run_group(2 * i + 1, pn1, pd1, semB, False)
        return 0
    lax.fori_loop(0, (NGRPE - 1) // 2, pair_body, 0)
    run_group(NGRPE - 1, pn0, pd0, semA, False)
    drain(pn0, pd0, semA)
    drain(pn1, pd1, semB)

    @pl.when(wid < NTAIL)
    def _():
        tb = TAIL0 + wid
        pltpu.sync_copy(dst_hbm.at[tb], dst_v.at[0])
        pltpu.sync_copy(pn_hbm.at[pl.ds(tb * BLK, BLK), :],
                        pn0.at[pl.ds(0, BLK), :])
        pltpu.sync_copy(pd_hbm.at[pl.ds(tb * BLK, BLK), :],
                        pd0.at[pl.ds(0, BLK), :])
        pltpu.sync_copy(pn0.at[pl.ds(0, BLK), :],
                        accn.at[dst_v.at[0]], add=True)
        pltpu.sync_copy(pd0.at[pl.ds(0, BLK), :],
                        accd.at[dst_v.at[0]], add=True)

    plsc.subcore_barrier()
    pltpu.sync_copy(accn.at[rsl, :], an_hbm.at[cid, rsl, :])
    pltpu.sync_copy(accd.at[rsl, :], ad_hbm.at[cid, rsl, :])


def _sc_proj_gather_body(pr_hbm, dst_hbm, out_hbm,
                         dst_v, pr0, pr1, sem_g, sw0, sw1):
    wid = lax.axis_index("s") * NC + lax.axis_index("c")
    b0 = wid * WB
    pltpu.sync_copy(dst_hbm.at[pl.ds(b0, WB), :], dst_v)

    def run_group(g, pr_v, sem_w, first):
        if not first:
            pltpu.make_async_copy(
                pr_v, out_hbm.at[pl.ds(0, GRP * BLK), :], sem_w).wait()
        for b in range(GRP):
            pltpu.async_copy(pr_hbm.at[dst_v.at[g * GRP + b]],
                             pr_v.at[pl.ds(b * BLK, BLK), :], sem_g)
        pltpu.make_async_copy(
            pr_hbm.at[pl.ds(0, GRP * BLK), :], pr_v, sem_g).wait()
        pltpu.async_copy(pr_v, out_hbm.at[pl.ds((b0 + g * GRP) * BLK,
                                                GRP * BLK), :], sem_w)

    def pair_body(i, _):
        @pl.when(i == 0)
        def _():
            run_group(2 * i, pr0, sw0, True)
            run_group(2 * i + 1, pr1, sw1, True)

        @pl.when(i > 0)
        def _():
            run_group(2 * i, pr0, sw0, False)
            run_group(2 * i + 1, pr1, sw1, False)
        return 0
    lax.fori_loop(0, NGRP // 2, pair_body, 0)
    pltpu.make_async_copy(pr0, out_hbm.at[pl.ds(0, GRP * BLK), :], sw0).wait()
    pltpu.make_async_copy(pr1, out_hbm.at[pl.ds(0, GRP * BLK), :], sw1).wait()

    @pl.when(wid < NTAIL)
    def _():
        tb = TAIL0 + wid
        pltpu.sync_copy(dst_hbm.at[tb], dst_v.at[0])
        pltpu.async_copy(pr_hbm.at[dst_v.at[0]],
                         pr0.at[pl.ds(0, BLK), :], sem_g).wait()
        pltpu.sync_copy(pr0.at[pl.ds(0, BLK), :],
                        out_hbm.at[pl.ds(tb * BLK, BLK), :])


_SC_MESH = plsc.VectorSubcoreMesh(core_axis_name="c", subcore_axis_name="s")
_SC_PARAMS = pltpu.CompilerParams(needs_layout_passes=False,
                                  use_tc_tiling_on_sc=False)
_f32 = jnp.float32


def kernel(node_features, edge_features, edge_index, node_tiers,
           w_node_score, b_node_score, w_edge_score, b_edge_score,
           wq, bq, wk, bk, wv, bv, wo, bo,
           w_c1, b_c1, w_c2, b_c2):
    del node_tiers
    src2d = edge_index[0].astype(jnp.int32).reshape(NBLK, BLK)
    dst2d = edge_index[1].astype(jnp.int32).reshape(NBLK, BLK)

    ns2, kmat = pl.pallas_call(
        _node_body,
        out_shape=[jax.ShapeDtypeStruct((N, 1), _f32),
                   jax.ShapeDtypeStruct((N, D), _f32)],
    )(node_features, w_node_score, b_node_score.reshape(1, 1),
      wk, bk.reshape(1, D))
    ns = ns2.reshape(N)

    sc_b = pl.kernel(
        _sc_score_gather_body,
        out_type=[jax.ShapeDtypeStruct((NBLK, BLK), _f32),
                  jax.ShapeDtypeStruct((NBLK, BLK), _f32),
                  jax.ShapeDtypeStruct((E, D), _f32)],
        mesh=_SC_MESH,
        scratch_types=[pltpu.VMEM((N,), _f32),
                       pltpu.VMEM((WB, BLK), jnp.int32),
                       pltpu.VMEM((WB, BLK), jnp.int32),
                       pltpu.VMEM((WB, BLK), _f32),
                       pltpu.VMEM((WB, BLK), _f32),
                       pltpu.VMEM((GRP * BLK, D), _f32),
                       pltpu.VMEM((GRP * BLK, D), _f32),
                       pltpu.SemaphoreType.DMA,
                       pltpu.SemaphoreType.DMA,
                       pltpu.SemaphoreType.DMA],
        compiler_params=_SC_PARAMS,
    )
    sumns, min_ns, kdst = sc_b(ns, src2d, dst2d, kmat)

    ns_pad = jnp.pad(ns, (0, NPAD - N), constant_values=-jnp.inf)
    ef2048 = edge_features.reshape(NBLK, BLK * D)
    w2048 = pl.pallas_call(
        _thresh_body,
        out_shape=jax.ShapeDtypeStruct((NBLK, BLK * D), _f32),
    )(ef2048, sumns, min_ns,
      ns_pad.reshape(NPAD // 128, 128), w_edge_score,
      b_edge_score.reshape(1, 1))
    w8 = w2048.reshape(EP8, 128)
    kd8 = kdst.reshape(EP8, 128)

    pay_n8, pay_d8 = pl.pallas_call(
        _payload_body,
        grid=(GE,),
        in_specs=[pl.BlockSpec((RB, 128), lambda i: (i, 0)),
                  pl.BlockSpec((RB, 128), lambda i: (i, 0)),
                  pl.BlockSpec((D, D), lambda i: (0, 0)),
                  pl.BlockSpec((1, D), lambda i: (0, 0)),
                  pl.BlockSpec((D, D), lambda i: (0, 0)),
                  pl.BlockSpec((1, D), lambda i: (0, 0))],
        out_specs=[pl.BlockSpec((RB, 128), lambda i: (i, 0)),
                   pl.BlockSpec((RB, 128), lambda i: (i, 0))],
        out_shape=[jax.ShapeDtypeStruct((EP8, 128), _f32),
                   jax.ShapeDtypeStruct((EP8, 128), _f32)],
    )(w8, kd8, wq, bq.reshape(1, D), wv, bv.reshape(1, D))
    pay_n = pay_n8.reshape(E, D)
    pay_d = pay_d8.reshape(E, D)

    zeros_nd = jnp.zeros((N, D), _f32)
    sc_e = pl.kernel(
        _sc_scatter_body,
        out_type=[jax.ShapeDtypeStruct((NC, N, D), _f32),
                  jax.ShapeDtypeStruct((NC, N, D), _f32)],
        mesh=_SC_MESH,
        scratch_types=[pltpu.VMEM_SHARED((N, D), _f32),
                       pltpu.VMEM_SHARED((N, D), _f32),
                       pltpu.VMEM((GRPE * BLK, D), _f32),
                       pltpu.VMEM((GRPE * BLK, D), _f32),
                       pltpu.VMEM((GRPE * BLK, D), _f32),
                       pltpu.VMEM((GRPE * BLK, D), _f32),
                       pltpu.VMEM((WB, BLK), jnp.int32),
                       pltpu.SemaphoreType.DMA,
                       pltpu.SemaphoreType.DMA],
        compiler_params=_SC_PARAMS,
    )
    acc_n, acc_d = sc_e(pay_n, pay_d, dst2d, zeros_nd)

    proj = pl.pallas_call(
        _proj_body,
        out_shape=jax.ShapeDtypeStruct((N, D), _f32),
    )(acc_n, acc_d, wo, bo.reshape(1, D))

    sc_g = pl.kernel(
        _sc_proj_gather_body,
        out_type=jax.ShapeDtypeStruct((E, D), _f32),
        mesh=_SC_MESH,
        scratch_types=[pltpu.VMEM((WB, BLK), jnp.int32),
                       pltpu.VMEM((GRP * BLK, D), _f32),
                       pltpu.VMEM((GRP * BLK, D), _f32),
                       pltpu.SemaphoreType.DMA,
                       pltpu.SemaphoreType.DMA,
                       pltpu.SemaphoreType.DMA],
        compiler_params=_SC_PARAMS,
    )
    projd = sc_g(proj, dst2d)

    pj8 = projd.reshape(EP8, 128)
    out8 = pl.pallas_call(
        _head_body,
        grid=(GE,),
        in_specs=[pl.BlockSpec((RB, 128), lambda i: (i, 0)),
                  pl.BlockSpec((RB, 128), lambda i: (i, 0)),
                  pl.BlockSpec((D, D), lambda i: (0, 0)),
                  pl.BlockSpec((1, D), lambda i: (0, 0)),
                  pl.BlockSpec((D, NUM_CLASSES), lambda i: (0, 0)),
                  pl.BlockSpec((1, NUM_CLASSES), lambda i: (0, 0))],
        out_specs=pl.BlockSpec((RB, 128), lambda i: (i, 0)),
        out_shape=jax.ShapeDtypeStruct((EP8, 128), _f32),
    )(w8, pj8, w_c1, b_c1.reshape(1, D), w_c2,
      b_c2.reshape(1, NUM_CLASSES))
    return out8.reshape(E, NUM_CLASSES)


# speedup vs baseline: 87.6642x; 1.0056x over previous
import functools

import numpy as np
import jax
import jax.numpy as jnp
from jax import lax
from jax.experimental import pallas as pl
from jax.experimental.pallas import tpu as pltpu
from jax.experimental.pallas import tpu_sc as plsc

N = 10000
E = 320000
D_NODE = 128
D = 16
H = 4
DH = 4
NUM_CLASSES = 16
KN = int(0.4 * N)
KE = int(0.4 * E)

NC = 2
NS = 16
NW = NC * NS
BLK = 128
NBLK = E // BLK
WB = NBLK // NW
TAIL0 = NW * WB
NTAIL = NBLK - TAIL0
GRP = 13
NGRP = WB // GRP
GRPE = 6
NGRPE = WB // GRPE
ROWS_PER_TILE = N // NS

NPAD = 10240

BE = 64000
GE = E // BE
EP8 = E * D // 128
RB = BE * D // 128

_MSB = np.int32(-2147483648)
_LOW = np.int32(2147483647)


def _iota2(shape, dim):
    return lax.broadcasted_iota(jnp.int32, shape, dim)


def _blockdiag16(w16):
    p_r = _iota2((128, D), 0) % D
    p_c = _iota2((128, D), 1)
    P16 = (p_r == p_c).astype(jnp.float32)
    q_r = _iota2((D, 128), 0)
    q_c = _iota2((D, 128), 1) % D
    Q16 = (q_r == q_c).astype(jnp.float32)
    blk_ok = (_iota2((128, 128), 0) // D == _iota2((128, 128), 1) // D)
    return (P16 @ w16 @ Q16) * blk_ok.astype(jnp.float32), Q16


def _node_body(nf_ref, wns_ref, bns_ref, wk_ref, bk_ref, ns_ref, km_ref):
    nf = nf_ref[...]
    ns_ref[...] = nf @ wns_ref[...] + bns_ref[0:1, 0:1]
    km_ref[...] = nf @ wk_ref[...] + bk_ref[...]


def _f32_key(x):
    b = lax.bitcast_convert_type(x, jnp.int32)
    return jnp.where(b < 0, b ^ _LOW, b)


def _thresh_body(ef_ref, sn_ref, mn_ref, ns_ref, wes_ref, bes_ref, mk_ref):
    k_i = lax.broadcasted_iota(jnp.int32, (BLK * D, D), 0)
    d_i = lax.broadcasted_iota(jnp.int32, (BLK * D, D), 1)
    M16T = ((k_i % D) == d_i).astype(jnp.float32)
    wtile = M16T @ wes_ref[...]
    b_k = lax.broadcasted_iota(jnp.int32, (BLK * D, BLK), 0) // D
    b_c = lax.broadcasted_iota(jnp.int32, (BLK * D, BLK), 1)
    W2 = (b_k == b_c).astype(jnp.float32) * wtile
    es2d = ef_ref[...] @ W2 + bes_ref[0:1, 0:1] + sn_ref[...]

    ekey = _f32_key(es2d)
    mkey = _f32_key(mn_ref[...])
    nkey = _f32_key(ns_ref[...])

    def select(keys, kth):
        def body(i, prefix_bits):
            cand_bits = prefix_bits | lax.shift_left(np.int32(1), 31 - i)
            cand_s = cand_bits ^ _MSB
            cnt = jnp.sum((keys >= cand_s).astype(jnp.int32))
            return jnp.where(cnt >= kth, cand_bits, prefix_bits)
        bits = lax.fori_loop(0, 32, body, np.int32(0))
        return bits ^ _MSB

    eth_k = select(ekey, np.int32(KE))
    nth_k = select(nkey, np.int32(KN))
    mask2d = ((ekey >= eth_k) & (mkey >= nth_k)).astype(jnp.float32)
    e_c = _iota2((BLK, BLK * D), 0)
    e_j = _iota2((BLK, BLK * D), 1) // D
    EXPL = (e_c == e_j).astype(jnp.float32)
    for st, sz in ((0, 624), (624, 624), (1248, 624), (1872, 628)):
        sl = pl.ds(st, sz)
        mk_ref[sl, :] = ef_ref[sl, :] * (mask2d[st:st + sz, :] @ EXPL)


def _payload_body(w_ref, kd_ref, wq_ref, bq_ref, wv_ref, bv_ref,
                  pn_ref, pd_ref):
    w8 = w_ref[...]
    W8q, Q16 = _blockdiag16(wq_ref[...])
    W8v, _ = _blockdiag16(wv_ref[...])
    q8 = w8 @ W8q + bq_ref[...] @ Q16
    v8 = w8 @ W8v + bv_ref[...] @ Q16
    p8 = q8 * kd_ref[...]
    r16 = _iota2((128, 128), 0) % D
    c16 = _iota2((128, 128), 1) % D
    blk_ok = (_iota2((128, 128), 0) // D == _iota2((128, 128), 1) // D)
    SB = (blk_ok & (r16 // DH == c16 // DH)).astype(jnp.float32)
    ex_big = jnp.exp((p8 @ SB) * (1.0 / (DH ** 0.5)))
    pn_ref[...] = v8 * ex_big
    SD = (blk_ok & (c16 < H) & (r16 // DH == c16)).astype(jnp.float32)
    lane4 = (_iota2((1, 128), 1) % D < H).astype(jnp.float32)
    pd_ref[...] = jnp.exp((p8 @ SD) * (1.0 / (DH ** 0.5))) * lane4


def _proj_body(an_ref, ad_ref, wo_ref, bo_ref, out_ref):
    numer = an_ref[0, :, :] + an_ref[1, :, :]
    den = ad_ref[0, :, :] + ad_ref[1, :, :]
    m_i = lax.broadcasted_iota(jnp.int32, (D, D), 0)
    m_j = lax.broadcasted_iota(jnp.int32, (D, D), 1) // DH
    M = (m_i == m_j).astype(jnp.float32)
    denb = den @ M
    pooled = numer / (denb + 1e-9)
    out_ref[...] = pooled @ wo_ref[...] + bo_ref[...]


def _erf(x):
    s = jnp.sign(x)
    a = jnp.abs(x)
    t = 1.0 / (1.0 + 0.3275911 * a)
    poly = ((((1.061405429 * t - 1.453152027) * t + 1.421413741) * t
             - 0.284496736) * t + 0.254829592) * t
    return s * (1.0 - poly * jnp.exp(-a * a))


def _head_body(wt_ref, pj_ref, w1_ref, b1_ref, w2_ref, b2_ref, out_ref):
    x8 = wt_ref[...] + pj_ref[...]
    W81, Q16 = _blockdiag16(w1_ref[...])
    W82, _ = _blockdiag16(w2_ref[...])
    h1 = x8 @ W81 + b1_ref[...] @ Q16
    h1 = 0.5 * h1 * (1.0 + _erf(h1 * 0.7071067811865476))
    out_ref[...] = h1 @ W82 + b2_ref[...] @ Q16


def _sc_score_gather_body(ns_hbm, src_hbm, dst_hbm,
                          sn_hbm, mn_hbm,
                          ns_v, src_v, dst_v, sn_v, mn_v):
    wid = lax.axis_index("s") * NC + lax.axis_index("c")
    b0 = wid * WB
    pltpu.sync_copy(ns_hbm, ns_v)
    pltpu.sync_copy(src_hbm.at[pl.ds(b0, WB), :], src_v)
    pltpu.sync_copy(dst_hbm.at[pl.ds(b0, WB), :], dst_v)

    def blk_compute(j, _):
        def grp(g, _):
            sl = pl.ds(g * 16, 16)
            sv = plsc.load_gather(ns_v, [src_v[j, sl]])
            dv = plsc.load_gather(ns_v, [dst_v[j, sl]])
            sn_v[j, sl] = 0.5 * (sv + dv)
            mn_v[j, sl] = jnp.minimum(sv, dv)
            return 0
        lax.fori_loop(0, BLK // 16, grp, 0, unroll=True)
        return 0
    lax.fori_loop(0, WB, blk_compute, 0)
    pltpu.sync_copy(sn_v, sn_hbm.at[pl.ds(b0, WB), :])
    pltpu.sync_copy(mn_v, mn_hbm.at[pl.ds(b0, WB), :])

    @pl.when(wid < NTAIL)
    def _():
        tb = TAIL0 + wid
        pltpu.sync_copy(src_hbm.at[tb], src_v.at[0])
        pltpu.sync_copy(dst_hbm.at[tb], dst_v.at[0])

        def grp(g, _):
            sl = pl.ds(g * 16, 16)
            sv = plsc.load_gather(ns_v, [src_v[0, sl]])
            dv = plsc.load_gather(ns_v, [dst_v[0, sl]])
            sn_v[0, sl] = 0.5 * (sv + dv)
            mn_v[0, sl] = jnp.minimum(sv, dv)
            return 0
        lax.fori_loop(0, BLK // 16, grp, 0, unroll=True)
        pltpu.sync_copy(sn_v.at[0], sn_hbm.at[tb])
        pltpu.sync_copy(mn_v.at[0], mn_hbm.at[tb])


def _sc_scatter_body(pn_hbm, pd_hbm, dst_hbm, zz_hbm,
                     an_hbm, ad_hbm,
                     accn, accd, pn0, pn1, pd0, pd1, dst_v, semA, semB):
    cid = lax.axis_index("c")
    sid = lax.axis_index("s")
    wid = sid * NC + cid
    b0 = wid * WB
    r0 = sid * ROWS_PER_TILE
    rsl = pl.ds(r0, ROWS_PER_TILE)
    pltpu.sync_copy(zz_hbm.at[rsl, :], accn.at[rsl, :])
    pltpu.sync_copy(zz_hbm.at[rsl, :], accd.at[rsl, :])
    pltpu.sync_copy(dst_hbm.at[pl.ds(b0, WB), :], dst_v)
    plsc.subcore_barrier()

    def drain(pn_v, pd_v, sem):
        for b in range(GRPE):
            pltpu.make_async_copy(pn_v.at[pl.ds(b * BLK, BLK), :],
                                  accn.at[dst_v.at[0]], sem).wait()
            pltpu.make_async_copy(pd_v.at[pl.ds(b * BLK, BLK), :],
                                  accd.at[dst_v.at[0]], sem).wait()

    def run_group(g, pn_v, pd_v, sem, first):
        if not first:
            drain(pn_v, pd_v, sem)
        base = (b0 + g * GRPE) * BLK
        pltpu.sync_copy(pn_hbm.at[pl.ds(base, GRPE * BLK), :], pn_v)
        pltpu.sync_copy(pd_hbm.at[pl.ds(base, GRPE * BLK), :], pd_v)
        for b in range(GRPE):
            idx = dst_v.at[g * GRPE + b]
            pltpu.async_copy(pn_v.at[pl.ds(b * BLK, BLK), :],
                             accn.at[idx], sem, add=True)
            pltpu.async_copy(pd_v.at[pl.ds(b * BLK, BLK), :],
                             accd.at[idx], sem, add=True)

    def pair_body(i, _):
        @pl.when(i == 0)
        def _():
            run_group(2 * i, pn0, pd0, semA, True)
            run_group(2 * i + 1, pn1, pd1, semB, True)

        @pl.when(i > 0)
        def _():
            run_group(2 * i, pn0, pd0, semA, False)
            run_group(2 * i + 1, pn1, pd1, semB, False)
        return 0
    lax.fori_loop(0, (NGRPE - 1) // 2, pair_body, 0)
    run_group(NGRPE - 1, pn0, pd0, semA, False)
    drain(pn0, pd0, semA)
    drain(pn1, pd1, semB)

    @pl.when(wid < NTAIL)
    def _():
        tb = TAIL0 + wid
        pltpu.sync_copy(dst_hbm.at[tb], dst_v.at[0])
        pltpu.sync_copy(pn_hbm.at[pl.ds(tb * BLK, BLK), :],
                        pn0.at[pl.ds(0, BLK), :])
        pltpu.sync_copy(pd_hbm.at[pl.ds(tb * BLK, BLK), :],
                        pd0.at[pl.ds(0, BLK), :])
        pltpu.sync_copy(pn0.at[pl.ds(0, BLK), :],
                        accn.at[dst_v.at[0]], add=True)
        pltpu.sync_copy(pd0.at[pl.ds(0, BLK), :],
                        accd.at[dst_v.at[0]], add=True)

    plsc.subcore_barrier()
    pltpu.sync_copy(accn.at[rsl, :], an_hbm.at[cid, rsl, :])
    pltpu.sync_copy(accd.at[rsl, :], ad_hbm.at[cid, rsl, :])


def _sc_proj_gather_body(pr_hbm, dst_hbm, out_hbm,
                         dst_v, pr0, pr1, sem_g, sw0, sw1):
    wid = lax.axis_index("s") * NC + lax.axis_index("c")
    b0 = wid * WB
    pltpu.sync_copy(dst_hbm.at[pl.ds(b0, WB), :], dst_v)

    def run_group(g, pr_v, sem_w, first):
        if not first:
            pltpu.make_async_copy(
                pr_v, out_hbm.at[pl.ds(0, GRP * BLK), :], sem_w).wait()
        for b in range(GRP):
            pltpu.async_copy(pr_hbm.at[dst_v.at[g * GRP + b]],
                             pr_v.at[pl.ds(b * BLK, BLK), :], sem_g)
        pltpu.make_async_copy(
            pr_hbm.at[pl.ds(0, GRP * BLK), :], pr_v, sem_g).wait()
        pltpu.async_copy(pr_v, out_hbm.at[pl.ds((b0 + g * GRP) * BLK,
                                                GRP * BLK), :], sem_w)

    def pair_body(i, _):
        @pl.when(i == 0)
        def _():
            run_group(2 * i, pr0, sw0, True)
            run_group(2 * i + 1, pr1, sw1, True)

        @pl.when(i > 0)
        def _():
            run_group(2 * i, pr0, sw0, False)
            run_group(2 * i + 1, pr1, sw1, False)
        return 0
    lax.fori_loop(0, NGRP // 2, pair_body, 0)
    pltpu.make_async_copy(pr0, out_hbm.at[pl.ds(0, GRP * BLK), :], sw0).wait()
    pltpu.make_async_copy(pr1, out_hbm.at[pl.ds(0, GRP * BLK), :], sw1).wait()

    @pl.when(wid < NTAIL)
    def _():
        tb = TAIL0 + wid
        pltpu.sync_copy(dst_hbm.at[tb], dst_v.at[0])
        pltpu.async_copy(pr_hbm.at[dst_v.at[0]],
                         pr0.at[pl.ds(0, BLK), :], sem_g).wait()
        pltpu.sync_copy(pr0.at[pl.ds(0, BLK), :],
                        out_hbm.at[pl.ds(tb * BLK, BLK), :])


_SC_MESH = plsc.VectorSubcoreMesh(core_axis_name="c", subcore_axis_name="s")
_SC_PARAMS = pltpu.CompilerParams(needs_layout_passes=False,
                                  use_tc_tiling_on_sc=False)
_f32 = jnp.float32


def kernel(node_features, edge_features, edge_index, node_tiers,
           w_node_score, b_node_score, w_edge_score, b_edge_score,
           wq, bq, wk, bk, wv, bv, wo, bo,
           w_c1, b_c1, w_c2, b_c2):
    del node_tiers
    src2d = edge_index[0].astype(jnp.int32).reshape(NBLK, BLK)
    dst2d = edge_index[1].astype(jnp.int32).reshape(NBLK, BLK)

    ns2, kmat = pl.pallas_call(
        _node_body,
        out_shape=[jax.ShapeDtypeStruct((N, 1), _f32),
                   jax.ShapeDtypeStruct((N, D), _f32)],
    )(node_features, w_node_score, b_node_score.reshape(1, 1),
      wk, bk.reshape(1, D))
    ns = ns2.reshape(N)

    sc_b = pl.kernel(
        _sc_score_gather_body,
        out_type=[jax.ShapeDtypeStruct((NBLK, BLK), _f32),
                  jax.ShapeDtypeStruct((NBLK, BLK), _f32)],
        mesh=_SC_MESH,
        scratch_types=[pltpu.VMEM((N,), _f32),
                       pltpu.VMEM((WB, BLK), jnp.int32),
                       pltpu.VMEM((WB, BLK), jnp.int32),
                       pltpu.VMEM((WB, BLK), _f32),
                       pltpu.VMEM((WB, BLK), _f32)],
        compiler_params=_SC_PARAMS,
    )
    sumns, min_ns = sc_b(ns, src2d, dst2d)

    sc_rowgather = pl.kernel(
        _sc_proj_gather_body,
        out_type=jax.ShapeDtypeStruct((E, D), _f32),
        mesh=_SC_MESH,
        scratch_types=[pltpu.VMEM((WB, BLK), jnp.int32),
                       pltpu.VMEM((GRP * BLK, D), _f32),
                       pltpu.VMEM((GRP * BLK, D), _f32),
                       pltpu.SemaphoreType.DMA,
                       pltpu.SemaphoreType.DMA,
                       pltpu.SemaphoreType.DMA],
        compiler_params=_SC_PARAMS,
    )
    kdst = sc_rowgather(kmat, dst2d)

    ns_pad = jnp.pad(ns, (0, NPAD - N), constant_values=-jnp.inf)
    ef2048 = edge_features.reshape(NBLK, BLK * D)
    w2048 = pl.pallas_call(
        _thresh_body,
        out_shape=jax.ShapeDtypeStruct((NBLK, BLK * D), _f32),
    )(ef2048, sumns, min_ns,
      ns_pad.reshape(NPAD // 128, 128), w_edge_score,
      b_edge_score.reshape(1, 1))
    w8 = w2048.reshape(EP8, 128)
    kd8 = kdst.reshape(EP8, 128)

    pay_n8, pay_d8 = pl.pallas_call(
        _payload_body,
        grid=(GE,),
        in_specs=[pl.BlockSpec((RB, 128), lambda i: (i, 0)),
                  pl.BlockSpec((RB, 128), lambda i: (i, 0)),
                  pl.BlockSpec((D, D), lambda i: (0, 0)),
                  pl.BlockSpec((1, D), lambda i: (0, 0)),
                  pl.BlockSpec((D, D), lambda i: (0, 0)),
                  pl.BlockSpec((1, D), lambda i: (0, 0))],
        out_specs=[pl.BlockSpec((RB, 128), lambda i: (i, 0)),
                   pl.BlockSpec((RB, 128), lambda i: (i, 0))],
        out_shape=[jax.ShapeDtypeStruct((EP8, 128), _f32),
                   jax.ShapeDtypeStruct((EP8, 128), _f32)],
    )(w8, kd8, wq, bq.reshape(1, D), wv, bv.reshape(1, D))
    pay_n = pay_n8.reshape(E, D)
    pay_d = pay_d8.reshape(E, D)

    zeros_nd = jnp.zeros((N, D), _f32)
    sc_e = pl.kernel(
        _sc_scatter_body,
        out_type=[jax.ShapeDtypeStruct((NC, N, D), _f32),
                  jax.ShapeDtypeStruct((NC, N, D), _f32)],
        mesh=_SC_MESH,
        scratch_types=[pltpu.VMEM_SHARED((N, D), _f32),
                       pltpu.VMEM_SHARED((N, D), _f32),
                       pltpu.VMEM((GRPE * BLK, D), _f32),
                       pltpu.VMEM((GRPE * BLK, D), _f32),
                       pltpu.VMEM((GRPE * BLK, D), _f32),
                       pltpu.VMEM((GRPE * BLK, D), _f32),
                       pltpu.VMEM((WB, BLK), jnp.int32),
                       pltpu.SemaphoreType.DMA,
                       pltpu.SemaphoreType.DMA],
        compiler_params=_SC_PARAMS,
    )
    acc_n, acc_d = sc_e(pay_n, pay_d, dst2d, zeros_nd)

    proj = pl.pallas_call(
        _proj_body,
        out_shape=jax.ShapeDtypeStruct((N, D), _f32),
    )(acc_n, acc_d, wo, bo.reshape(1, D))

    projd = sc_rowgather(proj, dst2d)

    pj8 = projd.reshape(EP8, 128)
    out8 = pl.pallas_call(
        _head_body,
        grid=(GE,),
        in_specs=[pl.BlockSpec((RB, 128), lambda i: (i, 0)),
                  pl.BlockSpec((RB, 128), lambda i: (i, 0)),
                  pl.BlockSpec((D, D), lambda i: (0, 0)),
                  pl.BlockSpec((1, D), lambda i: (0, 0)),
                  pl.BlockSpec((D, NUM_CLASSES), lambda i: (0, 0)),
                  pl.BlockSpec((1, NUM_CLASSES), lambda i: (0, 0))],
        out_specs=pl.BlockSpec((RB, 128), lambda i: (i, 0)),
        out_shape=jax.ShapeDtypeStruct((EP8, 128), _f32),
    )(w8, pj8, w_c1, b_c1.reshape(1, D), w_c2,
      b_c2.reshape(1, NUM_CLASSES))
    return out8.reshape(E, NUM_CLASSES)
